# Initial kernel scaffold; baseline (speedup 1.0000x reference)
import jax
import jax.numpy as jnp
from jax.experimental import pallas as pl


def kernel(X, A, S, R, X2, A2, S2, R2, y_pred, Theta, weight, weight2, weight31, weight32, W11, W12, Wd11, Wd12, W21, W22, Wd21, Wd22, W31, Wd31):
    raise NotImplementedError("write your pallas kernel here")


# speedup vs baseline: 1.2602x; 1.2602x over previous
import functools

import jax
import jax.numpy as jnp
from jax import lax
from jax.experimental import pallas as pl
from jax.experimental.pallas import tpu as pltpu
from jax.experimental.pallas import tpu_sc as plsc


def _mm_body(x_ref, w_ref, o_ref):
    o_ref[:, :] = jnp.dot(x_ref[:, :], w_ref[:, :],
                          preferred_element_type=jnp.float32)


def _mm(x, w, bm=256):
    m, k = x.shape
    _, n = w.shape
    return pl.pallas_call(
        _mm_body,
        grid=(m // bm,),
        in_specs=[pl.BlockSpec((bm, k), lambda i: (i, 0)),
                  pl.BlockSpec((k, n), lambda i: (0, 0))],
        out_specs=pl.BlockSpec((bm, n), lambda i: (i, 0)),
        out_shape=jax.ShapeDtypeStruct((m, n), jnp.float32),
    )(x, w)


def _elu(x):
    return jnp.where(x > 0, x, jnp.exp(x) - 1.0)


def _amm_elu_body(a_ref, p_ref, o_ref, acc_ref, *, nk):
    k = pl.program_id(1)

    @pl.when(k == 0)
    def _():
        acc_ref[:, :] = jnp.zeros_like(acc_ref)

    acc_ref[:, :] += jnp.dot(a_ref[:, :], p_ref[:, :],
                             preferred_element_type=jnp.float32)

    @pl.when(k == nk - 1)
    def _():
        o_ref[:, :] = _elu(acc_ref[:, :])


def _amm_elu(a, p, bm=256, bk=512):
    m, kk = a.shape
    _, n = p.shape
    nk = kk // bk
    return pl.pallas_call(
        functools.partial(_amm_elu_body, nk=nk),
        grid=(m // bm, nk),
        in_specs=[pl.BlockSpec((bm, bk), lambda i, k: (i, k)),
                  pl.BlockSpec((bk, n), lambda i, k: (k, 0))],
        out_specs=pl.BlockSpec((bm, n), lambda i, k: (i, 0)),
        out_shape=jax.ShapeDtypeStruct((m, n), jnp.float32),
        scratch_shapes=[pltpu.VMEM((bm, n), jnp.float32)],
    )(a, p)


def _amm_elu_ft_body(a_ref, p_ref, t_ref, o_ref, acc_ref, *, nk):
    i = pl.program_id(0)
    k = pl.program_id(1)

    @pl.when((i == 0) & (k == 0))
    def _():
        o_ref[0, 0] = 0.0

    @pl.when(k == 0)
    def _():
        acc_ref[:, :] = jnp.zeros_like(acc_ref)

    acc_ref[:, :] += jnp.dot(a_ref[:, :], p_ref[:, :],
                             preferred_element_type=jnp.float32)

    @pl.when(k == nk - 1)
    def _():
        d = t_ref[:, :] - _elu(acc_ref[:, :])
        o_ref[0, 0] += jnp.sum(d * d)


def _amm_elu_ft(a, p, t, bm=256, bk=512):
    m, kk = a.shape
    _, n = p.shape
    nk = kk // bk
    out = pl.pallas_call(
        functools.partial(_amm_elu_ft_body, nk=nk),
        grid=(m // bm, nk),
        in_specs=[pl.BlockSpec((bm, bk), lambda i, k: (i, k)),
                  pl.BlockSpec((bk, n), lambda i, k: (k, 0)),
                  pl.BlockSpec((bm, n), lambda i, k: (i, 0))],
        out_specs=pl.BlockSpec((1, 1), lambda i, k: (0, 0),
                               memory_space=pltpu.SMEM),
        out_shape=jax.ShapeDtypeStruct((1, 1), jnp.float32),
        scratch_shapes=[pltpu.VMEM((bm, n), jnp.float32)],
    )(a, p, t)
    return out[0, 0]


def _coef_mm_body(w_ref, h_ref, hi_ref, o_ref, se_ref, acc_ref, *, nk, bm, bk):
    i = pl.program_id(0)
    k = pl.program_id(1)

    @pl.when((i == 0) & (k == 0))
    def _():
        se_ref[0, 0] = 0.0

    @pl.when(k == 0)
    def _():
        acc_ref[:, :] = jnp.zeros_like(acc_ref)

    rows = lax.broadcasted_iota(jnp.int32, (bm, bk), 0) + i * bm
    cols = lax.broadcasted_iota(jnp.int32, (bm, bk), 1) + k * bk
    wblk = jnp.where(rows == cols, 0.0, w_ref[:, :])
    acc_ref[:, :] += jnp.dot(wblk, h_ref[:, :],
                             preferred_element_type=jnp.float32)

    @pl.when(k == nk - 1)
    def _():
        hc = acc_ref[:, :]
        o_ref[:, :] = hc
        d = hi_ref[:, :] - hc
        se_ref[0, 0] += jnp.sum(d * d)


def _coef_mm(w, h, bm=256, bk=512):
    m, kk = w.shape
    _, n = h.shape
    nk = kk // bk
    hc, se = pl.pallas_call(
        functools.partial(_coef_mm_body, nk=nk, bm=bm, bk=bk),
        grid=(m // bm, nk),
        in_specs=[pl.BlockSpec((bm, bk), lambda i, k: (i, k)),
                  pl.BlockSpec((bk, n), lambda i, k: (k, 0)),
                  pl.BlockSpec((bm, n), lambda i, k: (i, 0))],
        out_specs=[pl.BlockSpec((bm, n), lambda i, k: (i, 0)),
                   pl.BlockSpec((1, 1), lambda i, k: (0, 0),
                                memory_space=pltpu.SMEM)],
        out_shape=[jax.ShapeDtypeStruct((m, n), jnp.float32),
                   jax.ShapeDtypeStruct((1, 1), jnp.float32)],
        scratch_shapes=[pltpu.VMEM((bm, n), jnp.float32)],
    )(w, h, h)
    return hc, se[0, 0]


def _prep_body(w_ref, w2_ref, w31_ref, w32_ref, tt_ref,
               c3_ref, zis_ref, zjs_ref, pos_ref,
               creg_ref, cq_ref, cons_ref, *, bm):
    i = pl.program_id(0)

    @pl.when(i == 0)
    def _():
        creg_ref[0, 0] = 0.0
        cq_ref[0, 0] = 0.0
        cons_ref[0, 0] = 0.0

    n = w_ref.shape[1]
    rows = lax.broadcasted_iota(jnp.int32, (bm, n), 0) + i * bm
    cols = lax.broadcasted_iota(jnp.int32, (bm, n), 1)
    diag = rows == cols
    c = jnp.where(diag, 0.0, w_ref[:, :])
    c2 = jnp.where(diag, 0.0, w2_ref[:, :])
    c31 = jnp.where(diag, 0.0, w31_ref[:, :])
    c32 = jnp.where(diag, 0.0, w32_ref[:, :])
    c3 = 0.7 * c31 + 0.3 * c32
    c3_ref[:, :] = c3
    creg_ref[0, 0] += (jnp.sum(jnp.abs(c)) + jnp.sum(jnp.abs(c2))
                       + jnp.sum(jnp.abs(c31)) + jnp.sum(jnp.abs(c32)))
    cq_ref[0, 0] += jnp.sum(jnp.abs(c3 * tt_ref[:, :]))
    cons_ref[0, 0] += jnp.sum((c3 - c) ** 2) + jnp.sum((c3 - c2) ** 2)
    n31 = jnp.sqrt(jnp.sum(c31 * c31, axis=1, keepdims=True))
    n32 = jnp.sqrt(jnp.sum(c32 * c32, axis=1, keepdims=True))
    zis = c31 / jnp.maximum(n31, 1e-12)
    zjs = c32 / jnp.maximum(n32, 1e-12)
    zis_ref[:, :] = zis.astype(jnp.bfloat16)
    zjs_ref[:, :] = zjs.astype(jnp.bfloat16)
    pos_ref[:, :] = jnp.sum(zis * zjs, axis=1, keepdims=True)


def _prep(w, w2, w31, w32, theta_t, bm=128):
    n = w.shape[0]
    outs = pl.pallas_call(
        functools.partial(_prep_body, bm=bm),
        grid=(n // bm,),
        in_specs=[pl.BlockSpec((bm, n), lambda i: (i, 0))] * 5,
        out_specs=[pl.BlockSpec((bm, n), lambda i: (i, 0)),
                   pl.BlockSpec((bm, n), lambda i: (i, 0)),
                   pl.BlockSpec((bm, n), lambda i: (i, 0)),
                   pl.BlockSpec((bm, 1), lambda i: (i, 0)),
                   pl.BlockSpec((1, 1), lambda i: (0, 0),
                                memory_space=pltpu.SMEM),
                   pl.BlockSpec((1, 1), lambda i: (0, 0),
                                memory_space=pltpu.SMEM),
                   pl.BlockSpec((1, 1), lambda i: (0, 0),
                                memory_space=pltpu.SMEM)],
        out_shape=[jax.ShapeDtypeStruct((n, n), jnp.float32),
                   jax.ShapeDtypeStruct((n, n), jnp.bfloat16),
                   jax.ShapeDtypeStruct((n, n), jnp.bfloat16),
                   jax.ShapeDtypeStruct((n, 1), jnp.float32),
                   jax.ShapeDtypeStruct((1, 1), jnp.float32),
                   jax.ShapeDtypeStruct((1, 1), jnp.float32),
                   jax.ShapeDtypeStruct((1, 1), jnp.float32)],
    )(w, w2, w31, w32, theta_t)
    c3, zis, zjs, pos, creg, cq, cons = outs
    return c3, zis, zjs, pos, creg[0, 0], cq[0, 0], cons[0, 0]


_DN = (((1,), (1,)), ((), ()))


def _gram_body(zis_i, zjs_i, zis_j, zjs_j, y_i, yt_j, pos_ref, cl_ref,
               a1, a2, a3, neg1, neg2, *, nmi, nmj, nk, bm, bn):
    i = pl.program_id(0)
    j = pl.program_id(1)
    k = pl.program_id(2)

    @pl.when((i == 0) & (j == 0) & (k == 0))
    def _():
        neg1[:, :] = jnp.zeros_like(neg1)
        neg2[:, :] = jnp.zeros_like(neg2)

    @pl.when(k == 0)
    def _():
        a1[:, :] = jnp.zeros_like(a1)
        a2[:, :] = jnp.zeros_like(a2)
        a3[:, :] = jnp.zeros_like(a3)

    a1[:, :] += lax.dot_general(zis_i[:, :], zjs_j[:, :], _DN,
                                preferred_element_type=jnp.float32)
    a2[:, :] += lax.dot_general(zis_i[:, :], zis_j[:, :], _DN,
                                preferred_element_type=jnp.float32)
    a3[:, :] += lax.dot_general(zjs_i[:, :], zjs_j[:, :], _DN,
                                preferred_element_type=jnp.float32)

    @pl.when(k == nk - 1)
    def _():
        nm = (y_i[:, :] != yt_j[:, :]).astype(jnp.float32)
        e1 = jnp.exp(a1[:, :]) * nm
        e2 = jnp.exp(a2[:, :]) * nm
        e3 = jnp.exp(a3[:, :]) * nm
        neg1[pl.ds(i * bm, bm), :] += jnp.sum(e1 + e2, axis=1, keepdims=True)
        neg2[pl.ds(i * bm, bm), :] += jnp.sum(e3, axis=1, keepdims=True)
        neg2[pl.ds(j * bn, bn), :] += jnp.sum(e1, axis=0)[:, None]

        @pl.when((i == nmi - 1) & (j == nmj - 1))
        def _():
            p = pos_ref[:, :]
            lp = jnp.exp(p)
            cl_ref[0, 0] = jnp.sum(jnp.log(lp + neg1[:, :])
                                   + jnp.log(lp + neg2[:, :]) - 2.0 * p)


def _gram(zis, zjs, y, yt, pos, bm=256, bn=256, bk=512):
    n = zis.shape[0]
    nmi, nmj, nk = n // bm, n // bn, n // bk
    cl = pl.pallas_call(
        functools.partial(_gram_body, nmi=nmi, nmj=nmj, nk=nk, bm=bm, bn=bn),
        grid=(nmi, nmj, nk),
        in_specs=[pl.BlockSpec((bm, bk), lambda i, j, k: (i, k)),
                  pl.BlockSpec((bm, bk), lambda i, j, k: (i, k)),
                  pl.BlockSpec((bn, bk), lambda i, j, k: (j, k)),
                  pl.BlockSpec((bn, bk), lambda i, j, k: (j, k)),
                  pl.BlockSpec((bm, 1), lambda i, j, k: (i, 0)),
                  pl.BlockSpec((1, bn), lambda i, j, k: (0, j)),
                  pl.BlockSpec((n, 1), lambda i, j, k: (0, 0))],
        out_specs=pl.BlockSpec((1, 1), lambda i, j, k: (0, 0),
                               memory_space=pltpu.SMEM),
        out_shape=jax.ShapeDtypeStruct((1, 1), jnp.float32),
        scratch_shapes=[pltpu.VMEM((bm, bn), jnp.float32),
                        pltpu.VMEM((bm, bn), jnp.float32),
                        pltpu.VMEM((bm, bn), jnp.float32),
                        pltpu.VMEM((n, 1), jnp.float32),
                        pltpu.VMEM((n, 1), jnp.float32)],
    )(zis, zis, zjs, zjs, y, yt, pos)
    return cl[0, 0]


def _edge_dots(h, s, r):
    n, d = h.shape
    e = s.shape[0]
    info = plsc.get_sparse_core_info()
    nw = info.num_cores * info.num_subcores
    per_w = e // nw
    ch = 128
    nch = per_w // ch
    mesh = plsc.VectorSubcoreMesh(core_axis_name="c", subcore_axis_name="s")

    def body(h_hbm, s_hbm, r_hbm, out_hbm, sidx, ridx, arow, brow, ovec,
             sem1, sem2):
        wid = lax.axis_index("s") * info.num_cores + lax.axis_index("c")

        def chunk(c, carry):
            base = wid * per_w + c * ch
            pltpu.sync_copy(s_hbm.at[pl.ds(base, ch)], sidx)
            pltpu.sync_copy(r_hbm.at[pl.ds(base, ch)], ridx)
            cp1 = pltpu.async_copy(h_hbm.at[sidx], arow, sem1)
            cp2 = pltpu.async_copy(h_hbm.at[ridx], brow, sem2)
            cp1.wait()
            cp2.wait()

            def edge(ei, cc):
                acc = arow[ei, pl.ds(0, 16)] * brow[ei, pl.ds(0, 16)]
                for g in range(1, d // 16):
                    acc = acc + (arow[ei, pl.ds(g * 16, 16)]
                                 * brow[ei, pl.ds(g * 16, 16)])
                ovec[ei, :] = acc
                return cc

            lax.fori_loop(0, ch, edge, 0)
            pltpu.sync_copy(ovec, out_hbm.at[pl.ds(base, ch)])
            return carry

        lax.fori_loop(0, nch, chunk, 0)

    return pl.kernel(
        body,
        out_type=jax.ShapeDtypeStruct((e, 16), jnp.float32),
        mesh=mesh,
        scratch_types=[pltpu.VMEM((ch,), jnp.int32),
                       pltpu.VMEM((ch,), jnp.int32),
                       pltpu.VMEM((ch, d), jnp.float32),
                       pltpu.VMEM((ch, d), jnp.float32),
                       pltpu.VMEM((ch, 16), jnp.float32),
                       pltpu.SemaphoreType.DMA,
                       pltpu.SemaphoreType.DMA],
    )(h, s, r)


def _st_body(d1, d2, d3, d4, o_ref):
    i = pl.program_id(0)

    @pl.when(i == 0)
    def _():
        o_ref[0, 0] = 0.0

    tot = 0.0
    for dref in (d1, d2, d3, d4):
        dot = jnp.sum(dref[:, :], axis=1)
        tot += jnp.sum(jnp.maximum(-dot, 0.0)
                       + jnp.log(1.0 + jnp.exp(-jnp.abs(dot))))
    o_ref[0, 0] += tot


def _st_reduce(d1, d2, d3, d4, be=8192):
    e = d1.shape[0]
    out = pl.pallas_call(
        _st_body,
        grid=(e // be,),
        in_specs=[pl.BlockSpec((be, 16), lambda i: (i, 0))] * 4,
        out_specs=pl.BlockSpec((1, 1), lambda i: (0, 0),
                               memory_space=pltpu.SMEM),
        out_shape=jax.ShapeDtypeStruct((1, 1), jnp.float32),
    )(d1, d2, d3, d4)
    return out[0, 0]


def kernel(X, A, S, R, X2, A2, S2, R2, y_pred, Theta,
           weight, weight2, weight31, weight32,
           W11, W12, Wd11, Wd12, W21, W22, Wd21, Wd22, W31, Wd31):
    n, f1 = X.shape
    f2 = X2.shape[1]
    h2 = W12.shape[1]
    h3 = W31.shape[1]

    pad = 128 - h3
    W31p = jnp.pad(W31, ((0, 0), (0, pad)))
    Wd31p = jnp.pad(Wd31, ((0, pad), (0, 0)))

    H = _amm_elu(A, _mm(_amm_elu(A, _mm(X, W11)), W12))
    Hb = _amm_elu(A2, _mm(_amm_elu(A2, _mm(X2, W21)), W22))

    d1 = _edge_dots(H, S, R)
    d2 = _edge_dots(Hb, S2, R2)

    c3, zis, zjs, pos, creg, cq, cons = _prep(
        weight, weight2, weight31, weight32, Theta.T)

    HC, se1 = _coef_mm(weight, H)
    ft1 = _amm_elu_ft(A, _mm(_amm_elu(A, _mm(HC, Wd11)), Wd12), X)
    HC2, se2 = _coef_mm(weight2, Hb)
    ft2 = _amm_elu_ft(A2, _mm(_amm_elu(A2, _mm(HC2, Wd21)), Wd22), X2)

    H31 = _amm_elu(A, _mm(H, W31p))
    H32 = _amm_elu(A2, _mm(Hb, W31p))
    d3 = _edge_dots(H31, S, R)
    d4 = _edge_dots(H32, S2, R2)
    HC31, se3 = _coef_mm(weight31, H31)
    HC32, se4 = _coef_mm(weight32, H32)
    ft3 = _amm_elu_ft(A, _mm(HC31, Wd31p), H)
    ft4 = _amm_elu_ft(A2, _mm(HC32, Wd31p), Hb)

    yt = y_pred.reshape(1, n)
    cl_sum = _gram(zis, zjs, y_pred, yt, pos)

    st_loss = _st_reduce(d1, d2, d3, d4)

    ft_loss = (ft1 / (n * f1) + ft2 / (n * f2)
               + ft3 / (n * h2) + ft4 / (n * h2))
    se_loss = 0.5 * (se1 / (n * h2) + se2 / (n * h2)
                     + se3 / (n * h3) + se4 / (n * h3))
    cl_loss = cl_sum / (2.0 * n)

    loss = (ft_loss + 0.1 * st_loss + se_loss + 0.1 * creg
            + 0.1 * cl_loss + 0.1 * cq + 0.1 * cons)
    return (loss, ft_loss, st_loss, se_loss, creg, cons, cl_loss, cq, c3)


# speedup vs baseline: 2.0053x; 1.5913x over previous
import functools

import jax
import jax.numpy as jnp
from jax import lax
from jax.experimental import pallas as pl
from jax.experimental.pallas import tpu as pltpu
from jax.experimental.pallas import tpu_sc as plsc


def _mm_body(x_ref, w_ref, o_ref):
    o_ref[:, :] = jnp.dot(x_ref[:, :], w_ref[:, :],
                          preferred_element_type=jnp.float32)


def _mm(x, w, bm=256):
    m, k = x.shape
    _, n = w.shape
    return pl.pallas_call(
        _mm_body,
        grid=(m // bm,),
        in_specs=[pl.BlockSpec((bm, k), lambda i: (i, 0)),
                  pl.BlockSpec((k, n), lambda i: (0, 0))],
        out_specs=pl.BlockSpec((bm, n), lambda i: (i, 0)),
        out_shape=jax.ShapeDtypeStruct((m, n), jnp.float32),
    )(x, w)


def _elu(x):
    return jnp.where(x > 0, x, jnp.exp(x) - 1.0)


def _amm_elu_body(a_ref, p_ref, o_ref, acc_ref, *, nk):
    k = pl.program_id(1)

    @pl.when(k == 0)
    def _():
        acc_ref[:, :] = jnp.zeros_like(acc_ref)

    acc_ref[:, :] += jnp.dot(a_ref[:, :], p_ref[:, :],
                             preferred_element_type=jnp.float32)

    @pl.when(k == nk - 1)
    def _():
        o_ref[:, :] = _elu(acc_ref[:, :])


def _amm_elu(a, p, bm=256, bk=512):
    m, kk = a.shape
    _, n = p.shape
    nk = kk // bk
    return pl.pallas_call(
        functools.partial(_amm_elu_body, nk=nk),
        grid=(m // bm, nk),
        in_specs=[pl.BlockSpec((bm, bk), lambda i, k: (i, k)),
                  pl.BlockSpec((bk, n), lambda i, k: (k, 0))],
        out_specs=pl.BlockSpec((bm, n), lambda i, k: (i, 0)),
        out_shape=jax.ShapeDtypeStruct((m, n), jnp.float32),
        scratch_shapes=[pltpu.VMEM((bm, n), jnp.float32)],
    )(a, p)


def _amm_elu_ft_body(a_ref, p_ref, t_ref, o_ref, acc_ref, *, nk):
    i = pl.program_id(0)
    k = pl.program_id(1)

    @pl.when((i == 0) & (k == 0))
    def _():
        o_ref[0, 0] = 0.0

    @pl.when(k == 0)
    def _():
        acc_ref[:, :] = jnp.zeros_like(acc_ref)

    acc_ref[:, :] += jnp.dot(a_ref[:, :], p_ref[:, :],
                             preferred_element_type=jnp.float32)

    @pl.when(k == nk - 1)
    def _():
        d = t_ref[:, :] - _elu(acc_ref[:, :])
        o_ref[0, 0] += jnp.sum(d * d)


def _amm_elu_ft(a, p, t, bm=256, bk=512):
    m, kk = a.shape
    _, n = p.shape
    nk = kk // bk
    out = pl.pallas_call(
        functools.partial(_amm_elu_ft_body, nk=nk),
        grid=(m // bm, nk),
        in_specs=[pl.BlockSpec((bm, bk), lambda i, k: (i, k)),
                  pl.BlockSpec((bk, n), lambda i, k: (k, 0)),
                  pl.BlockSpec((bm, n), lambda i, k: (i, 0))],
        out_specs=pl.BlockSpec((1, 1), lambda i, k: (0, 0),
                               memory_space=pltpu.SMEM),
        out_shape=jax.ShapeDtypeStruct((1, 1), jnp.float32),
        scratch_shapes=[pltpu.VMEM((bm, n), jnp.float32)],
    )(a, p, t)
    return out[0, 0]


def _coef_mm_body(w_ref, h_ref, hi_ref, o_ref, se_ref, acc_ref, *, nk, bm, bk):
    i = pl.program_id(0)
    k = pl.program_id(1)

    @pl.when((i == 0) & (k == 0))
    def _():
        se_ref[0, 0] = 0.0

    @pl.when(k == 0)
    def _():
        acc_ref[:, :] = jnp.zeros_like(acc_ref)

    rows = lax.broadcasted_iota(jnp.int32, (bm, bk), 0) + i * bm
    cols = lax.broadcasted_iota(jnp.int32, (bm, bk), 1) + k * bk
    wblk = jnp.where(rows == cols, 0.0, w_ref[:, :])
    acc_ref[:, :] += jnp.dot(wblk, h_ref[:, :],
                             preferred_element_type=jnp.float32)

    @pl.when(k == nk - 1)
    def _():
        hc = acc_ref[:, :]
        o_ref[:, :] = hc
        d = hi_ref[:, :] - hc
        se_ref[0, 0] += jnp.sum(d * d)


def _coef_mm(w, h, bm=256, bk=512):
    m, kk = w.shape
    _, n = h.shape
    nk = kk // bk
    hc, se = pl.pallas_call(
        functools.partial(_coef_mm_body, nk=nk, bm=bm, bk=bk),
        grid=(m // bm, nk),
        in_specs=[pl.BlockSpec((bm, bk), lambda i, k: (i, k)),
                  pl.BlockSpec((bk, n), lambda i, k: (k, 0)),
                  pl.BlockSpec((bm, n), lambda i, k: (i, 0))],
        out_specs=[pl.BlockSpec((bm, n), lambda i, k: (i, 0)),
                   pl.BlockSpec((1, 1), lambda i, k: (0, 0),
                                memory_space=pltpu.SMEM)],
        out_shape=[jax.ShapeDtypeStruct((m, n), jnp.float32),
                   jax.ShapeDtypeStruct((1, 1), jnp.float32)],
        scratch_shapes=[pltpu.VMEM((bm, n), jnp.float32)],
    )(w, h, h)
    return hc, se[0, 0]


def _prep_body(w_ref, w2_ref, w31_ref, w32_ref, tt_ref,
               c3_ref, zis_ref, zjs_ref, pos_ref,
               creg_ref, cq_ref, cons_ref, *, bm):
    i = pl.program_id(0)

    @pl.when(i == 0)
    def _():
        creg_ref[0, 0] = 0.0
        cq_ref[0, 0] = 0.0
        cons_ref[0, 0] = 0.0

    n = w_ref.shape[1]
    rows = lax.broadcasted_iota(jnp.int32, (bm, n), 0) + i * bm
    cols = lax.broadcasted_iota(jnp.int32, (bm, n), 1)
    diag = rows == cols
    c = jnp.where(diag, 0.0, w_ref[:, :])
    c2 = jnp.where(diag, 0.0, w2_ref[:, :])
    c31 = jnp.where(diag, 0.0, w31_ref[:, :])
    c32 = jnp.where(diag, 0.0, w32_ref[:, :])
    c3 = 0.7 * c31 + 0.3 * c32
    c3_ref[:, :] = c3
    creg_ref[0, 0] += (jnp.sum(jnp.abs(c)) + jnp.sum(jnp.abs(c2))
                       + jnp.sum(jnp.abs(c31)) + jnp.sum(jnp.abs(c32)))
    cq_ref[0, 0] += jnp.sum(jnp.abs(c3 * tt_ref[:, :]))
    cons_ref[0, 0] += jnp.sum((c3 - c) ** 2) + jnp.sum((c3 - c2) ** 2)
    n31 = jnp.sqrt(jnp.sum(c31 * c31, axis=1, keepdims=True))
    n32 = jnp.sqrt(jnp.sum(c32 * c32, axis=1, keepdims=True))
    zis = c31 / jnp.maximum(n31, 1e-12)
    zjs = c32 / jnp.maximum(n32, 1e-12)
    zis_ref[:, :] = zis.astype(jnp.bfloat16)
    zjs_ref[:, :] = zjs.astype(jnp.bfloat16)
    pos_ref[:, :] = jnp.sum(zis * zjs, axis=1, keepdims=True)


def _prep(w, w2, w31, w32, theta_t, bm=128):
    n = w.shape[0]
    outs = pl.pallas_call(
        functools.partial(_prep_body, bm=bm),
        grid=(n // bm,),
        in_specs=[pl.BlockSpec((bm, n), lambda i: (i, 0))] * 5,
        out_specs=[pl.BlockSpec((bm, n), lambda i: (i, 0)),
                   pl.BlockSpec((bm, n), lambda i: (i, 0)),
                   pl.BlockSpec((bm, n), lambda i: (i, 0)),
                   pl.BlockSpec((bm, 1), lambda i: (i, 0)),
                   pl.BlockSpec((1, 1), lambda i: (0, 0),
                                memory_space=pltpu.SMEM),
                   pl.BlockSpec((1, 1), lambda i: (0, 0),
                                memory_space=pltpu.SMEM),
                   pl.BlockSpec((1, 1), lambda i: (0, 0),
                                memory_space=pltpu.SMEM)],
        out_shape=[jax.ShapeDtypeStruct((n, n), jnp.float32),
                   jax.ShapeDtypeStruct((n, n), jnp.bfloat16),
                   jax.ShapeDtypeStruct((n, n), jnp.bfloat16),
                   jax.ShapeDtypeStruct((n, 1), jnp.float32),
                   jax.ShapeDtypeStruct((1, 1), jnp.float32),
                   jax.ShapeDtypeStruct((1, 1), jnp.float32),
                   jax.ShapeDtypeStruct((1, 1), jnp.float32)],
    )(w, w2, w31, w32, theta_t)
    c3, zis, zjs, pos, creg, cq, cons = outs
    return c3, zis, zjs, pos, creg[0, 0], cq[0, 0], cons[0, 0]


_DN = (((1,), (1,)), ((), ()))


def _gram_body(zis_i, zjs_i, zis_j, zjs_j, y_i, yt_j, pos_ref, post_ref,
               cl_ref, a1, a2, a3, neg1, neg2, *, nmi, nmj, nk, bm, bn):
    i = pl.program_id(0)
    j = pl.program_id(1)
    k = pl.program_id(2)

    @pl.when((i == 0) & (j == 0) & (k == 0))
    def _():
        neg1[:, :] = jnp.zeros_like(neg1)
        neg2[:, :] = jnp.zeros_like(neg2)

    @pl.when(k == 0)
    def _():
        a1[:, :] = jnp.zeros_like(a1)
        a2[:, :] = jnp.zeros_like(a2)
        a3[:, :] = jnp.zeros_like(a3)

    a1[:, :] += lax.dot_general(zis_i[:, :], zjs_j[:, :], _DN,
                                preferred_element_type=jnp.float32)
    a2[:, :] += lax.dot_general(zis_i[:, :], zis_j[:, :], _DN,
                                preferred_element_type=jnp.float32)
    a3[:, :] += lax.dot_general(zjs_i[:, :], zjs_j[:, :], _DN,
                                preferred_element_type=jnp.float32)

    @pl.when(k == nk - 1)
    def _():
        nm = (y_i[:, :] != yt_j[:, :]).astype(jnp.float32)
        e1 = jnp.exp(a1[:, :]) * nm
        e2 = jnp.exp(a2[:, :]) * nm
        e3 = jnp.exp(a3[:, :]) * nm
        neg1[pl.ds(i * bm, bm), :] += jnp.sum(e1 + e2, axis=1, keepdims=True)
        neg2[:, pl.ds(j * bn, bn)] += jnp.sum(e1 + e3, axis=0)[None, :]

        @pl.when((i == nmi - 1) & (j == nmj - 1))
        def _():
            p = pos_ref[:, :]
            pt = post_ref[:, :]
            cl_ref[0, 0] = (jnp.sum(jnp.log(jnp.exp(p) + neg1[:, :]) - p)
                            + jnp.sum(jnp.log(jnp.exp(pt) + neg2[:, :]) - pt))


def _gram(zis, zjs, y, yt, pos, post, bm=512, bn=512, bk=1024):
    n = zis.shape[0]
    nmi, nmj, nk = n // bm, n // bn, n // bk
    cl = pl.pallas_call(
        functools.partial(_gram_body, nmi=nmi, nmj=nmj, nk=nk, bm=bm, bn=bn),
        grid=(nmi, nmj, nk),
        in_specs=[pl.BlockSpec((bm, bk), lambda i, j, k: (i, k)),
                  pl.BlockSpec((bm, bk), lambda i, j, k: (i, k)),
                  pl.BlockSpec((bn, bk), lambda i, j, k: (j, k)),
                  pl.BlockSpec((bn, bk), lambda i, j, k: (j, k)),
                  pl.BlockSpec((bm, 1), lambda i, j, k: (i, 0)),
                  pl.BlockSpec((1, bn), lambda i, j, k: (0, j)),
                  pl.BlockSpec((n, 1), lambda i, j, k: (0, 0)),
                  pl.BlockSpec((1, n), lambda i, j, k: (0, 0))],
        out_specs=pl.BlockSpec((1, 1), lambda i, j, k: (0, 0),
                               memory_space=pltpu.SMEM),
        out_shape=jax.ShapeDtypeStruct((1, 1), jnp.float32),
        scratch_shapes=[pltpu.VMEM((bm, bn), jnp.float32),
                        pltpu.VMEM((bm, bn), jnp.float32),
                        pltpu.VMEM((bm, bn), jnp.float32),
                        pltpu.VMEM((n, 1), jnp.float32),
                        pltpu.VMEM((1, n), jnp.float32)],
    )(zis, zis, zjs, zjs, y, yt, pos, post)
    return cl[0, 0]


def _edge_dots(h, s, r):
    n, d = h.shape
    e = s.shape[0]
    info = plsc.get_sparse_core_info()
    nw = info.num_cores * info.num_subcores
    per_w = e // nw
    ch = 128
    nch = per_w // ch
    mesh = plsc.VectorSubcoreMesh(core_axis_name="c", subcore_axis_name="s")

    def body(h_hbm, s_hbm, r_hbm, out_hbm, sidx, ridx, arow, brow, ovec,
             sem1, sem2):
        wid = lax.axis_index("s") * info.num_cores + lax.axis_index("c")

        def chunk(c, carry):
            base = wid * per_w + c * ch
            pltpu.sync_copy(s_hbm.at[pl.ds(base, ch)], sidx)
            pltpu.sync_copy(r_hbm.at[pl.ds(base, ch)], ridx)
            cp1 = pltpu.async_copy(h_hbm.at[sidx], arow, sem1)
            cp2 = pltpu.async_copy(h_hbm.at[ridx], brow, sem2)
            cp1.wait()
            cp2.wait()

            def edge(eo, cc):
                for sub in range(8):
                    ei = eo * 8 + sub
                    acc = arow[ei, pl.ds(0, 16)] * brow[ei, pl.ds(0, 16)]
                    for g in range(1, d // 16):
                        acc = acc + (arow[ei, pl.ds(g * 16, 16)]
                                     * brow[ei, pl.ds(g * 16, 16)])
                    ovec[eo, pl.ds(sub * 16, 16)] = acc
                return cc

            lax.fori_loop(0, ch // 8, edge, 0)
            obase = pl.multiple_of(base // 8, 8)
            pltpu.sync_copy(ovec, out_hbm.at[pl.ds(obase, ch // 8)])
            return carry

        lax.fori_loop(0, nch, chunk, 0)

    return pl.kernel(
        body,
        out_type=jax.ShapeDtypeStruct((e // 8, 128), jnp.float32),
        mesh=mesh,
        scratch_types=[pltpu.VMEM((ch,), jnp.int32),
                       pltpu.VMEM((ch,), jnp.int32),
                       pltpu.VMEM((ch, d), jnp.float32),
                       pltpu.VMEM((ch, d), jnp.float32),
                       pltpu.VMEM((ch // 8, 128), jnp.float32),
                       pltpu.SemaphoreType.DMA,
                       pltpu.SemaphoreType.DMA],
    )(h, s, r)


def _st_body(d1, d2, d3, d4, o_ref):
    i = pl.program_id(0)

    @pl.when(i == 0)
    def _():
        o_ref[0, 0] = 0.0

    seg = (lax.broadcasted_iota(jnp.int32, (128, 128), 0) // 16
           == lax.broadcasted_iota(jnp.int32, (128, 128), 1)
           ).astype(jnp.float32)
    colmask = lax.broadcasted_iota(jnp.int32, d1.shape, 1) < 8
    tot = 0.0
    for dref in (d1, d2, d3, d4):
        dot = jnp.dot(dref[:, :], seg, preferred_element_type=jnp.float32)
        sp = jnp.maximum(-dot, 0.0) + jnp.log(1.0 + jnp.exp(-jnp.abs(dot)))
        tot += jnp.sum(jnp.where(colmask, sp, 0.0))
    o_ref[0, 0] += tot


def _st_reduce(d1, d2, d3, d4, be=4096):
    e8 = d1.shape[0]
    out = pl.pallas_call(
        _st_body,
        grid=(e8 // be,),
        in_specs=[pl.BlockSpec((be, 128), lambda i: (i, 0))] * 4,
        out_specs=pl.BlockSpec((1, 1), lambda i: (0, 0),
                               memory_space=pltpu.SMEM),
        out_shape=jax.ShapeDtypeStruct((1, 1), jnp.float32),
    )(d1, d2, d3, d4)
    return out[0, 0]


def kernel(X, A, S, R, X2, A2, S2, R2, y_pred, Theta,
           weight, weight2, weight31, weight32,
           W11, W12, Wd11, Wd12, W21, W22, Wd21, Wd22, W31, Wd31):
    n, f1 = X.shape
    f2 = X2.shape[1]
    h2 = W12.shape[1]
    h3 = W31.shape[1]

    pad = 128 - h3
    W31p = jnp.pad(W31, ((0, 0), (0, pad)))
    Wd31p = jnp.pad(Wd31, ((0, pad), (0, 0)))

    H = _amm_elu(A, _mm(_amm_elu(A, _mm(X, W11)), W12))
    Hb = _amm_elu(A2, _mm(_amm_elu(A2, _mm(X2, W21)), W22))

    d1 = _edge_dots(H, S, R)
    d2 = _edge_dots(Hb, S2, R2)

    c3, zis, zjs, pos, creg, cq, cons = _prep(
        weight, weight2, weight31, weight32, Theta.T)

    HC, se1 = _coef_mm(weight, H)
    ft1 = _amm_elu_ft(A, _mm(_amm_elu(A, _mm(HC, Wd11)), Wd12), X)
    HC2, se2 = _coef_mm(weight2, Hb)
    ft2 = _amm_elu_ft(A2, _mm(_amm_elu(A2, _mm(HC2, Wd21)), Wd22), X2)

    H31 = _amm_elu(A, _mm(H, W31p))
    H32 = _amm_elu(A2, _mm(Hb, W31p))
    d3 = _edge_dots(H31, S, R)
    d4 = _edge_dots(H32, S2, R2)
    HC31, se3 = _coef_mm(weight31, H31)
    HC32, se4 = _coef_mm(weight32, H32)
    ft3 = _amm_elu_ft(A, _mm(HC31, Wd31p), H)
    ft4 = _amm_elu_ft(A2, _mm(HC32, Wd31p), Hb)

    yt = y_pred.reshape(1, n)
    cl_sum = _gram(zis, zjs, y_pred, yt, pos, pos.reshape(1, n))

    st_loss = _st_reduce(d1, d2, d3, d4)

    ft_loss = (ft1 / (n * f1) + ft2 / (n * f2)
               + ft3 / (n * h2) + ft4 / (n * h2))
    se_loss = 0.5 * (se1 / (n * h2) + se2 / (n * h2)
                     + se3 / (n * h3) + se4 / (n * h3))
    cl_loss = cl_sum / (2.0 * n)

    loss = (ft_loss + 0.1 * st_loss + se_loss + 0.1 * creg
            + 0.1 * cl_loss + 0.1 * cq + 0.1 * cons)
    return (loss, ft_loss, st_loss, se_loss, creg, cons, cl_loss, cq, c3)


# speedup vs baseline: 2.2437x; 1.1189x over previous
import functools

import jax
import jax.numpy as jnp
from jax import lax
from jax.experimental import pallas as pl
from jax.experimental.pallas import tpu as pltpu
from jax.experimental.pallas import tpu_sc as plsc


def _split_body(x_ref, hi_ref, lo_ref):
    x = x_ref[:, :]
    hi = x.astype(jnp.bfloat16)
    hi_ref[:, :] = hi
    lo_ref[:, :] = (x - hi.astype(jnp.float32)).astype(jnp.bfloat16)


def _split(x, bm=256):
    m, k = x.shape
    return pl.pallas_call(
        _split_body,
        grid=(m // bm,),
        in_specs=[pl.BlockSpec((bm, k), lambda i: (i, 0))],
        out_specs=[pl.BlockSpec((bm, k), lambda i: (i, 0)),
                   pl.BlockSpec((bm, k), lambda i: (i, 0))],
        out_shape=[jax.ShapeDtypeStruct((m, k), jnp.bfloat16),
                   jax.ShapeDtypeStruct((m, k), jnp.bfloat16)],
    )(x)


def _mm_body(x_ref, w_ref, ph_ref, pl_ref):
    p = jnp.dot(x_ref[:, :], w_ref[:, :], preferred_element_type=jnp.float32)
    ph = p.astype(jnp.bfloat16)
    ph_ref[:, :] = ph
    pl_ref[:, :] = (p - ph.astype(jnp.float32)).astype(jnp.bfloat16)


def _mm(x, w, bm=256):
    m, k = x.shape
    _, n = w.shape
    return pl.pallas_call(
        _mm_body,
        grid=(m // bm,),
        in_specs=[pl.BlockSpec((bm, k), lambda i: (i, 0)),
                  pl.BlockSpec((k, n), lambda i: (0, 0))],
        out_specs=[pl.BlockSpec((bm, n), lambda i: (i, 0)),
                   pl.BlockSpec((bm, n), lambda i: (i, 0))],
        out_shape=[jax.ShapeDtypeStruct((m, n), jnp.bfloat16),
                   jax.ShapeDtypeStruct((m, n), jnp.bfloat16)],
    )(x, w)


def _elu(x):
    return jnp.where(x > 0, x, jnp.exp(x) - 1.0)


def _split_dot(ah, al, ph, pl_):
    acc = jnp.dot(ah, ph, preferred_element_type=jnp.float32)
    acc += jnp.dot(al, ph, preferred_element_type=jnp.float32)
    acc += jnp.dot(ah, pl_, preferred_element_type=jnp.float32)
    return acc


def _amm_elu_body(ah_ref, al_ref, ph_ref, pl_ref, o_ref, acc_ref, *, nk):
    k = pl.program_id(1)

    @pl.when(k == 0)
    def _():
        acc_ref[:, :] = jnp.zeros_like(acc_ref)

    acc_ref[:, :] += _split_dot(ah_ref[:, :], al_ref[:, :],
                                ph_ref[:, :], pl_ref[:, :])

    @pl.when(k == nk - 1)
    def _():
        o_ref[:, :] = _elu(acc_ref[:, :])


def _amm_elu(ahl, phl, bm=256, bk=1024):
    ah, al = ahl
    ph, pl_ = phl
    m, kk = ah.shape
    _, n = ph.shape
    nk = kk // bk
    return pl.pallas_call(
        functools.partial(_amm_elu_body, nk=nk),
        grid=(m // bm, nk),
        in_specs=[pl.BlockSpec((bm, bk), lambda i, k: (i, k)),
                  pl.BlockSpec((bm, bk), lambda i, k: (i, k)),
                  pl.BlockSpec((bk, n), lambda i, k: (k, 0)),
                  pl.BlockSpec((bk, n), lambda i, k: (k, 0))],
        out_specs=pl.BlockSpec((bm, n), lambda i, k: (i, 0)),
        out_shape=jax.ShapeDtypeStruct((m, n), jnp.float32),
        scratch_shapes=[pltpu.VMEM((bm, n), jnp.float32)],
    )(ah, al, ph, pl_)


def _amm_elu_ft_body(ah_ref, al_ref, ph_ref, pl_ref, t_ref, o_ref, acc_ref,
                     *, nk):
    i = pl.program_id(0)
    k = pl.program_id(1)

    @pl.when((i == 0) & (k == 0))
    def _():
        o_ref[0, 0] = 0.0

    @pl.when(k == 0)
    def _():
        acc_ref[:, :] = jnp.zeros_like(acc_ref)

    acc_ref[:, :] += _split_dot(ah_ref[:, :], al_ref[:, :],
                                ph_ref[:, :], pl_ref[:, :])

    @pl.when(k == nk - 1)
    def _():
        d = t_ref[:, :] - _elu(acc_ref[:, :])
        o_ref[0, 0] += jnp.sum(d * d)


def _amm_elu_ft(ahl, phl, t, bm=256, bk=1024):
    ah, al = ahl
    ph, pl_ = phl
    m, kk = ah.shape
    _, n = ph.shape
    nk = kk // bk
    out = pl.pallas_call(
        functools.partial(_amm_elu_ft_body, nk=nk),
        grid=(m // bm, nk),
        in_specs=[pl.BlockSpec((bm, bk), lambda i, k: (i, k)),
                  pl.BlockSpec((bm, bk), lambda i, k: (i, k)),
                  pl.BlockSpec((bk, n), lambda i, k: (k, 0)),
                  pl.BlockSpec((bk, n), lambda i, k: (k, 0)),
                  pl.BlockSpec((bm, n), lambda i, k: (i, 0))],
        out_specs=pl.BlockSpec((1, 1), lambda i, k: (0, 0),
                               memory_space=pltpu.SMEM),
        out_shape=jax.ShapeDtypeStruct((1, 1), jnp.float32),
        scratch_shapes=[pltpu.VMEM((bm, n), jnp.float32)],
    )(ah, al, ph, pl_, t)
    return out[0, 0]


def _coef_mm_body(w_ref, h_ref, hi_ref, dw_ref, o_ref, se_ref, acc_ref, *, nk):
    i = pl.program_id(0)
    k = pl.program_id(1)

    @pl.when((i == 0) & (k == 0))
    def _():
        se_ref[0, 0] = 0.0

    @pl.when(k == 0)
    def _():
        acc_ref[:, :] = jnp.zeros_like(acc_ref)

    acc_ref[:, :] += jnp.dot(w_ref[:, :], h_ref[:, :],
                             preferred_element_type=jnp.float32)

    @pl.when(k == nk - 1)
    def _():
        hi = hi_ref[:, :]
        hc = acc_ref[:, :] - dw_ref[:, :] * hi
        o_ref[:, :] = hc
        d = hi - hc
        se_ref[0, 0] += jnp.sum(d * d)


def _coef_mm(w, h, dw, bm=256, bk=512):
    m, kk = w.shape
    _, n = h.shape
    nk = kk // bk
    hc, se = pl.pallas_call(
        functools.partial(_coef_mm_body, nk=nk),
        grid=(m // bm, nk),
        in_specs=[pl.BlockSpec((bm, bk), lambda i, k: (i, k)),
                  pl.BlockSpec((bk, n), lambda i, k: (k, 0)),
                  pl.BlockSpec((bm, n), lambda i, k: (i, 0)),
                  pl.BlockSpec((bm, 1), lambda i, k: (i, 0))],
        out_specs=[pl.BlockSpec((bm, n), lambda i, k: (i, 0)),
                   pl.BlockSpec((1, 1), lambda i, k: (0, 0),
                                memory_space=pltpu.SMEM)],
        out_shape=[jax.ShapeDtypeStruct((m, n), jnp.float32),
                   jax.ShapeDtypeStruct((1, 1), jnp.float32)],
        scratch_shapes=[pltpu.VMEM((bm, n), jnp.float32)],
    )(w, h, h, dw)
    return hc, se[0, 0]


def _prep_body(w_ref, w2_ref, w31_ref, w32_ref, tt_ref,
               c3_ref, zis_ref, zjs_ref, pos_ref,
               dw_ref, dw2_ref, dw31_ref, dw32_ref,
               creg_ref, cq_ref, cons_ref, *, bm):
    i = pl.program_id(0)

    @pl.when(i == 0)
    def _():
        creg_ref[0, 0] = 0.0
        cq_ref[0, 0] = 0.0
        cons_ref[0, 0] = 0.0

    n = w_ref.shape[1]
    rows = lax.broadcasted_iota(jnp.int32, (bm, n), 0) + i * bm
    cols = lax.broadcasted_iota(jnp.int32, (bm, n), 1)
    diag = rows == cols
    c = jnp.where(diag, 0.0, w_ref[:, :])
    c2 = jnp.where(diag, 0.0, w2_ref[:, :])
    c31 = jnp.where(diag, 0.0, w31_ref[:, :])
    c32 = jnp.where(diag, 0.0, w32_ref[:, :])
    dw_ref[:, :] = jnp.sum(jnp.where(diag, w_ref[:, :], 0.0),
                           axis=1, keepdims=True)
    dw2_ref[:, :] = jnp.sum(jnp.where(diag, w2_ref[:, :], 0.0),
                            axis=1, keepdims=True)
    dw31_ref[:, :] = jnp.sum(jnp.where(diag, w31_ref[:, :], 0.0),
                             axis=1, keepdims=True)
    dw32_ref[:, :] = jnp.sum(jnp.where(diag, w32_ref[:, :], 0.0),
                             axis=1, keepdims=True)
    c3 = 0.7 * c31 + 0.3 * c32
    c3_ref[:, :] = c3
    creg_ref[0, 0] += (jnp.sum(jnp.abs(c)) + jnp.sum(jnp.abs(c2))
                       + jnp.sum(jnp.abs(c31)) + jnp.sum(jnp.abs(c32)))
    cq_ref[0, 0] += jnp.sum(jnp.abs(c3 * tt_ref[:, :]))
    cons_ref[0, 0] += jnp.sum((c3 - c) ** 2) + jnp.sum((c3 - c2) ** 2)
    n31 = jnp.sqrt(jnp.sum(c31 * c31, axis=1, keepdims=True))
    n32 = jnp.sqrt(jnp.sum(c32 * c32, axis=1, keepdims=True))
    zis = c31 / jnp.maximum(n31, 1e-12)
    zjs = c32 / jnp.maximum(n32, 1e-12)
    zis_ref[:, :] = zis.astype(jnp.bfloat16)
    zjs_ref[:, :] = zjs.astype(jnp.bfloat16)
    pos_ref[:, :] = jnp.sum(zis * zjs, axis=1, keepdims=True)


def _prep(w, w2, w31, w32, theta_t, bm=128):
    n = w.shape[0]
    outs = pl.pallas_call(
        functools.partial(_prep_body, bm=bm),
        grid=(n // bm,),
        in_specs=[pl.BlockSpec((bm, n), lambda i: (i, 0))] * 5,
        out_specs=[pl.BlockSpec((bm, n), lambda i: (i, 0)),
                   pl.BlockSpec((bm, n), lambda i: (i, 0)),
                   pl.BlockSpec((bm, n), lambda i: (i, 0)),
                   pl.BlockSpec((bm, 1), lambda i: (i, 0)),
                   pl.BlockSpec((bm, 1), lambda i: (i, 0)),
                   pl.BlockSpec((bm, 1), lambda i: (i, 0)),
                   pl.BlockSpec((bm, 1), lambda i: (i, 0)),
                   pl.BlockSpec((bm, 1), lambda i: (i, 0)),
                   pl.BlockSpec((1, 1), lambda i: (0, 0),
                                memory_space=pltpu.SMEM),
                   pl.BlockSpec((1, 1), lambda i: (0, 0),
                                memory_space=pltpu.SMEM),
                   pl.BlockSpec((1, 1), lambda i: (0, 0),
                                memory_space=pltpu.SMEM)],
        out_shape=[jax.ShapeDtypeStruct((n, n), jnp.float32),
                   jax.ShapeDtypeStruct((n, n), jnp.bfloat16),
                   jax.ShapeDtypeStruct((n, n), jnp.bfloat16),
                   jax.ShapeDtypeStruct((n, 1), jnp.float32),
                   jax.ShapeDtypeStruct((n, 1), jnp.float32),
                   jax.ShapeDtypeStruct((n, 1), jnp.float32),
                   jax.ShapeDtypeStruct((n, 1), jnp.float32),
                   jax.ShapeDtypeStruct((n, 1), jnp.float32),
                   jax.ShapeDtypeStruct((1, 1), jnp.float32),
                   jax.ShapeDtypeStruct((1, 1), jnp.float32),
                   jax.ShapeDtypeStruct((1, 1), jnp.float32)],
    )(w, w2, w31, w32, theta_t)
    (c3, zis, zjs, pos, dw, dw2, dw31, dw32, creg, cq, cons) = outs
    return (c3, zis, zjs, pos, dw, dw2, dw31, dw32,
            creg[0, 0], cq[0, 0], cons[0, 0])


_DN = (((1,), (1,)), ((), ()))


def _gram_body(zis_i, zjs_i, zis_j, zjs_j, y_i, yt_j, pos_ref, post_ref,
               cl_ref, a1, a2, a3, neg1, neg2, *, nmi, nmj, nk, bm, bn):
    i = pl.program_id(0)
    j = pl.program_id(1)
    k = pl.program_id(2)

    @pl.when((i == 0) & (j == 0) & (k == 0))
    def _():
        neg1[:, :] = jnp.zeros_like(neg1)
        neg2[:, :] = jnp.zeros_like(neg2)

    @pl.when(k == 0)
    def _():
        a1[:, :] = jnp.zeros_like(a1)
        a2[:, :] = jnp.zeros_like(a2)
        a3[:, :] = jnp.zeros_like(a3)

    a1[:, :] += lax.dot_general(zis_i[:, :], zjs_j[:, :], _DN,
                                preferred_element_type=jnp.float32)
    a2[:, :] += lax.dot_general(zis_i[:, :], zis_j[:, :], _DN,
                                preferred_element_type=jnp.float32)
    a3[:, :] += lax.dot_general(zjs_i[:, :], zjs_j[:, :], _DN,
                                preferred_element_type=jnp.float32)

    @pl.when(k == nk - 1)
    def _():
        nm = (y_i[:, :] != yt_j[:, :]).astype(jnp.float32)
        e1 = jnp.exp(a1[:, :]) * nm
        e2 = jnp.exp(a2[:, :]) * nm
        e3 = jnp.exp(a3[:, :]) * nm
        neg1[pl.ds(i * bm, bm), :] += jnp.sum(e1 + e2, axis=1, keepdims=True)
        neg2[:, pl.ds(j * bn, bn)] += jnp.sum(e1 + e3, axis=0)[None, :]

        @pl.when((i == nmi - 1) & (j == nmj - 1))
        def _():
            p = pos_ref[:, :]
            pt = post_ref[:, :]
            cl_ref[0, 0] = (jnp.sum(jnp.log(jnp.exp(p) + neg1[:, :]) - p)
                            + jnp.sum(jnp.log(jnp.exp(pt) + neg2[:, :]) - pt))


def _gram(zis, zjs, y, yt, pos, post, bm=1024, bn=1024, bk=1024):
    n = zis.shape[0]
    nmi, nmj, nk = n // bm, n // bn, n // bk
    cl = pl.pallas_call(
        functools.partial(_gram_body, nmi=nmi, nmj=nmj, nk=nk, bm=bm, bn=bn),
        grid=(nmi, nmj, nk),
        in_specs=[pl.BlockSpec((bm, bk), lambda i, j, k: (i, k)),
                  pl.BlockSpec((bm, bk), lambda i, j, k: (i, k)),
                  pl.BlockSpec((bn, bk), lambda i, j, k: (j, k)),
                  pl.BlockSpec((bn, bk), lambda i, j, k: (j, k)),
                  pl.BlockSpec((bm, 1), lambda i, j, k: (i, 0)),
                  pl.BlockSpec((1, bn), lambda i, j, k: (0, j)),
                  pl.BlockSpec((n, 1), lambda i, j, k: (0, 0)),
                  pl.BlockSpec((1, n), lambda i, j, k: (0, 0))],
        out_specs=pl.BlockSpec((1, 1), lambda i, j, k: (0, 0),
                               memory_space=pltpu.SMEM),
        out_shape=jax.ShapeDtypeStruct((1, 1), jnp.float32),
        scratch_shapes=[pltpu.VMEM((bm, bn), jnp.float32),
                        pltpu.VMEM((bm, bn), jnp.float32),
                        pltpu.VMEM((bm, bn), jnp.float32),
                        pltpu.VMEM((n, 1), jnp.float32),
                        pltpu.VMEM((1, n), jnp.float32)],
    )(zis, zis, zjs, zjs, y, yt, pos, post)
    return cl[0, 0]


def _edge_dots(h, s, r):
    n, d = h.shape
    e = s.shape[0]
    info = plsc.get_sparse_core_info()
    nw = info.num_cores * info.num_subcores
    per_w = e // nw
    ch = 128
    nch = per_w // ch
    mesh = plsc.VectorSubcoreMesh(core_axis_name="c", subcore_axis_name="s")

    def body(h_hbm, s_hbm, r_hbm, out_hbm, sidx, ridx, arow, brow, ovec,
             sem1, sem2):
        wid = lax.axis_index("s") * info.num_cores + lax.axis_index("c")

        def chunk(c, carry):
            base = wid * per_w + c * ch
            pltpu.sync_copy(s_hbm.at[pl.ds(base, ch)], sidx)
            pltpu.sync_copy(r_hbm.at[pl.ds(base, ch)], ridx)
            cp1 = pltpu.async_copy(h_hbm.at[sidx], arow, sem1)
            cp2 = pltpu.async_copy(h_hbm.at[ridx], brow, sem2)
            cp1.wait()
            cp2.wait()

            def edge(eo, cc):
                for sub in range(8):
                    ei = eo * 8 + sub
                    acc = arow[ei, pl.ds(0, 16)] * brow[ei, pl.ds(0, 16)]
                    for g in range(1, d // 16):
                        acc = acc + (arow[ei, pl.ds(g * 16, 16)]
                                     * brow[ei, pl.ds(g * 16, 16)])
                    ovec[eo, pl.ds(sub * 16, 16)] = acc
                return cc

            lax.fori_loop(0, ch // 8, edge, 0)
            obase = pl.multiple_of(base // 8, 8)
            pltpu.sync_copy(ovec, out_hbm.at[pl.ds(obase, ch // 8)])
            return carry

        lax.fori_loop(0, nch, chunk, 0)

    return pl.kernel(
        body,
        out_type=jax.ShapeDtypeStruct((e // 8, 128), jnp.float32),
        mesh=mesh,
        scratch_types=[pltpu.VMEM((ch,), jnp.int32),
                       pltpu.VMEM((ch,), jnp.int32),
                       pltpu.VMEM((ch, d), jnp.float32),
                       pltpu.VMEM((ch, d), jnp.float32),
                       pltpu.VMEM((ch // 8, 128), jnp.float32),
                       pltpu.SemaphoreType.DMA,
                       pltpu.SemaphoreType.DMA],
    )(h, s, r)


def _st_body(d1, d2, d3, d4, o_ref):
    i = pl.program_id(0)

    @pl.when(i == 0)
    def _():
        o_ref[0, 0] = 0.0

    seg = (lax.broadcasted_iota(jnp.int32, (128, 128), 0) // 16
           == lax.broadcasted_iota(jnp.int32, (128, 128), 1)
           ).astype(jnp.float32)
    colmask = lax.broadcasted_iota(jnp.int32, d1.shape, 1) < 8
    tot = 0.0
    for dref in (d1, d2, d3, d4):
        dot = jnp.dot(dref[:, :], seg, preferred_element_type=jnp.float32)
        sp = jnp.maximum(-dot, 0.0) + jnp.log(1.0 + jnp.exp(-jnp.abs(dot)))
        tot += jnp.sum(jnp.where(colmask, sp, 0.0))
    o_ref[0, 0] += tot


def _st_reduce(d1, d2, d3, d4, be=4096):
    e8 = d1.shape[0]
    out = pl.pallas_call(
        _st_body,
        grid=(e8 // be,),
        in_specs=[pl.BlockSpec((be, 128), lambda i: (i, 0))] * 4,
        out_specs=pl.BlockSpec((1, 1), lambda i: (0, 0),
                               memory_space=pltpu.SMEM),
        out_shape=jax.ShapeDtypeStruct((1, 1), jnp.float32),
    )(d1, d2, d3, d4)
    return out[0, 0]


def kernel(X, A, S, R, X2, A2, S2, R2, y_pred, Theta,
           weight, weight2, weight31, weight32,
           W11, W12, Wd11, Wd12, W21, W22, Wd21, Wd22, W31, Wd31):
    n, f1 = X.shape
    f2 = X2.shape[1]
    h2 = W12.shape[1]
    h3 = W31.shape[1]

    pad = 128 - h3
    W31p = jnp.pad(W31, ((0, 0), (0, pad)))
    Wd31p = jnp.pad(Wd31, ((0, pad), (0, 0)))

    Ahl = _split(A)
    A2hl = _split(A2)

    H = _amm_elu(Ahl, _mm(_amm_elu(Ahl, _mm(X, W11)), W12))
    Hb = _amm_elu(A2hl, _mm(_amm_elu(A2hl, _mm(X2, W21)), W22))

    d1 = _edge_dots(H, S, R)
    d2 = _edge_dots(Hb, S2, R2)

    (c3, zis, zjs, pos, dw, dw2, dw31, dw32, creg, cq, cons) = _prep(
        weight, weight2, weight31, weight32, Theta.T)

    HC, se1 = _coef_mm(weight, H, dw)
    ft1 = _amm_elu_ft(Ahl, _mm(_amm_elu(Ahl, _mm(HC, Wd11)), Wd12), X)
    HC2, se2 = _coef_mm(weight2, Hb, dw2)
    ft2 = _amm_elu_ft(A2hl, _mm(_amm_elu(A2hl, _mm(HC2, Wd21)), Wd22), X2)

    H31 = _amm_elu(Ahl, _mm(H, W31p))
    H32 = _amm_elu(A2hl, _mm(Hb, W31p))
    d3 = _edge_dots(H31, S, R)
    d4 = _edge_dots(H32, S2, R2)
    HC31, se3 = _coef_mm(weight31, H31, dw31)
    HC32, se4 = _coef_mm(weight32, H32, dw32)
    ft3 = _amm_elu_ft(Ahl, _mm(HC31, Wd31p), H)
    ft4 = _amm_elu_ft(A2hl, _mm(HC32, Wd31p), Hb)

    yt = y_pred.reshape(1, n)
    cl_sum = _gram(zis, zjs, y_pred, yt, pos, pos.reshape(1, n))

    st_loss = _st_reduce(d1, d2, d3, d4)

    ft_loss = (ft1 / (n * f1) + ft2 / (n * f2)
               + ft3 / (n * h2) + ft4 / (n * h2))
    se_loss = 0.5 * (se1 / (n * h2) + se2 / (n * h2)
                     + se3 / (n * h3) + se4 / (n * h3))
    cl_loss = cl_sum / (2.0 * n)

    loss = (ft_loss + 0.1 * st_loss + se_loss + 0.1 * creg
            + 0.1 * cl_loss + 0.1 * cq + 0.1 * cons)
    return (loss, ft_loss, st_loss, se_loss, creg, cons, cl_loss, cq, c3)


# speedup vs baseline: 2.6687x; 1.1894x over previous
import functools

import jax
import jax.numpy as jnp
from jax import lax
from jax.experimental import pallas as pl
from jax.experimental.pallas import tpu as pltpu
from jax.experimental.pallas import tpu_sc as plsc


def _split_body(x_ref, hi_ref, lo_ref):
    x = x_ref[:, :]
    hi = x.astype(jnp.bfloat16)
    hi_ref[:, :] = hi
    lo_ref[:, :] = (x - hi.astype(jnp.float32)).astype(jnp.bfloat16)


def _split(x, bm=256):
    m, k = x.shape
    return pl.pallas_call(
        _split_body,
        grid=(m // bm,),
        in_specs=[pl.BlockSpec((bm, k), lambda i: (i, 0))],
        out_specs=[pl.BlockSpec((bm, k), lambda i: (i, 0)),
                   pl.BlockSpec((bm, k), lambda i: (i, 0))],
        out_shape=[jax.ShapeDtypeStruct((m, k), jnp.bfloat16),
                   jax.ShapeDtypeStruct((m, k), jnp.bfloat16)],
    )(x)


def _mm_body(x_ref, w_ref, ph_ref, pl_ref):
    p = jnp.dot(x_ref[:, :], w_ref[:, :], preferred_element_type=jnp.float32)
    ph = p.astype(jnp.bfloat16)
    ph_ref[:, :] = ph
    pl_ref[:, :] = (p - ph.astype(jnp.float32)).astype(jnp.bfloat16)


def _mm(x, w, bm=256):
    m, k = x.shape
    _, n = w.shape
    return pl.pallas_call(
        _mm_body,
        grid=(m // bm,),
        in_specs=[pl.BlockSpec((bm, k), lambda i: (i, 0)),
                  pl.BlockSpec((k, n), lambda i: (0, 0))],
        out_specs=[pl.BlockSpec((bm, n), lambda i: (i, 0)),
                   pl.BlockSpec((bm, n), lambda i: (i, 0))],
        out_shape=[jax.ShapeDtypeStruct((m, n), jnp.bfloat16),
                   jax.ShapeDtypeStruct((m, n), jnp.bfloat16)],
    )(x, w)


def _elu(x):
    return jnp.where(x > 0, x, jnp.exp(x) - 1.0)


def _split_dot(ah, al, ph, pl_):
    acc = jnp.dot(ah, ph, preferred_element_type=jnp.float32)
    acc += jnp.dot(al, ph, preferred_element_type=jnp.float32)
    acc += jnp.dot(ah, pl_, preferred_element_type=jnp.float32)
    return acc


def _amm_elu_body(ah_ref, al_ref, ph_ref, pl_ref, o_ref, acc_ref, *, nk):
    k = pl.program_id(1)

    @pl.when(k == 0)
    def _():
        acc_ref[:, :] = jnp.zeros_like(acc_ref)

    acc_ref[:, :] += _split_dot(ah_ref[:, :], al_ref[:, :],
                                ph_ref[:, :], pl_ref[:, :])

    @pl.when(k == nk - 1)
    def _():
        o_ref[:, :] = _elu(acc_ref[:, :])


def _amm_elu(ahl, phl, bm=512, bk=1024):
    ah, al = ahl
    ph, pl_ = phl
    m, kk = ah.shape
    _, n = ph.shape
    nk = kk // bk
    return pl.pallas_call(
        functools.partial(_amm_elu_body, nk=nk),
        grid=(m // bm, nk),
        in_specs=[pl.BlockSpec((bm, bk), lambda i, k: (i, k)),
                  pl.BlockSpec((bm, bk), lambda i, k: (i, k)),
                  pl.BlockSpec((bk, n), lambda i, k: (k, 0)),
                  pl.BlockSpec((bk, n), lambda i, k: (k, 0))],
        out_specs=pl.BlockSpec((bm, n), lambda i, k: (i, 0)),
        out_shape=jax.ShapeDtypeStruct((m, n), jnp.float32),
        scratch_shapes=[pltpu.VMEM((bm, n), jnp.float32)],
    )(ah, al, ph, pl_)


def _amm_elu_ft_body(ah_ref, al_ref, ph_ref, pl_ref, t_ref, o_ref, acc_ref,
                     *, nk):
    i = pl.program_id(0)
    k = pl.program_id(1)

    @pl.when((i == 0) & (k == 0))
    def _():
        o_ref[0, 0] = 0.0

    @pl.when(k == 0)
    def _():
        acc_ref[:, :] = jnp.zeros_like(acc_ref)

    acc_ref[:, :] += _split_dot(ah_ref[:, :], al_ref[:, :],
                                ph_ref[:, :], pl_ref[:, :])

    @pl.when(k == nk - 1)
    def _():
        d = t_ref[:, :] - _elu(acc_ref[:, :])
        o_ref[0, 0] += jnp.sum(d * d)


def _amm_elu_ft(ahl, phl, t, bm=512, bk=1024):
    ah, al = ahl
    ph, pl_ = phl
    m, kk = ah.shape
    _, n = ph.shape
    nk = kk // bk
    out = pl.pallas_call(
        functools.partial(_amm_elu_ft_body, nk=nk),
        grid=(m // bm, nk),
        in_specs=[pl.BlockSpec((bm, bk), lambda i, k: (i, k)),
                  pl.BlockSpec((bm, bk), lambda i, k: (i, k)),
                  pl.BlockSpec((bk, n), lambda i, k: (k, 0)),
                  pl.BlockSpec((bk, n), lambda i, k: (k, 0)),
                  pl.BlockSpec((bm, n), lambda i, k: (i, 0))],
        out_specs=pl.BlockSpec((1, 1), lambda i, k: (0, 0),
                               memory_space=pltpu.SMEM),
        out_shape=jax.ShapeDtypeStruct((1, 1), jnp.float32),
        scratch_shapes=[pltpu.VMEM((bm, n), jnp.float32)],
    )(ah, al, ph, pl_, t)
    return out[0, 0]


def _coef_mm_body(w_ref, h_ref, hi_ref, dw_ref, o_ref, se_ref, acc_ref, *, nk):
    i = pl.program_id(0)
    k = pl.program_id(1)

    @pl.when((i == 0) & (k == 0))
    def _():
        se_ref[0, 0] = 0.0

    @pl.when(k == 0)
    def _():
        acc_ref[:, :] = jnp.zeros_like(acc_ref)

    w = w_ref[:, :]
    wh = w.astype(jnp.bfloat16)
    wl = (w - wh.astype(jnp.float32)).astype(jnp.bfloat16)
    h = h_ref[:, :]
    hh = h.astype(jnp.bfloat16)
    hl = (h - hh.astype(jnp.float32)).astype(jnp.bfloat16)
    acc_ref[:, :] += _split_dot(wh, wl, hh, hl)

    @pl.when(k == nk - 1)
    def _():
        hi = hi_ref[:, :]
        hc = acc_ref[:, :] - dw_ref[:, :] * hi
        o_ref[:, :] = hc
        d = hi - hc
        se_ref[0, 0] += jnp.sum(d * d)


def _coef_mm(w, h, dw, bm=256, bk=1024):
    m, kk = w.shape
    _, n = h.shape
    nk = kk // bk
    hc, se = pl.pallas_call(
        functools.partial(_coef_mm_body, nk=nk),
        grid=(m // bm, nk),
        in_specs=[pl.BlockSpec((bm, bk), lambda i, k: (i, k)),
                  pl.BlockSpec((bk, n), lambda i, k: (k, 0)),
                  pl.BlockSpec((bm, n), lambda i, k: (i, 0)),
                  pl.BlockSpec((bm, 1), lambda i, k: (i, 0))],
        out_specs=[pl.BlockSpec((bm, n), lambda i, k: (i, 0)),
                   pl.BlockSpec((1, 1), lambda i, k: (0, 0),
                                memory_space=pltpu.SMEM)],
        out_shape=[jax.ShapeDtypeStruct((m, n), jnp.float32),
                   jax.ShapeDtypeStruct((1, 1), jnp.float32)],
        scratch_shapes=[pltpu.VMEM((bm, n), jnp.float32)],
    )(w, h, h, dw)
    return hc, se[0, 0]


def _prep_body(w_ref, w2_ref, w31_ref, w32_ref, tt_ref,
               c3_ref, zis_ref, zjs_ref, pos_ref,
               dw_ref, dw2_ref, dw31_ref, dw32_ref,
               creg_ref, cq_ref, cons_ref, *, bm):
    i = pl.program_id(0)

    @pl.when(i == 0)
    def _():
        creg_ref[0, 0] = 0.0
        cq_ref[0, 0] = 0.0
        cons_ref[0, 0] = 0.0

    n = w_ref.shape[1]
    rows = lax.broadcasted_iota(jnp.int32, (bm, n), 0) + i * bm
    cols = lax.broadcasted_iota(jnp.int32, (bm, n), 1)
    diag = rows == cols
    c = jnp.where(diag, 0.0, w_ref[:, :])
    c2 = jnp.where(diag, 0.0, w2_ref[:, :])
    c31 = jnp.where(diag, 0.0, w31_ref[:, :])
    c32 = jnp.where(diag, 0.0, w32_ref[:, :])
    dw_ref[:, :] = jnp.sum(jnp.where(diag, w_ref[:, :], 0.0),
                           axis=1, keepdims=True)
    dw2_ref[:, :] = jnp.sum(jnp.where(diag, w2_ref[:, :], 0.0),
                            axis=1, keepdims=True)
    dw31_ref[:, :] = jnp.sum(jnp.where(diag, w31_ref[:, :], 0.0),
                             axis=1, keepdims=True)
    dw32_ref[:, :] = jnp.sum(jnp.where(diag, w32_ref[:, :], 0.0),
                             axis=1, keepdims=True)
    c3 = 0.7 * c31 + 0.3 * c32
    c3_ref[:, :] = c3
    creg_ref[0, 0] += (jnp.sum(jnp.abs(c)) + jnp.sum(jnp.abs(c2))
                       + jnp.sum(jnp.abs(c31)) + jnp.sum(jnp.abs(c32)))
    cq_ref[0, 0] += jnp.sum(jnp.abs(c3 * tt_ref[:, :]))
    cons_ref[0, 0] += jnp.sum((c3 - c) ** 2) + jnp.sum((c3 - c2) ** 2)
    n31 = jnp.sqrt(jnp.sum(c31 * c31, axis=1, keepdims=True))
    n32 = jnp.sqrt(jnp.sum(c32 * c32, axis=1, keepdims=True))
    zis = c31 / jnp.maximum(n31, 1e-12)
    zjs = c32 / jnp.maximum(n32, 1e-12)
    zis_ref[:, :] = zis.astype(jnp.bfloat16)
    zjs_ref[:, :] = zjs.astype(jnp.bfloat16)
    pos_ref[:, :] = jnp.sum(zis * zjs, axis=1, keepdims=True)


def _prep(w, w2, w31, w32, theta_t, bm=128):
    n = w.shape[0]
    outs = pl.pallas_call(
        functools.partial(_prep_body, bm=bm),
        grid=(n // bm,),
        in_specs=[pl.BlockSpec((bm, n), lambda i: (i, 0))] * 5,
        out_specs=[pl.BlockSpec((bm, n), lambda i: (i, 0)),
                   pl.BlockSpec((bm, n), lambda i: (i, 0)),
                   pl.BlockSpec((bm, n), lambda i: (i, 0)),
                   pl.BlockSpec((bm, 1), lambda i: (i, 0)),
                   pl.BlockSpec((bm, 1), lambda i: (i, 0)),
                   pl.BlockSpec((bm, 1), lambda i: (i, 0)),
                   pl.BlockSpec((bm, 1), lambda i: (i, 0)),
                   pl.BlockSpec((bm, 1), lambda i: (i, 0)),
                   pl.BlockSpec((1, 1), lambda i: (0, 0),
                                memory_space=pltpu.SMEM),
                   pl.BlockSpec((1, 1), lambda i: (0, 0),
                                memory_space=pltpu.SMEM),
                   pl.BlockSpec((1, 1), lambda i: (0, 0),
                                memory_space=pltpu.SMEM)],
        out_shape=[jax.ShapeDtypeStruct((n, n), jnp.float32),
                   jax.ShapeDtypeStruct((n, n), jnp.bfloat16),
                   jax.ShapeDtypeStruct((n, n), jnp.bfloat16),
                   jax.ShapeDtypeStruct((n, 1), jnp.float32),
                   jax.ShapeDtypeStruct((n, 1), jnp.float32),
                   jax.ShapeDtypeStruct((n, 1), jnp.float32),
                   jax.ShapeDtypeStruct((n, 1), jnp.float32),
                   jax.ShapeDtypeStruct((n, 1), jnp.float32),
                   jax.ShapeDtypeStruct((1, 1), jnp.float32),
                   jax.ShapeDtypeStruct((1, 1), jnp.float32),
                   jax.ShapeDtypeStruct((1, 1), jnp.float32)],
    )(w, w2, w31, w32, theta_t)
    (c3, zis, zjs, pos, dw, dw2, dw31, dw32, creg, cq, cons) = outs
    return (c3, zis, zjs, pos, dw, dw2, dw31, dw32,
            creg[0, 0], cq[0, 0], cons[0, 0])


_DN = (((1,), (1,)), ((), ()))


def _gram_body(zis_i, zjs_i, zis_j, zjs_j, y_i, yt_j, pos_ref, post_ref,
               cl_ref, a1, a2, a3, neg1, neg2, *, nmi, nmj, nk, bm, bn):
    i = pl.program_id(0)
    j = pl.program_id(1)
    k = pl.program_id(2)

    @pl.when((i == 0) & (j == 0) & (k == 0))
    def _():
        neg1[:, :] = jnp.zeros_like(neg1)
        neg2[:, :] = jnp.zeros_like(neg2)

    @pl.when(k == 0)
    def _():
        a1[:, :] = jnp.zeros_like(a1)
        a2[:, :] = jnp.zeros_like(a2)
        a3[:, :] = jnp.zeros_like(a3)

    a1[:, :] += lax.dot_general(zis_i[:, :], zjs_j[:, :], _DN,
                                preferred_element_type=jnp.float32)
    a2[:, :] += lax.dot_general(zis_i[:, :], zis_j[:, :], _DN,
                                preferred_element_type=jnp.float32)
    a3[:, :] += lax.dot_general(zjs_i[:, :], zjs_j[:, :], _DN,
                                preferred_element_type=jnp.float32)

    @pl.when(k == nk - 1)
    def _():
        nm = (y_i[:, :] != yt_j[:, :]).astype(jnp.float32)
        e1 = jnp.exp(a1[:, :]) * nm
        e2 = jnp.exp(a2[:, :]) * nm
        e3 = jnp.exp(a3[:, :]) * nm
        neg1[pl.ds(i * bm, bm), :] += jnp.sum(e1 + e2, axis=1, keepdims=True)
        neg2[:, pl.ds(j * bn, bn)] += jnp.sum(e1 + e3, axis=0)[None, :]

        @pl.when((i == nmi - 1) & (j == nmj - 1))
        def _():
            p = pos_ref[:, :]
            pt = post_ref[:, :]
            cl_ref[0, 0] = (jnp.sum(jnp.log(jnp.exp(p) + neg1[:, :]) - p)
                            + jnp.sum(jnp.log(jnp.exp(pt) + neg2[:, :]) - pt))


def _gram(zis, zjs, y, yt, pos, post, bm=1024, bn=1024, bk=1024):
    n = zis.shape[0]
    nmi, nmj, nk = n // bm, n // bn, n // bk
    cl = pl.pallas_call(
        functools.partial(_gram_body, nmi=nmi, nmj=nmj, nk=nk, bm=bm, bn=bn),
        grid=(nmi, nmj, nk),
        in_specs=[pl.BlockSpec((bm, bk), lambda i, j, k: (i, k)),
                  pl.BlockSpec((bm, bk), lambda i, j, k: (i, k)),
                  pl.BlockSpec((bn, bk), lambda i, j, k: (j, k)),
                  pl.BlockSpec((bn, bk), lambda i, j, k: (j, k)),
                  pl.BlockSpec((bm, 1), lambda i, j, k: (i, 0)),
                  pl.BlockSpec((1, bn), lambda i, j, k: (0, j)),
                  pl.BlockSpec((n, 1), lambda i, j, k: (0, 0)),
                  pl.BlockSpec((1, n), lambda i, j, k: (0, 0))],
        out_specs=pl.BlockSpec((1, 1), lambda i, j, k: (0, 0),
                               memory_space=pltpu.SMEM),
        out_shape=jax.ShapeDtypeStruct((1, 1), jnp.float32),
        scratch_shapes=[pltpu.VMEM((bm, bn), jnp.float32),
                        pltpu.VMEM((bm, bn), jnp.float32),
                        pltpu.VMEM((bm, bn), jnp.float32),
                        pltpu.VMEM((n, 1), jnp.float32),
                        pltpu.VMEM((1, n), jnp.float32)],
    )(zis, zis, zjs, zjs, y, yt, pos, post)
    return cl[0, 0]


def _edge_dots(h, s, r):
    n, d = h.shape
    e = s.shape[0]
    info = plsc.get_sparse_core_info()
    nw = info.num_cores * info.num_subcores
    per_w = e // nw
    ch = 128
    nch = per_w // ch
    mesh = plsc.VectorSubcoreMesh(core_axis_name="c", subcore_axis_name="s")

    def body(h_hbm, s_hbm, r_hbm, out_hbm, sidx, ridx, arow, brow, ovec,
             sem1, sem2):
        wid = lax.axis_index("s") * info.num_cores + lax.axis_index("c")

        def chunk(c, carry):
            base = wid * per_w + c * ch
            pltpu.sync_copy(s_hbm.at[pl.ds(base, ch)], sidx)
            pltpu.sync_copy(r_hbm.at[pl.ds(base, ch)], ridx)
            cp1 = pltpu.async_copy(h_hbm.at[sidx], arow, sem1)
            cp2 = pltpu.async_copy(h_hbm.at[ridx], brow, sem2)
            cp1.wait()
            cp2.wait()

            def edge(eo, cc):
                for sub in range(8):
                    ei = eo * 8 + sub
                    acc = arow[ei, pl.ds(0, 16)] * brow[ei, pl.ds(0, 16)]
                    for g in range(1, d // 16):
                        acc = acc + (arow[ei, pl.ds(g * 16, 16)]
                                     * brow[ei, pl.ds(g * 16, 16)])
                    ovec[eo, pl.ds(sub * 16, 16)] = acc
                return cc

            lax.fori_loop(0, ch // 8, edge, 0)
            obase = pl.multiple_of(base // 8, 8)
            pltpu.sync_copy(ovec, out_hbm.at[pl.ds(obase, ch // 8)])
            return carry

        lax.fori_loop(0, nch, chunk, 0)

    return pl.kernel(
        body,
        out_type=jax.ShapeDtypeStruct((e // 8, 128), jnp.float32),
        mesh=mesh,
        scratch_types=[pltpu.VMEM((ch,), jnp.int32),
                       pltpu.VMEM((ch,), jnp.int32),
                       pltpu.VMEM((ch, d), jnp.float32),
                       pltpu.VMEM((ch, d), jnp.float32),
                       pltpu.VMEM((ch // 8, 128), jnp.float32),
                       pltpu.SemaphoreType.DMA,
                       pltpu.SemaphoreType.DMA],
    )(h, s, r)


def _st_body(d1, d2, d3, d4, o_ref):
    i = pl.program_id(0)

    @pl.when(i == 0)
    def _():
        o_ref[0, 0] = 0.0

    seg = (lax.broadcasted_iota(jnp.int32, (128, 128), 0) // 16
           == lax.broadcasted_iota(jnp.int32, (128, 128), 1)
           ).astype(jnp.float32)
    colmask = lax.broadcasted_iota(jnp.int32, d1.shape, 1) < 8
    tot = 0.0
    for dref in (d1, d2, d3, d4):
        dot = jnp.dot(dref[:, :], seg, preferred_element_type=jnp.float32)
        sp = jnp.maximum(-dot, 0.0) + jnp.log(1.0 + jnp.exp(-jnp.abs(dot)))
        tot += jnp.sum(jnp.where(colmask, sp, 0.0))
    o_ref[0, 0] += tot


def _st_reduce(d1, d2, d3, d4, be=4096):
    e8 = d1.shape[0]
    out = pl.pallas_call(
        _st_body,
        grid=(e8 // be,),
        in_specs=[pl.BlockSpec((be, 128), lambda i: (i, 0))] * 4,
        out_specs=pl.BlockSpec((1, 1), lambda i: (0, 0),
                               memory_space=pltpu.SMEM),
        out_shape=jax.ShapeDtypeStruct((1, 1), jnp.float32),
    )(d1, d2, d3, d4)
    return out[0, 0]


def kernel(X, A, S, R, X2, A2, S2, R2, y_pred, Theta,
           weight, weight2, weight31, weight32,
           W11, W12, Wd11, Wd12, W21, W22, Wd21, Wd22, W31, Wd31):
    n, f1 = X.shape
    f2 = X2.shape[1]
    h2 = W12.shape[1]
    h3 = W31.shape[1]

    pad = 128 - h3
    W31p = jnp.pad(W31, ((0, 0), (0, pad)))
    Wd31p = jnp.pad(Wd31, ((0, pad), (0, 0)))

    Ahl = _split(A)
    A2hl = _split(A2)

    H = _amm_elu(Ahl, _mm(_amm_elu(Ahl, _mm(X, W11)), W12))
    Hb = _amm_elu(A2hl, _mm(_amm_elu(A2hl, _mm(X2, W21)), W22))

    d1 = _edge_dots(H, S, R)
    d2 = _edge_dots(Hb, S2, R2)

    (c3, zis, zjs, pos, dw, dw2, dw31, dw32, creg, cq, cons) = _prep(
        weight, weight2, weight31, weight32, Theta.T)

    HC, se1 = _coef_mm(weight, H, dw)
    ft1 = _amm_elu_ft(Ahl, _mm(_amm_elu(Ahl, _mm(HC, Wd11)), Wd12), X)
    HC2, se2 = _coef_mm(weight2, Hb, dw2)
    ft2 = _amm_elu_ft(A2hl, _mm(_amm_elu(A2hl, _mm(HC2, Wd21)), Wd22), X2)

    H31 = _amm_elu(Ahl, _mm(H, W31p))
    H32 = _amm_elu(A2hl, _mm(Hb, W31p))
    d3 = _edge_dots(H31, S, R)
    d4 = _edge_dots(H32, S2, R2)
    HC31, se3 = _coef_mm(weight31, H31, dw31)
    HC32, se4 = _coef_mm(weight32, H32, dw32)
    ft3 = _amm_elu_ft(Ahl, _mm(HC31, Wd31p), H)
    ft4 = _amm_elu_ft(A2hl, _mm(HC32, Wd31p), Hb)

    yt = y_pred.reshape(1, n)
    cl_sum = _gram(zis, zjs, y_pred, yt, pos, pos.reshape(1, n))

    st_loss = _st_reduce(d1, d2, d3, d4)

    ft_loss = (ft1 / (n * f1) + ft2 / (n * f2)
               + ft3 / (n * h2) + ft4 / (n * h2))
    se_loss = 0.5 * (se1 / (n * h2) + se2 / (n * h2)
                     + se3 / (n * h3) + se4 / (n * h3))
    cl_loss = cl_sum / (2.0 * n)

    loss = (ft_loss + 0.1 * st_loss + se_loss + 0.1 * creg
            + 0.1 * cl_loss + 0.1 * cq + 0.1 * cons)
    return (loss, ft_loss, st_loss, se_loss, creg, cons, cl_loss, cq, c3)


# speedup vs baseline: 2.8677x; 1.0746x over previous
import functools

import jax
import jax.numpy as jnp
from jax import lax
from jax.experimental import pallas as pl
from jax.experimental.pallas import tpu as pltpu
from jax.experimental.pallas import tpu_sc as plsc


def _split_body(x_ref, hi_ref, lo_ref):
    x = x_ref[:, :]
    hi = x.astype(jnp.bfloat16)
    hi_ref[:, :] = hi
    lo_ref[:, :] = (x - hi.astype(jnp.float32)).astype(jnp.bfloat16)


def _split(x, bm=256):
    m, k = x.shape
    return pl.pallas_call(
        _split_body,
        grid=(m // bm,),
        in_specs=[pl.BlockSpec((bm, k), lambda i: (i, 0))],
        out_specs=[pl.BlockSpec((bm, k), lambda i: (i, 0)),
                   pl.BlockSpec((bm, k), lambda i: (i, 0))],
        out_shape=[jax.ShapeDtypeStruct((m, k), jnp.bfloat16),
                   jax.ShapeDtypeStruct((m, k), jnp.bfloat16)],
    )(x)


def _mm_body(x_ref, w_ref, ph_ref, pl_ref):
    p = jnp.dot(x_ref[:, :], w_ref[:, :], preferred_element_type=jnp.float32)
    ph = p.astype(jnp.bfloat16)
    ph_ref[:, :] = ph
    pl_ref[:, :] = (p - ph.astype(jnp.float32)).astype(jnp.bfloat16)


def _mm(x, w, bm=256):
    m, k = x.shape
    _, n = w.shape
    return pl.pallas_call(
        _mm_body,
        grid=(m // bm,),
        in_specs=[pl.BlockSpec((bm, k), lambda i: (i, 0)),
                  pl.BlockSpec((k, n), lambda i: (0, 0))],
        out_specs=[pl.BlockSpec((bm, n), lambda i: (i, 0)),
                   pl.BlockSpec((bm, n), lambda i: (i, 0))],
        out_shape=[jax.ShapeDtypeStruct((m, n), jnp.bfloat16),
                   jax.ShapeDtypeStruct((m, n), jnp.bfloat16)],
    )(x, w)


def _mm2_body(x1_ref, w1_ref, x2_ref, w2_ref, ph_ref, pl_ref, *, n1):
    p1 = jnp.dot(x1_ref[:, :], w1_ref[:, :], preferred_element_type=jnp.float32)
    p2 = jnp.dot(x2_ref[:, :], w2_ref[:, :], preferred_element_type=jnp.float32)
    h1 = p1.astype(jnp.bfloat16)
    h2 = p2.astype(jnp.bfloat16)
    ph_ref[:, :n1] = h1
    ph_ref[:, n1:] = h2
    pl_ref[:, :n1] = (p1 - h1.astype(jnp.float32)).astype(jnp.bfloat16)
    pl_ref[:, n1:] = (p2 - h2.astype(jnp.float32)).astype(jnp.bfloat16)


def _mm2(x1, w1, x2, w2, bm=256):
    m, k1 = x1.shape
    _, n1 = w1.shape
    _, k2 = x2.shape
    _, n2 = w2.shape
    n = n1 + n2
    return pl.pallas_call(
        functools.partial(_mm2_body, n1=n1),
        grid=(m // bm,),
        in_specs=[pl.BlockSpec((bm, k1), lambda i: (i, 0)),
                  pl.BlockSpec((k1, n1), lambda i: (0, 0)),
                  pl.BlockSpec((bm, k2), lambda i: (i, 0)),
                  pl.BlockSpec((k2, n2), lambda i: (0, 0))],
        out_specs=[pl.BlockSpec((bm, n), lambda i: (i, 0)),
                   pl.BlockSpec((bm, n), lambda i: (i, 0))],
        out_shape=[jax.ShapeDtypeStruct((m, n), jnp.bfloat16),
                   jax.ShapeDtypeStruct((m, n), jnp.bfloat16)],
    )(x1, w1, x2, w2)


def _elu(x):
    return jnp.where(x > 0, x, jnp.exp(x) - 1.0)


def _split_dot(ah, al, ph, pl_):
    acc = jnp.dot(ah, ph, preferred_element_type=jnp.float32)
    acc += jnp.dot(al, ph, preferred_element_type=jnp.float32)
    acc += jnp.dot(ah, pl_, preferred_element_type=jnp.float32)
    return acc


def _amm_elu_body(ah_ref, al_ref, ph_ref, pl_ref, o_ref, acc_ref, *, nk):
    k = pl.program_id(1)

    @pl.when(k == 0)
    def _():
        acc_ref[:, :] = jnp.zeros_like(acc_ref)

    acc_ref[:, :] += _split_dot(ah_ref[:, :], al_ref[:, :],
                                ph_ref[:, :], pl_ref[:, :])

    @pl.when(k == nk - 1)
    def _():
        o_ref[:, :] = _elu(acc_ref[:, :])


def _amm_elu(ahl, phl, bm=512, bk=1024):
    ah, al = ahl
    ph, pl_ = phl
    m, kk = ah.shape
    _, n = ph.shape
    nk = kk // bk
    return pl.pallas_call(
        functools.partial(_amm_elu_body, nk=nk),
        grid=(m // bm, nk),
        in_specs=[pl.BlockSpec((bm, bk), lambda i, k: (i, k)),
                  pl.BlockSpec((bm, bk), lambda i, k: (i, k)),
                  pl.BlockSpec((bk, n), lambda i, k: (k, 0)),
                  pl.BlockSpec((bk, n), lambda i, k: (k, 0))],
        out_specs=pl.BlockSpec((bm, n), lambda i, k: (i, 0)),
        out_shape=jax.ShapeDtypeStruct((m, n), jnp.float32),
        scratch_shapes=[pltpu.VMEM((bm, n), jnp.float32)],
    )(ah, al, ph, pl_)


def _amm_elu_ft_body(ah_ref, al_ref, ph_ref, pl_ref, t_ref, o_ref, acc_ref,
                     *, nk):
    i = pl.program_id(0)
    k = pl.program_id(1)

    @pl.when((i == 0) & (k == 0))
    def _():
        o_ref[0, 0] = 0.0

    @pl.when(k == 0)
    def _():
        acc_ref[:, :] = jnp.zeros_like(acc_ref)

    acc_ref[:, :] += _split_dot(ah_ref[:, :], al_ref[:, :],
                                ph_ref[:, :], pl_ref[:, :])

    @pl.when(k == nk - 1)
    def _():
        d = t_ref[:, :] - _elu(acc_ref[:, :])
        o_ref[0, 0] += jnp.sum(d * d)


def _amm_elu_ft2_body(ah_ref, al_ref, ph_ref, pl_ref, t1_ref, t2_ref,
                      o1_ref, o2_ref, acc_ref, *, nk, n1):
    i = pl.program_id(0)
    k = pl.program_id(1)

    @pl.when((i == 0) & (k == 0))
    def _():
        o1_ref[0, 0] = 0.0
        o2_ref[0, 0] = 0.0

    @pl.when(k == 0)
    def _():
        acc_ref[:, :] = jnp.zeros_like(acc_ref)

    acc_ref[:, :] += _split_dot(ah_ref[:, :], al_ref[:, :],
                                ph_ref[:, :], pl_ref[:, :])

    @pl.when(k == nk - 1)
    def _():
        y = _elu(acc_ref[:, :])
        d1 = t1_ref[:, :] - y[:, :n1]
        d2 = t2_ref[:, :] - y[:, n1:]
        o1_ref[0, 0] += jnp.sum(d1 * d1)
        o2_ref[0, 0] += jnp.sum(d2 * d2)


def _amm_elu_ft2(ahl, phl, t1, t2, bm=512, bk=1024):
    ah, al = ahl
    ph, pl_ = phl
    m, kk = ah.shape
    _, n = ph.shape
    n1 = t1.shape[1]
    nk = kk // bk
    o1, o2 = pl.pallas_call(
        functools.partial(_amm_elu_ft2_body, nk=nk, n1=n1),
        grid=(m // bm, nk),
        in_specs=[pl.BlockSpec((bm, bk), lambda i, k: (i, k)),
                  pl.BlockSpec((bm, bk), lambda i, k: (i, k)),
                  pl.BlockSpec((bk, n), lambda i, k: (k, 0)),
                  pl.BlockSpec((bk, n), lambda i, k: (k, 0)),
                  pl.BlockSpec((bm, n1), lambda i, k: (i, 0)),
                  pl.BlockSpec((bm, n - n1), lambda i, k: (i, 0))],
        out_specs=[pl.BlockSpec((1, 1), lambda i, k: (0, 0),
                                memory_space=pltpu.SMEM),
                   pl.BlockSpec((1, 1), lambda i, k: (0, 0),
                                memory_space=pltpu.SMEM)],
        out_shape=[jax.ShapeDtypeStruct((1, 1), jnp.float32),
                   jax.ShapeDtypeStruct((1, 1), jnp.float32)],
        scratch_shapes=[pltpu.VMEM((bm, n), jnp.float32)],
    )(ah, al, ph, pl_, t1, t2)
    return o1[0, 0], o2[0, 0]


def _amm_elu_ft(ahl, phl, t, bm=512, bk=1024):
    ah, al = ahl
    ph, pl_ = phl
    m, kk = ah.shape
    _, n = ph.shape
    nk = kk // bk
    out = pl.pallas_call(
        functools.partial(_amm_elu_ft_body, nk=nk),
        grid=(m // bm, nk),
        in_specs=[pl.BlockSpec((bm, bk), lambda i, k: (i, k)),
                  pl.BlockSpec((bm, bk), lambda i, k: (i, k)),
                  pl.BlockSpec((bk, n), lambda i, k: (k, 0)),
                  pl.BlockSpec((bk, n), lambda i, k: (k, 0)),
                  pl.BlockSpec((bm, n), lambda i, k: (i, 0))],
        out_specs=pl.BlockSpec((1, 1), lambda i, k: (0, 0),
                               memory_space=pltpu.SMEM),
        out_shape=jax.ShapeDtypeStruct((1, 1), jnp.float32),
        scratch_shapes=[pltpu.VMEM((bm, n), jnp.float32)],
    )(ah, al, ph, pl_, t)
    return out[0, 0]


def _coef_mm_body(w_ref, h_ref, hi_ref, dw_ref, o_ref, se_ref, acc_ref, *, nk):
    i = pl.program_id(0)
    k = pl.program_id(1)

    @pl.when((i == 0) & (k == 0))
    def _():
        se_ref[0, 0] = 0.0

    @pl.when(k == 0)
    def _():
        acc_ref[:, :] = jnp.zeros_like(acc_ref)

    w = w_ref[:, :]
    wh = w.astype(jnp.bfloat16)
    wl = (w - wh.astype(jnp.float32)).astype(jnp.bfloat16)
    h = h_ref[:, :]
    hh = h.astype(jnp.bfloat16)
    hl = (h - hh.astype(jnp.float32)).astype(jnp.bfloat16)
    acc_ref[:, :] += _split_dot(wh, wl, hh, hl)

    @pl.when(k == nk - 1)
    def _():
        hi = hi_ref[:, :]
        hc = acc_ref[:, :] - dw_ref[:, :] * hi
        o_ref[:, :] = hc
        d = hi - hc
        se_ref[0, 0] += jnp.sum(d * d)


def _coef_mm(w, h, dw, bm=256, bk=1024):
    m, kk = w.shape
    _, n = h.shape
    nk = kk // bk
    hc, se = pl.pallas_call(
        functools.partial(_coef_mm_body, nk=nk),
        grid=(m // bm, nk),
        in_specs=[pl.BlockSpec((bm, bk), lambda i, k: (i, k)),
                  pl.BlockSpec((bk, n), lambda i, k: (k, 0)),
                  pl.BlockSpec((bm, n), lambda i, k: (i, 0)),
                  pl.BlockSpec((bm, 1), lambda i, k: (i, 0))],
        out_specs=[pl.BlockSpec((bm, n), lambda i, k: (i, 0)),
                   pl.BlockSpec((1, 1), lambda i, k: (0, 0),
                                memory_space=pltpu.SMEM)],
        out_shape=[jax.ShapeDtypeStruct((m, n), jnp.float32),
                   jax.ShapeDtypeStruct((1, 1), jnp.float32)],
        scratch_shapes=[pltpu.VMEM((bm, n), jnp.float32)],
    )(w, h, h, dw)
    return hc, se[0, 0]


def _prep_body(w_ref, w2_ref, w31_ref, w32_ref, tt_ref,
               c3_ref, zis_ref, zjs_ref, pos_ref,
               dw_ref, dw2_ref, dw31_ref, dw32_ref,
               creg_ref, cq_ref, cons_ref, *, bm):
    i = pl.program_id(0)

    @pl.when(i == 0)
    def _():
        creg_ref[0, 0] = 0.0
        cq_ref[0, 0] = 0.0
        cons_ref[0, 0] = 0.0

    n = w_ref.shape[1]
    rows = lax.broadcasted_iota(jnp.int32, (bm, n), 0) + i * bm
    cols = lax.broadcasted_iota(jnp.int32, (bm, n), 1)
    diag = rows == cols
    c = jnp.where(diag, 0.0, w_ref[:, :])
    c2 = jnp.where(diag, 0.0, w2_ref[:, :])
    c31 = jnp.where(diag, 0.0, w31_ref[:, :])
    c32 = jnp.where(diag, 0.0, w32_ref[:, :])
    dw_ref[:, :] = jnp.sum(jnp.where(diag, w_ref[:, :], 0.0),
                           axis=1, keepdims=True)
    dw2_ref[:, :] = jnp.sum(jnp.where(diag, w2_ref[:, :], 0.0),
                            axis=1, keepdims=True)
    dw31_ref[:, :] = jnp.sum(jnp.where(diag, w31_ref[:, :], 0.0),
                             axis=1, keepdims=True)
    dw32_ref[:, :] = jnp.sum(jnp.where(diag, w32_ref[:, :], 0.0),
                             axis=1, keepdims=True)
    c3 = 0.7 * c31 + 0.3 * c32
    c3_ref[:, :] = c3
    creg_ref[0, 0] += (jnp.sum(jnp.abs(c)) + jnp.sum(jnp.abs(c2))
                       + jnp.sum(jnp.abs(c31)) + jnp.sum(jnp.abs(c32)))
    cq_ref[0, 0] += jnp.sum(jnp.abs(c3 * tt_ref[:, :]))
    cons_ref[0, 0] += jnp.sum((c3 - c) ** 2) + jnp.sum((c3 - c2) ** 2)
    n31 = jnp.sqrt(jnp.sum(c31 * c31, axis=1, keepdims=True))
    n32 = jnp.sqrt(jnp.sum(c32 * c32, axis=1, keepdims=True))
    zis = c31 / jnp.maximum(n31, 1e-12)
    zjs = c32 / jnp.maximum(n32, 1e-12)
    zis_ref[:, :] = zis.astype(jnp.bfloat16)
    zjs_ref[:, :] = zjs.astype(jnp.bfloat16)
    pos_ref[:, :] = jnp.sum(zis * zjs, axis=1, keepdims=True)


def _prep(w, w2, w31, w32, theta_t, bm=128):
    n = w.shape[0]
    outs = pl.pallas_call(
        functools.partial(_prep_body, bm=bm),
        grid=(n // bm,),
        in_specs=[pl.BlockSpec((bm, n), lambda i: (i, 0))] * 5,
        out_specs=[pl.BlockSpec((bm, n), lambda i: (i, 0)),
                   pl.BlockSpec((bm, n), lambda i: (i, 0)),
                   pl.BlockSpec((bm, n), lambda i: (i, 0)),
                   pl.BlockSpec((bm, 1), lambda i: (i, 0)),
                   pl.BlockSpec((bm, 1), lambda i: (i, 0)),
                   pl.BlockSpec((bm, 1), lambda i: (i, 0)),
                   pl.BlockSpec((bm, 1), lambda i: (i, 0)),
                   pl.BlockSpec((bm, 1), lambda i: (i, 0)),
                   pl.BlockSpec((1, 1), lambda i: (0, 0),
                                memory_space=pltpu.SMEM),
                   pl.BlockSpec((1, 1), lambda i: (0, 0),
                                memory_space=pltpu.SMEM),
                   pl.BlockSpec((1, 1), lambda i: (0, 0),
                                memory_space=pltpu.SMEM)],
        out_shape=[jax.ShapeDtypeStruct((n, n), jnp.float32),
                   jax.ShapeDtypeStruct((n, n), jnp.bfloat16),
                   jax.ShapeDtypeStruct((n, n), jnp.bfloat16),
                   jax.ShapeDtypeStruct((n, 1), jnp.float32),
                   jax.ShapeDtypeStruct((n, 1), jnp.float32),
                   jax.ShapeDtypeStruct((n, 1), jnp.float32),
                   jax.ShapeDtypeStruct((n, 1), jnp.float32),
                   jax.ShapeDtypeStruct((n, 1), jnp.float32),
                   jax.ShapeDtypeStruct((1, 1), jnp.float32),
                   jax.ShapeDtypeStruct((1, 1), jnp.float32),
                   jax.ShapeDtypeStruct((1, 1), jnp.float32)],
    )(w, w2, w31, w32, theta_t)
    (c3, zis, zjs, pos, dw, dw2, dw31, dw32, creg, cq, cons) = outs
    return (c3, zis, zjs, pos, dw, dw2, dw31, dw32,
            creg[0, 0], cq[0, 0], cons[0, 0])


_DN = (((1,), (1,)), ((), ()))


def _gram_body(zis_i, zjs_i, zis_j, zjs_j, y_i, yt_j, pos_ref, post_ref,
               cl_ref, a1, a2, a3, neg1, neg2, *, nmi, nmj, nk, bm, bn):
    i = pl.program_id(0)
    j = pl.program_id(1)
    k = pl.program_id(2)

    @pl.when((i == 0) & (j == 0) & (k == 0))
    def _():
        neg1[:, :] = jnp.zeros_like(neg1)
        neg2[:, :] = jnp.zeros_like(neg2)

    @pl.when(k == 0)
    def _():
        a1[:, :] = jnp.zeros_like(a1)
        a2[:, :] = jnp.zeros_like(a2)
        a3[:, :] = jnp.zeros_like(a3)

    a1[:, :] += lax.dot_general(zis_i[:, :], zjs_j[:, :], _DN,
                                preferred_element_type=jnp.float32)
    a2[:, :] += lax.dot_general(zis_i[:, :], zis_j[:, :], _DN,
                                preferred_element_type=jnp.float32)
    a3[:, :] += lax.dot_general(zjs_i[:, :], zjs_j[:, :], _DN,
                                preferred_element_type=jnp.float32)

    @pl.when(k == nk - 1)
    def _():
        nm = (y_i[:, :] != yt_j[:, :]).astype(jnp.float32)
        e1 = jnp.exp(a1[:, :]) * nm
        e2 = jnp.exp(a2[:, :]) * nm
        e3 = jnp.exp(a3[:, :]) * nm
        neg1[pl.ds(i * bm, bm), :] += jnp.sum(e1 + e2, axis=1, keepdims=True)
        neg2[:, pl.ds(j * bn, bn)] += jnp.sum(e1 + e3, axis=0)[None, :]

        @pl.when((i == nmi - 1) & (j == nmj - 1))
        def _():
            p = pos_ref[:, :]
            pt = post_ref[:, :]
            cl_ref[0, 0] = (jnp.sum(jnp.log(jnp.exp(p) + neg1[:, :]) - p)
                            + jnp.sum(jnp.log(jnp.exp(pt) + neg2[:, :]) - pt))


def _gram(zis, zjs, y, yt, pos, post, bm=1024, bn=1024, bk=1024):
    n = zis.shape[0]
    nmi, nmj, nk = n // bm, n // bn, n // bk
    cl = pl.pallas_call(
        functools.partial(_gram_body, nmi=nmi, nmj=nmj, nk=nk, bm=bm, bn=bn),
        grid=(nmi, nmj, nk),
        in_specs=[pl.BlockSpec((bm, bk), lambda i, j, k: (i, k)),
                  pl.BlockSpec((bm, bk), lambda i, j, k: (i, k)),
                  pl.BlockSpec((bn, bk), lambda i, j, k: (j, k)),
                  pl.BlockSpec((bn, bk), lambda i, j, k: (j, k)),
                  pl.BlockSpec((bm, 1), lambda i, j, k: (i, 0)),
                  pl.BlockSpec((1, bn), lambda i, j, k: (0, j)),
                  pl.BlockSpec((n, 1), lambda i, j, k: (0, 0)),
                  pl.BlockSpec((1, n), lambda i, j, k: (0, 0))],
        out_specs=pl.BlockSpec((1, 1), lambda i, j, k: (0, 0),
                               memory_space=pltpu.SMEM),
        out_shape=jax.ShapeDtypeStruct((1, 1), jnp.float32),
        scratch_shapes=[pltpu.VMEM((bm, bn), jnp.float32),
                        pltpu.VMEM((bm, bn), jnp.float32),
                        pltpu.VMEM((bm, bn), jnp.float32),
                        pltpu.VMEM((n, 1), jnp.float32),
                        pltpu.VMEM((1, n), jnp.float32)],
    )(zis, zis, zjs, zjs, y, yt, pos, post)
    return cl[0, 0]


def _edge_dots(h, s, r):
    n, d = h.shape
    e = s.shape[0]
    info = plsc.get_sparse_core_info()
    nw = info.num_cores * info.num_subcores
    per_w = e // nw
    ch = 128
    nch = per_w // ch
    mesh = plsc.VectorSubcoreMesh(core_axis_name="c", subcore_axis_name="s")

    def body(h_hbm, s_hbm, r_hbm, out_hbm, sidx, ridx, arow, brow, ovec,
             sem1, sem2):
        wid = lax.axis_index("s") * info.num_cores + lax.axis_index("c")

        def chunk(c, carry):
            base = wid * per_w + c * ch
            pltpu.sync_copy(s_hbm.at[pl.ds(base, ch)], sidx)
            pltpu.sync_copy(r_hbm.at[pl.ds(base, ch)], ridx)
            cp1 = pltpu.async_copy(h_hbm.at[sidx], arow, sem1)
            cp2 = pltpu.async_copy(h_hbm.at[ridx], brow, sem2)
            cp1.wait()
            cp2.wait()

            def edge(eo, cc):
                for sub in range(8):
                    ei = eo * 8 + sub
                    acc = arow[ei, pl.ds(0, 16)] * brow[ei, pl.ds(0, 16)]
                    for g in range(1, d // 16):
                        acc = acc + (arow[ei, pl.ds(g * 16, 16)]
                                     * brow[ei, pl.ds(g * 16, 16)])
                    ovec[eo, pl.ds(sub * 16, 16)] = acc
                return cc

            lax.fori_loop(0, ch // 8, edge, 0)
            obase = pl.multiple_of(base // 8, 8)
            pltpu.sync_copy(ovec, out_hbm.at[pl.ds(obase, ch // 8)])
            return carry

        lax.fori_loop(0, nch, chunk, 0)

    return pl.kernel(
        body,
        out_type=jax.ShapeDtypeStruct((e // 8, 128), jnp.float32),
        mesh=mesh,
        scratch_types=[pltpu.VMEM((ch,), jnp.int32),
                       pltpu.VMEM((ch,), jnp.int32),
                       pltpu.VMEM((ch, d), jnp.float32),
                       pltpu.VMEM((ch, d), jnp.float32),
                       pltpu.VMEM((ch // 8, 128), jnp.float32),
                       pltpu.SemaphoreType.DMA,
                       pltpu.SemaphoreType.DMA],
    )(h, s, r)


def _st_body(d1, d2, d3, d4, o_ref):
    i = pl.program_id(0)

    @pl.when(i == 0)
    def _():
        o_ref[0, 0] = 0.0

    seg = (lax.broadcasted_iota(jnp.int32, (128, 128), 0) // 16
           == lax.broadcasted_iota(jnp.int32, (128, 128), 1)
           ).astype(jnp.float32)
    colmask = lax.broadcasted_iota(jnp.int32, d1.shape, 1) < 8
    tot = 0.0
    for dref in (d1, d2, d3, d4):
        dot = jnp.dot(dref[:, :], seg, preferred_element_type=jnp.float32)
        sp = jnp.maximum(-dot, 0.0) + jnp.log(1.0 + jnp.exp(-jnp.abs(dot)))
        tot += jnp.sum(jnp.where(colmask, sp, 0.0))
    o_ref[0, 0] += tot


def _st_reduce(d1, d2, d3, d4, be=4096):
    e8 = d1.shape[0]
    out = pl.pallas_call(
        _st_body,
        grid=(e8 // be,),
        in_specs=[pl.BlockSpec((be, 128), lambda i: (i, 0))] * 4,
        out_specs=pl.BlockSpec((1, 1), lambda i: (0, 0),
                               memory_space=pltpu.SMEM),
        out_shape=jax.ShapeDtypeStruct((1, 1), jnp.float32),
    )(d1, d2, d3, d4)
    return out[0, 0]


def kernel(X, A, S, R, X2, A2, S2, R2, y_pred, Theta,
           weight, weight2, weight31, weight32,
           W11, W12, Wd11, Wd12, W21, W22, Wd21, Wd22, W31, Wd31):
    n, f1 = X.shape
    f2 = X2.shape[1]
    h2 = W12.shape[1]
    h3 = W31.shape[1]

    pad = 128 - h3
    W31p = jnp.pad(W31, ((0, 0), (0, pad)))
    Wd31p = jnp.pad(Wd31, ((0, pad), (0, 0)))

    Ahl = _split(A)
    A2hl = _split(A2)

    H = _amm_elu(Ahl, _mm(_amm_elu(Ahl, _mm(X, W11)), W12))
    Hb = _amm_elu(A2hl, _mm(_amm_elu(A2hl, _mm(X2, W21)), W22))

    d1 = _edge_dots(H, S, R)
    d2 = _edge_dots(Hb, S2, R2)

    (c3, zis, zjs, pos, dw, dw2, dw31, dw32, creg, cq, cons) = _prep(
        weight, weight2, weight31, weight32, Theta.T)

    h1dim = Wd11.shape[1]
    HC, se1 = _coef_mm(weight, H, dw)
    M1 = _amm_elu(Ahl, _mm2(HC, Wd11, H, W31p))
    dec1 = M1[:, :h1dim]
    H31 = M1[:, h1dim:]
    d3 = _edge_dots(H31, S, R)
    HC31, se3 = _coef_mm(weight31, H31, dw31)
    ft1, ft3 = _amm_elu_ft2(Ahl, _mm2(dec1, Wd12, HC31, Wd31p), X, H)

    HC2, se2 = _coef_mm(weight2, Hb, dw2)
    M2 = _amm_elu(A2hl, _mm2(HC2, Wd21, Hb, W31p))
    dec2 = M2[:, :h1dim]
    H32 = M2[:, h1dim:]
    d4 = _edge_dots(H32, S2, R2)
    HC32, se4 = _coef_mm(weight32, H32, dw32)
    ft2, ft4 = _amm_elu_ft2(A2hl, _mm2(dec2, Wd22, HC32, Wd31p), X2, Hb)

    yt = y_pred.reshape(1, n)
    cl_sum = _gram(zis, zjs, y_pred, yt, pos, pos.reshape(1, n))

    st_loss = _st_reduce(d1, d2, d3, d4)

    ft_loss = (ft1 / (n * f1) + ft2 / (n * f2)
               + ft3 / (n * h2) + ft4 / (n * h2))
    se_loss = 0.5 * (se1 / (n * h2) + se2 / (n * h2)
                     + se3 / (n * h3) + se4 / (n * h3))
    cl_loss = cl_sum / (2.0 * n)

    loss = (ft_loss + 0.1 * st_loss + se_loss + 0.1 * creg
            + 0.1 * cl_loss + 0.1 * cq + 0.1 * cons)
    return (loss, ft_loss, st_loss, se_loss, creg, cons, cl_loss, cq, c3)


# speedup vs baseline: 3.1382x; 1.0943x over previous
import functools

import jax
import jax.numpy as jnp
from jax import lax
from jax.experimental import pallas as pl
from jax.experimental.pallas import tpu as pltpu
from jax.experimental.pallas import tpu_sc as plsc


def _split_body(x_ref, hi_ref, lo_ref):
    x = x_ref[:, :]
    hi = x.astype(jnp.bfloat16)
    hi_ref[:, :] = hi
    lo_ref[:, :] = (x - hi.astype(jnp.float32)).astype(jnp.bfloat16)


def _split(x, bm=256):
    m, k = x.shape
    return pl.pallas_call(
        _split_body,
        grid=(m // bm,),
        in_specs=[pl.BlockSpec((bm, k), lambda i: (i, 0))],
        out_specs=[pl.BlockSpec((bm, k), lambda i: (i, 0)),
                   pl.BlockSpec((bm, k), lambda i: (i, 0))],
        out_shape=[jax.ShapeDtypeStruct((m, k), jnp.bfloat16),
                   jax.ShapeDtypeStruct((m, k), jnp.bfloat16)],
    )(x)


def _mm_body(x_ref, w_ref, ph_ref, pl_ref):
    p = jnp.dot(x_ref[:, :], w_ref[:, :], preferred_element_type=jnp.float32)
    ph = p.astype(jnp.bfloat16)
    ph_ref[:, :] = ph
    pl_ref[:, :] = (p - ph.astype(jnp.float32)).astype(jnp.bfloat16)


def _mm(x, w, bm=256):
    m, k = x.shape
    _, n = w.shape
    return pl.pallas_call(
        _mm_body,
        grid=(m // bm,),
        in_specs=[pl.BlockSpec((bm, k), lambda i: (i, 0)),
                  pl.BlockSpec((k, n), lambda i: (0, 0))],
        out_specs=[pl.BlockSpec((bm, n), lambda i: (i, 0)),
                   pl.BlockSpec((bm, n), lambda i: (i, 0))],
        out_shape=[jax.ShapeDtypeStruct((m, n), jnp.bfloat16),
                   jax.ShapeDtypeStruct((m, n), jnp.bfloat16)],
    )(x, w)


def _mm2_body(x1_ref, w1_ref, x2_ref, w2_ref, ph_ref, pl_ref, *, n1):
    p1 = jnp.dot(x1_ref[:, :], w1_ref[:, :], preferred_element_type=jnp.float32)
    p2 = jnp.dot(x2_ref[:, :], w2_ref[:, :], preferred_element_type=jnp.float32)
    h1 = p1.astype(jnp.bfloat16)
    h2 = p2.astype(jnp.bfloat16)
    ph_ref[:, :n1] = h1
    ph_ref[:, n1:] = h2
    pl_ref[:, :n1] = (p1 - h1.astype(jnp.float32)).astype(jnp.bfloat16)
    pl_ref[:, n1:] = (p2 - h2.astype(jnp.float32)).astype(jnp.bfloat16)


def _mm2(x1, w1, x2, w2, bm=256):
    m, k1 = x1.shape
    _, n1 = w1.shape
    _, k2 = x2.shape
    _, n2 = w2.shape
    n = n1 + n2
    return pl.pallas_call(
        functools.partial(_mm2_body, n1=n1),
        grid=(m // bm,),
        in_specs=[pl.BlockSpec((bm, k1), lambda i: (i, 0)),
                  pl.BlockSpec((k1, n1), lambda i: (0, 0)),
                  pl.BlockSpec((bm, k2), lambda i: (i, 0)),
                  pl.BlockSpec((k2, n2), lambda i: (0, 0))],
        out_specs=[pl.BlockSpec((bm, n), lambda i: (i, 0)),
                   pl.BlockSpec((bm, n), lambda i: (i, 0))],
        out_shape=[jax.ShapeDtypeStruct((m, n), jnp.bfloat16),
                   jax.ShapeDtypeStruct((m, n), jnp.bfloat16)],
    )(x1, w1, x2, w2)


def _elu(x):
    return jnp.where(x > 0, x, jnp.exp(x) - 1.0)


def _split_dot(ah, al, ph, pl_):
    acc = jnp.dot(ah, ph, preferred_element_type=jnp.float32)
    acc += jnp.dot(al, ph, preferred_element_type=jnp.float32)
    acc += jnp.dot(ah, pl_, preferred_element_type=jnp.float32)
    return acc


def _amm_elu_body(ah_ref, al_ref, ph_ref, pl_ref, o_ref, acc_ref, *, nk):
    k = pl.program_id(1)

    @pl.when(k == 0)
    def _():
        acc_ref[:, :] = jnp.zeros_like(acc_ref)

    acc_ref[:, :] += _split_dot(ah_ref[:, :], al_ref[:, :],
                                ph_ref[:, :], pl_ref[:, :])

    @pl.when(k == nk - 1)
    def _():
        o_ref[:, :] = _elu(acc_ref[:, :])


def _amm_elu_split_body(a_ref, ph_ref, pl_ref, o_ref, ah_ref, al_ref,
                        acc_ref, *, nk):
    k = pl.program_id(1)

    @pl.when(k == 0)
    def _():
        acc_ref[:, :] = jnp.zeros_like(acc_ref)

    a = a_ref[:, :]
    ah = a.astype(jnp.bfloat16)
    al = (a - ah.astype(jnp.float32)).astype(jnp.bfloat16)
    ah_ref[:, :] = ah
    al_ref[:, :] = al
    acc_ref[:, :] += _split_dot(ah, al, ph_ref[:, :], pl_ref[:, :])

    @pl.when(k == nk - 1)
    def _():
        o_ref[:, :] = _elu(acc_ref[:, :])


def _amm_elu_split(a, phl, bm=512, bk=1024):
    ph, pl_ = phl
    m, kk = a.shape
    _, n = ph.shape
    nk = kk // bk
    o, ah, al = pl.pallas_call(
        functools.partial(_amm_elu_split_body, nk=nk),
        grid=(m // bm, nk),
        in_specs=[pl.BlockSpec((bm, bk), lambda i, k: (i, k)),
                  pl.BlockSpec((bk, n), lambda i, k: (k, 0)),
                  pl.BlockSpec((bk, n), lambda i, k: (k, 0))],
        out_specs=[pl.BlockSpec((bm, n), lambda i, k: (i, 0)),
                   pl.BlockSpec((bm, bk), lambda i, k: (i, k)),
                   pl.BlockSpec((bm, bk), lambda i, k: (i, k))],
        out_shape=[jax.ShapeDtypeStruct((m, n), jnp.float32),
                   jax.ShapeDtypeStruct((m, kk), jnp.bfloat16),
                   jax.ShapeDtypeStruct((m, kk), jnp.bfloat16)],
        scratch_shapes=[pltpu.VMEM((bm, n), jnp.float32)],
    )(a, ph, pl_)
    return o, (ah, al)


def _amm_elu(ahl, phl, bm=512, bk=1024):
    ah, al = ahl
    ph, pl_ = phl
    m, kk = ah.shape
    _, n = ph.shape
    nk = kk // bk
    return pl.pallas_call(
        functools.partial(_amm_elu_body, nk=nk),
        grid=(m // bm, nk),
        in_specs=[pl.BlockSpec((bm, bk), lambda i, k: (i, k)),
                  pl.BlockSpec((bm, bk), lambda i, k: (i, k)),
                  pl.BlockSpec((bk, n), lambda i, k: (k, 0)),
                  pl.BlockSpec((bk, n), lambda i, k: (k, 0))],
        out_specs=pl.BlockSpec((bm, n), lambda i, k: (i, 0)),
        out_shape=jax.ShapeDtypeStruct((m, n), jnp.float32),
        scratch_shapes=[pltpu.VMEM((bm, n), jnp.float32)],
    )(ah, al, ph, pl_)


def _amm_elu_ft_body(ah_ref, al_ref, ph_ref, pl_ref, t_ref, o_ref, acc_ref,
                     *, nk):
    i = pl.program_id(0)
    k = pl.program_id(1)

    @pl.when((i == 0) & (k == 0))
    def _():
        o_ref[0, 0] = 0.0

    @pl.when(k == 0)
    def _():
        acc_ref[:, :] = jnp.zeros_like(acc_ref)

    acc_ref[:, :] += _split_dot(ah_ref[:, :], al_ref[:, :],
                                ph_ref[:, :], pl_ref[:, :])

    @pl.when(k == nk - 1)
    def _():
        d = t_ref[:, :] - _elu(acc_ref[:, :])
        o_ref[0, 0] += jnp.sum(d * d)


def _amm_elu_ft2_body(ah_ref, al_ref, ph_ref, pl_ref, t1_ref, t2_ref,
                      o1_ref, o2_ref, acc_ref, *, nk, n1):
    i = pl.program_id(0)
    k = pl.program_id(1)

    @pl.when((i == 0) & (k == 0))
    def _():
        o1_ref[0, 0] = 0.0
        o2_ref[0, 0] = 0.0

    @pl.when(k == 0)
    def _():
        acc_ref[:, :] = jnp.zeros_like(acc_ref)

    acc_ref[:, :] += _split_dot(ah_ref[:, :], al_ref[:, :],
                                ph_ref[:, :], pl_ref[:, :])

    @pl.when(k == nk - 1)
    def _():
        y = _elu(acc_ref[:, :])
        d1 = t1_ref[:, :] - y[:, :n1]
        d2 = t2_ref[:, :] - y[:, n1:]
        o1_ref[0, 0] += jnp.sum(d1 * d1)
        o2_ref[0, 0] += jnp.sum(d2 * d2)


def _amm_elu_ft2(ahl, phl, t1, t2, bm=512, bk=1024):
    ah, al = ahl
    ph, pl_ = phl
    m, kk = ah.shape
    _, n = ph.shape
    n1 = t1.shape[1]
    nk = kk // bk
    o1, o2 = pl.pallas_call(
        functools.partial(_amm_elu_ft2_body, nk=nk, n1=n1),
        grid=(m // bm, nk),
        in_specs=[pl.BlockSpec((bm, bk), lambda i, k: (i, k)),
                  pl.BlockSpec((bm, bk), lambda i, k: (i, k)),
                  pl.BlockSpec((bk, n), lambda i, k: (k, 0)),
                  pl.BlockSpec((bk, n), lambda i, k: (k, 0)),
                  pl.BlockSpec((bm, n1), lambda i, k: (i, 0)),
                  pl.BlockSpec((bm, n - n1), lambda i, k: (i, 0))],
        out_specs=[pl.BlockSpec((1, 1), lambda i, k: (0, 0),
                                memory_space=pltpu.SMEM),
                   pl.BlockSpec((1, 1), lambda i, k: (0, 0),
                                memory_space=pltpu.SMEM)],
        out_shape=[jax.ShapeDtypeStruct((1, 1), jnp.float32),
                   jax.ShapeDtypeStruct((1, 1), jnp.float32)],
        scratch_shapes=[pltpu.VMEM((bm, n), jnp.float32)],
    )(ah, al, ph, pl_, t1, t2)
    return o1[0, 0], o2[0, 0]


def _amm_elu_ft(ahl, phl, t, bm=512, bk=1024):
    ah, al = ahl
    ph, pl_ = phl
    m, kk = ah.shape
    _, n = ph.shape
    nk = kk // bk
    out = pl.pallas_call(
        functools.partial(_amm_elu_ft_body, nk=nk),
        grid=(m // bm, nk),
        in_specs=[pl.BlockSpec((bm, bk), lambda i, k: (i, k)),
                  pl.BlockSpec((bm, bk), lambda i, k: (i, k)),
                  pl.BlockSpec((bk, n), lambda i, k: (k, 0)),
                  pl.BlockSpec((bk, n), lambda i, k: (k, 0)),
                  pl.BlockSpec((bm, n), lambda i, k: (i, 0))],
        out_specs=pl.BlockSpec((1, 1), lambda i, k: (0, 0),
                               memory_space=pltpu.SMEM),
        out_shape=jax.ShapeDtypeStruct((1, 1), jnp.float32),
        scratch_shapes=[pltpu.VMEM((bm, n), jnp.float32)],
    )(ah, al, ph, pl_, t)
    return out[0, 0]


def _coef_pass(w_ref, h_ref, acc_ref):
    w = w_ref[:, :]
    wh = w.astype(jnp.bfloat16)
    wl = (w - wh.astype(jnp.float32)).astype(jnp.bfloat16)
    h = h_ref[:, :]
    hh = h.astype(jnp.bfloat16)
    hl = (h - hh.astype(jnp.float32)).astype(jnp.bfloat16)
    acc_ref[:, :] += _split_dot(wh, wl, hh, hl)


def _coef_mm2_body(w_ref, h_ref, hi_ref, dw_ref,
                   wb_ref, hb_ref, hib_ref, dwb_ref,
                   o_ref, se_ref, ob_ref, seb_ref, acc_ref, accb_ref, *, nk):
    i = pl.program_id(0)
    k = pl.program_id(1)

    @pl.when((i == 0) & (k == 0))
    def _():
        se_ref[0, 0] = 0.0
        seb_ref[0, 0] = 0.0

    @pl.when(k == 0)
    def _():
        acc_ref[:, :] = jnp.zeros_like(acc_ref)
        accb_ref[:, :] = jnp.zeros_like(accb_ref)

    _coef_pass(w_ref, h_ref, acc_ref)
    _coef_pass(wb_ref, hb_ref, accb_ref)

    @pl.when(k == nk - 1)
    def _():
        hi = hi_ref[:, :]
        hc = acc_ref[:, :] - dw_ref[:, :] * hi
        o_ref[:, :] = hc
        d = hi - hc
        se_ref[0, 0] += jnp.sum(d * d)
        hib = hib_ref[:, :]
        hcb = accb_ref[:, :] - dwb_ref[:, :] * hib
        ob_ref[:, :] = hcb
        db = hib - hcb
        seb_ref[0, 0] += jnp.sum(db * db)


def _coef_mm2(w, h, dw, wb, hb, dwb, bm=256, bk=1024):
    m, kk = w.shape
    _, n = h.shape
    nk = kk // bk
    hc, se, hcb, seb = pl.pallas_call(
        functools.partial(_coef_mm2_body, nk=nk),
        grid=(m // bm, nk),
        in_specs=[pl.BlockSpec((bm, bk), lambda i, k: (i, k)),
                  pl.BlockSpec((bk, n), lambda i, k: (k, 0)),
                  pl.BlockSpec((bm, n), lambda i, k: (i, 0)),
                  pl.BlockSpec((bm, 1), lambda i, k: (i, 0)),
                  pl.BlockSpec((bm, bk), lambda i, k: (i, k)),
                  pl.BlockSpec((bk, n), lambda i, k: (k, 0)),
                  pl.BlockSpec((bm, n), lambda i, k: (i, 0)),
                  pl.BlockSpec((bm, 1), lambda i, k: (i, 0))],
        out_specs=[pl.BlockSpec((bm, n), lambda i, k: (i, 0)),
                   pl.BlockSpec((1, 1), lambda i, k: (0, 0),
                                memory_space=pltpu.SMEM),
                   pl.BlockSpec((bm, n), lambda i, k: (i, 0)),
                   pl.BlockSpec((1, 1), lambda i, k: (0, 0),
                                memory_space=pltpu.SMEM)],
        out_shape=[jax.ShapeDtypeStruct((m, n), jnp.float32),
                   jax.ShapeDtypeStruct((1, 1), jnp.float32),
                   jax.ShapeDtypeStruct((m, n), jnp.float32),
                   jax.ShapeDtypeStruct((1, 1), jnp.float32)],
        scratch_shapes=[pltpu.VMEM((bm, n), jnp.float32),
                        pltpu.VMEM((bm, n), jnp.float32)],
    )(w, h, h, dw, wb, hb, hb, dwb)
    return hc, se[0, 0], hcb, seb[0, 0]


def _prep_body(w_ref, w2_ref, w31_ref, w32_ref, tt_ref,
               c3_ref, zis_ref, zjs_ref, pos_ref,
               dw_ref, dw2_ref, dw31_ref, dw32_ref,
               creg_ref, cq_ref, cons_ref, *, bm):
    i = pl.program_id(0)

    @pl.when(i == 0)
    def _():
        creg_ref[0, 0] = 0.0
        cq_ref[0, 0] = 0.0
        cons_ref[0, 0] = 0.0

    n = w_ref.shape[1]
    rows = lax.broadcasted_iota(jnp.int32, (bm, n), 0) + i * bm
    cols = lax.broadcasted_iota(jnp.int32, (bm, n), 1)
    diag = rows == cols
    c = jnp.where(diag, 0.0, w_ref[:, :])
    c2 = jnp.where(diag, 0.0, w2_ref[:, :])
    c31 = jnp.where(diag, 0.0, w31_ref[:, :])
    c32 = jnp.where(diag, 0.0, w32_ref[:, :])
    dw_ref[:, :] = jnp.sum(jnp.where(diag, w_ref[:, :], 0.0),
                           axis=1, keepdims=True)
    dw2_ref[:, :] = jnp.sum(jnp.where(diag, w2_ref[:, :], 0.0),
                            axis=1, keepdims=True)
    dw31_ref[:, :] = jnp.sum(jnp.where(diag, w31_ref[:, :], 0.0),
                             axis=1, keepdims=True)
    dw32_ref[:, :] = jnp.sum(jnp.where(diag, w32_ref[:, :], 0.0),
                             axis=1, keepdims=True)
    c3 = 0.7 * c31 + 0.3 * c32
    c3_ref[:, :] = c3
    creg_ref[0, 0] += (jnp.sum(jnp.abs(c)) + jnp.sum(jnp.abs(c2))
                       + jnp.sum(jnp.abs(c31)) + jnp.sum(jnp.abs(c32)))
    cq_ref[0, 0] += jnp.sum(jnp.abs(c3 * tt_ref[:, :]))
    cons_ref[0, 0] += jnp.sum((c3 - c) ** 2) + jnp.sum((c3 - c2) ** 2)
    n31 = jnp.sqrt(jnp.sum(c31 * c31, axis=1, keepdims=True))
    n32 = jnp.sqrt(jnp.sum(c32 * c32, axis=1, keepdims=True))
    zis = c31 / jnp.maximum(n31, 1e-12)
    zjs = c32 / jnp.maximum(n32, 1e-12)
    zis_ref[:, :] = zis.astype(jnp.bfloat16)
    zjs_ref[:, :] = zjs.astype(jnp.bfloat16)
    pos_ref[:, :] = jnp.sum(zis * zjs, axis=1, keepdims=True)


def _prep(w, w2, w31, w32, theta_t, bm=128):
    n = w.shape[0]
    outs = pl.pallas_call(
        functools.partial(_prep_body, bm=bm),
        grid=(n // bm,),
        in_specs=[pl.BlockSpec((bm, n), lambda i: (i, 0))] * 5,
        out_specs=[pl.BlockSpec((bm, n), lambda i: (i, 0)),
                   pl.BlockSpec((bm, n), lambda i: (i, 0)),
                   pl.BlockSpec((bm, n), lambda i: (i, 0)),
                   pl.BlockSpec((bm, 1), lambda i: (i, 0)),
                   pl.BlockSpec((bm, 1), lambda i: (i, 0)),
                   pl.BlockSpec((bm, 1), lambda i: (i, 0)),
                   pl.BlockSpec((bm, 1), lambda i: (i, 0)),
                   pl.BlockSpec((bm, 1), lambda i: (i, 0)),
                   pl.BlockSpec((1, 1), lambda i: (0, 0),
                                memory_space=pltpu.SMEM),
                   pl.BlockSpec((1, 1), lambda i: (0, 0),
                                memory_space=pltpu.SMEM),
                   pl.BlockSpec((1, 1), lambda i: (0, 0),
                                memory_space=pltpu.SMEM)],
        out_shape=[jax.ShapeDtypeStruct((n, n), jnp.float32),
                   jax.ShapeDtypeStruct((n, n), jnp.bfloat16),
                   jax.ShapeDtypeStruct((n, n), jnp.bfloat16),
                   jax.ShapeDtypeStruct((n, 1), jnp.float32),
                   jax.ShapeDtypeStruct((n, 1), jnp.float32),
                   jax.ShapeDtypeStruct((n, 1), jnp.float32),
                   jax.ShapeDtypeStruct((n, 1), jnp.float32),
                   jax.ShapeDtypeStruct((n, 1), jnp.float32),
                   jax.ShapeDtypeStruct((1, 1), jnp.float32),
                   jax.ShapeDtypeStruct((1, 1), jnp.float32),
                   jax.ShapeDtypeStruct((1, 1), jnp.float32)],
    )(w, w2, w31, w32, theta_t)
    (c3, zis, zjs, pos, dw, dw2, dw31, dw32, creg, cq, cons) = outs
    return (c3, zis, zjs, pos, dw, dw2, dw31, dw32,
            creg[0, 0], cq[0, 0], cons[0, 0])


_DN = (((1,), (1,)), ((), ()))


def _gram_body(zis_i, zjs_i, zis_j, zjs_j, y_i, yt_j, pos_ref, post_ref,
               cl_ref, a1, a2, a3, neg1, neg2, *, nmi, nmj, nk, bm, bn):
    i = pl.program_id(0)
    j = pl.program_id(1)
    k = pl.program_id(2)

    @pl.when((i == 0) & (j == 0) & (k == 0))
    def _():
        neg1[:, :] = jnp.zeros_like(neg1)
        neg2[:, :] = jnp.zeros_like(neg2)

    @pl.when(k == 0)
    def _():
        a1[:, :] = jnp.zeros_like(a1)
        a2[:, :] = jnp.zeros_like(a2)
        a3[:, :] = jnp.zeros_like(a3)

    a1[:, :] += lax.dot_general(zis_i[:, :], zjs_j[:, :], _DN,
                                preferred_element_type=jnp.float32)
    a2[:, :] += lax.dot_general(zis_i[:, :], zis_j[:, :], _DN,
                                preferred_element_type=jnp.float32)
    a3[:, :] += lax.dot_general(zjs_i[:, :], zjs_j[:, :], _DN,
                                preferred_element_type=jnp.float32)

    @pl.when(k == nk - 1)
    def _():
        nm = (y_i[:, :] != yt_j[:, :]).astype(jnp.float32)
        e1 = jnp.exp(a1[:, :]) * nm
        e2 = jnp.exp(a2[:, :]) * nm
        e3 = jnp.exp(a3[:, :]) * nm
        neg1[pl.ds(i * bm, bm), :] += jnp.sum(e1 + e2, axis=1, keepdims=True)
        neg2[:, pl.ds(j * bn, bn)] += jnp.sum(e1 + e3, axis=0)[None, :]

        @pl.when((i == nmi - 1) & (j == nmj - 1))
        def _():
            p = pos_ref[:, :]
            pt = post_ref[:, :]
            cl_ref[0, 0] = (jnp.sum(jnp.log(jnp.exp(p) + neg1[:, :]) - p)
                            + jnp.sum(jnp.log(jnp.exp(pt) + neg2[:, :]) - pt))


def _gram(zis, zjs, y, yt, pos, post, bm=1024, bn=1024, bk=1024):
    n = zis.shape[0]
    nmi, nmj, nk = n // bm, n // bn, n // bk
    cl = pl.pallas_call(
        functools.partial(_gram_body, nmi=nmi, nmj=nmj, nk=nk, bm=bm, bn=bn),
        grid=(nmi, nmj, nk),
        in_specs=[pl.BlockSpec((bm, bk), lambda i, j, k: (i, k)),
                  pl.BlockSpec((bm, bk), lambda i, j, k: (i, k)),
                  pl.BlockSpec((bn, bk), lambda i, j, k: (j, k)),
                  pl.BlockSpec((bn, bk), lambda i, j, k: (j, k)),
                  pl.BlockSpec((bm, 1), lambda i, j, k: (i, 0)),
                  pl.BlockSpec((1, bn), lambda i, j, k: (0, j)),
                  pl.BlockSpec((n, 1), lambda i, j, k: (0, 0)),
                  pl.BlockSpec((1, n), lambda i, j, k: (0, 0))],
        out_specs=pl.BlockSpec((1, 1), lambda i, j, k: (0, 0),
                               memory_space=pltpu.SMEM),
        out_shape=jax.ShapeDtypeStruct((1, 1), jnp.float32),
        scratch_shapes=[pltpu.VMEM((bm, bn), jnp.float32),
                        pltpu.VMEM((bm, bn), jnp.float32),
                        pltpu.VMEM((bm, bn), jnp.float32),
                        pltpu.VMEM((n, 1), jnp.float32),
                        pltpu.VMEM((1, n), jnp.float32)],
    )(zis, zis, zjs, zjs, y, yt, pos, post)
    return cl[0, 0]


def _edge_dots(h, s, r):
    n, d = h.shape
    e = s.shape[0]
    info = plsc.get_sparse_core_info()
    nw = info.num_cores * info.num_subcores
    per_w = e // nw
    ch = 128
    nch = per_w // ch
    mesh = plsc.VectorSubcoreMesh(core_axis_name="c", subcore_axis_name="s")

    def body(h_hbm, s_hbm, r_hbm, out_hbm, sidx, ridx, arow, brow, ovec,
             sem1, sem2):
        wid = lax.axis_index("s") * info.num_cores + lax.axis_index("c")

        def chunk(c, carry):
            base = wid * per_w + c * ch
            pltpu.sync_copy(s_hbm.at[pl.ds(base, ch)], sidx)
            pltpu.sync_copy(r_hbm.at[pl.ds(base, ch)], ridx)
            cp1 = pltpu.async_copy(h_hbm.at[sidx], arow, sem1)
            cp2 = pltpu.async_copy(h_hbm.at[ridx], brow, sem2)
            cp1.wait()
            cp2.wait()

            def edge(eo, cc):
                for sub in range(8):
                    ei = eo * 8 + sub
                    acc = arow[ei, pl.ds(0, 16)] * brow[ei, pl.ds(0, 16)]
                    for g in range(1, d // 16):
                        acc = acc + (arow[ei, pl.ds(g * 16, 16)]
                                     * brow[ei, pl.ds(g * 16, 16)])
                    ovec[eo, pl.ds(sub * 16, 16)] = acc
                return cc

            lax.fori_loop(0, ch // 8, edge, 0)
            obase = pl.multiple_of(base // 8, 8)
            pltpu.sync_copy(ovec, out_hbm.at[pl.ds(obase, ch // 8)])
            return carry

        lax.fori_loop(0, nch, chunk, 0)

    return pl.kernel(
        body,
        out_type=jax.ShapeDtypeStruct((e // 8, 128), jnp.float32),
        mesh=mesh,
        scratch_types=[pltpu.VMEM((ch,), jnp.int32),
                       pltpu.VMEM((ch,), jnp.int32),
                       pltpu.VMEM((ch, d), jnp.float32),
                       pltpu.VMEM((ch, d), jnp.float32),
                       pltpu.VMEM((ch // 8, 128), jnp.float32),
                       pltpu.SemaphoreType.DMA,
                       pltpu.SemaphoreType.DMA],
    )(h, s, r)


def _st_body(d1, d2, d3, d4, o_ref):
    i = pl.program_id(0)

    @pl.when(i == 0)
    def _():
        o_ref[0, 0] = 0.0

    seg = (lax.broadcasted_iota(jnp.int32, (128, 128), 0) // 16
           == lax.broadcasted_iota(jnp.int32, (128, 128), 1)
           ).astype(jnp.float32)
    colmask = lax.broadcasted_iota(jnp.int32, d1.shape, 1) < 8
    tot = 0.0
    for dref in (d1, d2, d3, d4):
        dot = jnp.dot(dref[:, :], seg, preferred_element_type=jnp.float32)
        sp = jnp.maximum(-dot, 0.0) + jnp.log(1.0 + jnp.exp(-jnp.abs(dot)))
        tot += jnp.sum(jnp.where(colmask, sp, 0.0))
    o_ref[0, 0] += tot


def _st_reduce(d1, d2, d3, d4, be=4096):
    e8 = d1.shape[0]
    out = pl.pallas_call(
        _st_body,
        grid=(e8 // be,),
        in_specs=[pl.BlockSpec((be, 128), lambda i: (i, 0))] * 4,
        out_specs=pl.BlockSpec((1, 1), lambda i: (0, 0),
                               memory_space=pltpu.SMEM),
        out_shape=jax.ShapeDtypeStruct((1, 1), jnp.float32),
    )(d1, d2, d3, d4)
    return out[0, 0]


def kernel(X, A, S, R, X2, A2, S2, R2, y_pred, Theta,
           weight, weight2, weight31, weight32,
           W11, W12, Wd11, Wd12, W21, W22, Wd21, Wd22, W31, Wd31):
    n, f1 = X.shape
    f2 = X2.shape[1]
    h2 = W12.shape[1]
    h3 = W31.shape[1]

    pad = 128 - h3
    W31p = jnp.pad(W31, ((0, 0), (0, pad)))
    Wd31p = jnp.pad(Wd31, ((0, pad), (0, 0)))

    E1, Ahl = _amm_elu_split(A, _mm(X, W11))
    H = _amm_elu(Ahl, _mm(E1, W12))
    E2, A2hl = _amm_elu_split(A2, _mm(X2, W21))
    Hb = _amm_elu(A2hl, _mm(E2, W22))

    d1 = _edge_dots(H, S, R)
    d2 = _edge_dots(Hb, S2, R2)

    (c3, zis, zjs, pos, dw, dw2, dw31, dw32, creg, cq, cons) = _prep(
        weight, weight2, weight31, weight32, Theta.T)

    h1dim = Wd11.shape[1]
    HC, se1, HC2, se2 = _coef_mm2(weight, H, dw, weight2, Hb, dw2)
    M1 = _amm_elu(Ahl, _mm2(HC, Wd11, H, W31p))
    dec1 = M1[:, :h1dim]
    H31 = M1[:, h1dim:]
    d3 = _edge_dots(H31, S, R)
    M2 = _amm_elu(A2hl, _mm2(HC2, Wd21, Hb, W31p))
    dec2 = M2[:, :h1dim]
    H32 = M2[:, h1dim:]
    d4 = _edge_dots(H32, S2, R2)
    HC31, se3, HC32, se4 = _coef_mm2(weight31, H31, dw31, weight32, H32, dw32)
    ft1, ft3 = _amm_elu_ft2(Ahl, _mm2(dec1, Wd12, HC31, Wd31p), X, H)
    ft2, ft4 = _amm_elu_ft2(A2hl, _mm2(dec2, Wd22, HC32, Wd31p), X2, Hb)

    yt = y_pred.reshape(1, n)
    cl_sum = _gram(zis, zjs, y_pred, yt, pos, pos.reshape(1, n))

    st_loss = _st_reduce(d1, d2, d3, d4)

    ft_loss = (ft1 / (n * f1) + ft2 / (n * f2)
               + ft3 / (n * h2) + ft4 / (n * h2))
    se_loss = 0.5 * (se1 / (n * h2) + se2 / (n * h2)
                     + se3 / (n * h3) + se4 / (n * h3))
    cl_loss = cl_sum / (2.0 * n)

    loss = (ft_loss + 0.1 * st_loss + se_loss + 0.1 * creg
            + 0.1 * cl_loss + 0.1 * cq + 0.1 * cons)
    return (loss, ft_loss, st_loss, se_loss, creg, cons, cl_loss, cq, c3)


# speedup vs baseline: 3.1627x; 1.0078x over previous
import functools

import jax
import jax.numpy as jnp
from jax import lax
from jax.experimental import pallas as pl
from jax.experimental.pallas import tpu as pltpu
from jax.experimental.pallas import tpu_sc as plsc


def _split_body(x_ref, hi_ref, lo_ref):
    x = x_ref[:, :]
    hi = x.astype(jnp.bfloat16)
    hi_ref[:, :] = hi
    lo_ref[:, :] = (x - hi.astype(jnp.float32)).astype(jnp.bfloat16)


def _split(x, bm=256):
    m, k = x.shape
    return pl.pallas_call(
        _split_body,
        grid=(m // bm,),
        in_specs=[pl.BlockSpec((bm, k), lambda i: (i, 0))],
        out_specs=[pl.BlockSpec((bm, k), lambda i: (i, 0)),
                   pl.BlockSpec((bm, k), lambda i: (i, 0))],
        out_shape=[jax.ShapeDtypeStruct((m, k), jnp.bfloat16),
                   jax.ShapeDtypeStruct((m, k), jnp.bfloat16)],
    )(x)


def _mm_body(x_ref, w_ref, ph_ref, pl_ref):
    p = jnp.dot(x_ref[:, :], w_ref[:, :], preferred_element_type=jnp.float32)
    ph = p.astype(jnp.bfloat16)
    ph_ref[:, :] = ph
    pl_ref[:, :] = (p - ph.astype(jnp.float32)).astype(jnp.bfloat16)


def _mm(x, w, bm=256):
    m, k = x.shape
    _, n = w.shape
    return pl.pallas_call(
        _mm_body,
        grid=(m // bm,),
        in_specs=[pl.BlockSpec((bm, k), lambda i: (i, 0)),
                  pl.BlockSpec((k, n), lambda i: (0, 0))],
        out_specs=[pl.BlockSpec((bm, n), lambda i: (i, 0)),
                   pl.BlockSpec((bm, n), lambda i: (i, 0))],
        out_shape=[jax.ShapeDtypeStruct((m, n), jnp.bfloat16),
                   jax.ShapeDtypeStruct((m, n), jnp.bfloat16)],
    )(x, w)


def _mm2_body(x1_ref, w1_ref, x2_ref, w2_ref, ph_ref, pl_ref, *, n1):
    p1 = jnp.dot(x1_ref[:, :], w1_ref[:, :], preferred_element_type=jnp.float32)
    p2 = jnp.dot(x2_ref[:, :], w2_ref[:, :], preferred_element_type=jnp.float32)
    h1 = p1.astype(jnp.bfloat16)
    h2 = p2.astype(jnp.bfloat16)
    ph_ref[:, :n1] = h1
    ph_ref[:, n1:] = h2
    pl_ref[:, :n1] = (p1 - h1.astype(jnp.float32)).astype(jnp.bfloat16)
    pl_ref[:, n1:] = (p2 - h2.astype(jnp.float32)).astype(jnp.bfloat16)


def _mm2(x1, w1, x2, w2, bm=256):
    m, k1 = x1.shape
    _, n1 = w1.shape
    _, k2 = x2.shape
    _, n2 = w2.shape
    n = n1 + n2
    return pl.pallas_call(
        functools.partial(_mm2_body, n1=n1),
        grid=(m // bm,),
        in_specs=[pl.BlockSpec((bm, k1), lambda i: (i, 0)),
                  pl.BlockSpec((k1, n1), lambda i: (0, 0)),
                  pl.BlockSpec((bm, k2), lambda i: (i, 0)),
                  pl.BlockSpec((k2, n2), lambda i: (0, 0))],
        out_specs=[pl.BlockSpec((bm, n), lambda i: (i, 0)),
                   pl.BlockSpec((bm, n), lambda i: (i, 0))],
        out_shape=[jax.ShapeDtypeStruct((m, n), jnp.bfloat16),
                   jax.ShapeDtypeStruct((m, n), jnp.bfloat16)],
    )(x1, w1, x2, w2)


def _elu(x):
    return jnp.where(x > 0, x, jnp.exp(x) - 1.0)


def _split_dot(ah, al, ph, pl_):
    acc = jnp.dot(ah, ph, preferred_element_type=jnp.float32)
    acc += jnp.dot(al, ph, preferred_element_type=jnp.float32)
    acc += jnp.dot(ah, pl_, preferred_element_type=jnp.float32)
    return acc


def _amm_elu_body(ah_ref, al_ref, ph_ref, pl_ref, o_ref, acc_ref, *, nk):
    k = pl.program_id(1)

    @pl.when(k == 0)
    def _():
        acc_ref[:, :] = jnp.zeros_like(acc_ref)

    acc_ref[:, :] += _split_dot(ah_ref[:, :], al_ref[:, :],
                                ph_ref[:, :], pl_ref[:, :])

    @pl.when(k == nk - 1)
    def _():
        o_ref[:, :] = _elu(acc_ref[:, :])


def _amm_elu_split_body(a_ref, ph_ref, pl_ref, o_ref, ah_ref, al_ref,
                        acc_ref, *, nk):
    k = pl.program_id(1)

    @pl.when(k == 0)
    def _():
        acc_ref[:, :] = jnp.zeros_like(acc_ref)

    a = a_ref[:, :]
    ah = a.astype(jnp.bfloat16)
    al = (a - ah.astype(jnp.float32)).astype(jnp.bfloat16)
    ah_ref[:, :] = ah
    al_ref[:, :] = al
    acc_ref[:, :] += _split_dot(ah, al, ph_ref[:, :], pl_ref[:, :])

    @pl.when(k == nk - 1)
    def _():
        o_ref[:, :] = _elu(acc_ref[:, :])


def _amm_elu_split(a, phl, bm=512, bk=1024):
    ph, pl_ = phl
    m, kk = a.shape
    _, n = ph.shape
    nk = kk // bk
    o, ah, al = pl.pallas_call(
        functools.partial(_amm_elu_split_body, nk=nk),
        grid=(m // bm, nk),
        in_specs=[pl.BlockSpec((bm, bk), lambda i, k: (i, k)),
                  pl.BlockSpec((bk, n), lambda i, k: (k, 0)),
                  pl.BlockSpec((bk, n), lambda i, k: (k, 0))],
        out_specs=[pl.BlockSpec((bm, n), lambda i, k: (i, 0)),
                   pl.BlockSpec((bm, bk), lambda i, k: (i, k)),
                   pl.BlockSpec((bm, bk), lambda i, k: (i, k))],
        out_shape=[jax.ShapeDtypeStruct((m, n), jnp.float32),
                   jax.ShapeDtypeStruct((m, kk), jnp.bfloat16),
                   jax.ShapeDtypeStruct((m, kk), jnp.bfloat16)],
        scratch_shapes=[pltpu.VMEM((bm, n), jnp.float32)],
    )(a, ph, pl_)
    return o, (ah, al)


def _amm_elu(ahl, phl, bm=512, bk=1024):
    ah, al = ahl
    ph, pl_ = phl
    m, kk = ah.shape
    _, n = ph.shape
    nk = kk // bk
    return pl.pallas_call(
        functools.partial(_amm_elu_body, nk=nk),
        grid=(m // bm, nk),
        in_specs=[pl.BlockSpec((bm, bk), lambda i, k: (i, k)),
                  pl.BlockSpec((bm, bk), lambda i, k: (i, k)),
                  pl.BlockSpec((bk, n), lambda i, k: (k, 0)),
                  pl.BlockSpec((bk, n), lambda i, k: (k, 0))],
        out_specs=pl.BlockSpec((bm, n), lambda i, k: (i, 0)),
        out_shape=jax.ShapeDtypeStruct((m, n), jnp.float32),
        scratch_shapes=[pltpu.VMEM((bm, n), jnp.float32)],
    )(ah, al, ph, pl_)


def _amm_elu2_body(ah_ref, al_ref, ph_ref, pl_ref, o1_ref, o2_ref, acc_ref,
                   *, nk, n1):
    k = pl.program_id(1)

    @pl.when(k == 0)
    def _():
        acc_ref[:, :] = jnp.zeros_like(acc_ref)

    acc_ref[:, :] += _split_dot(ah_ref[:, :], al_ref[:, :],
                                ph_ref[:, :], pl_ref[:, :])

    @pl.when(k == nk - 1)
    def _():
        y = _elu(acc_ref[:, :])
        o1_ref[:, :] = y[:, :n1]
        o2_ref[:, :] = y[:, n1:]


def _amm_elu2(ahl, phl, n1, bm=512, bk=1024):
    ah, al = ahl
    ph, pl_ = phl
    m, kk = ah.shape
    _, n = ph.shape
    nk = kk // bk
    return pl.pallas_call(
        functools.partial(_amm_elu2_body, nk=nk, n1=n1),
        grid=(m // bm, nk),
        in_specs=[pl.BlockSpec((bm, bk), lambda i, k: (i, k)),
                  pl.BlockSpec((bm, bk), lambda i, k: (i, k)),
                  pl.BlockSpec((bk, n), lambda i, k: (k, 0)),
                  pl.BlockSpec((bk, n), lambda i, k: (k, 0))],
        out_specs=[pl.BlockSpec((bm, n1), lambda i, k: (i, 0)),
                   pl.BlockSpec((bm, n - n1), lambda i, k: (i, 0))],
        out_shape=[jax.ShapeDtypeStruct((m, n1), jnp.float32),
                   jax.ShapeDtypeStruct((m, n - n1), jnp.float32)],
        scratch_shapes=[pltpu.VMEM((bm, n), jnp.float32)],
    )(ah, al, ph, pl_)


def _amm_elu_ft_body(ah_ref, al_ref, ph_ref, pl_ref, t_ref, o_ref, acc_ref,
                     *, nk):
    i = pl.program_id(0)
    k = pl.program_id(1)

    @pl.when((i == 0) & (k == 0))
    def _():
        o_ref[0, 0] = 0.0

    @pl.when(k == 0)
    def _():
        acc_ref[:, :] = jnp.zeros_like(acc_ref)

    acc_ref[:, :] += _split_dot(ah_ref[:, :], al_ref[:, :],
                                ph_ref[:, :], pl_ref[:, :])

    @pl.when(k == nk - 1)
    def _():
        d = t_ref[:, :] - _elu(acc_ref[:, :])
        o_ref[0, 0] += jnp.sum(d * d)


def _amm_elu_ft2_body(ah_ref, al_ref, ph_ref, pl_ref, t1_ref, t2_ref,
                      o1_ref, o2_ref, acc_ref, *, nk, n1):
    i = pl.program_id(0)
    k = pl.program_id(1)

    @pl.when((i == 0) & (k == 0))
    def _():
        o1_ref[0, 0] = 0.0
        o2_ref[0, 0] = 0.0

    @pl.when(k == 0)
    def _():
        acc_ref[:, :] = jnp.zeros_like(acc_ref)

    acc_ref[:, :] += _split_dot(ah_ref[:, :], al_ref[:, :],
                                ph_ref[:, :], pl_ref[:, :])

    @pl.when(k == nk - 1)
    def _():
        y = _elu(acc_ref[:, :])
        d1 = t1_ref[:, :] - y[:, :n1]
        d2 = t2_ref[:, :] - y[:, n1:]
        o1_ref[0, 0] += jnp.sum(d1 * d1)
        o2_ref[0, 0] += jnp.sum(d2 * d2)


def _amm_elu_ft2(ahl, phl, t1, t2, bm=512, bk=1024):
    ah, al = ahl
    ph, pl_ = phl
    m, kk = ah.shape
    _, n = ph.shape
    n1 = t1.shape[1]
    nk = kk // bk
    o1, o2 = pl.pallas_call(
        functools.partial(_amm_elu_ft2_body, nk=nk, n1=n1),
        grid=(m // bm, nk),
        in_specs=[pl.BlockSpec((bm, bk), lambda i, k: (i, k)),
                  pl.BlockSpec((bm, bk), lambda i, k: (i, k)),
                  pl.BlockSpec((bk, n), lambda i, k: (k, 0)),
                  pl.BlockSpec((bk, n), lambda i, k: (k, 0)),
                  pl.BlockSpec((bm, n1), lambda i, k: (i, 0)),
                  pl.BlockSpec((bm, n - n1), lambda i, k: (i, 0))],
        out_specs=[pl.BlockSpec((1, 1), lambda i, k: (0, 0),
                                memory_space=pltpu.SMEM),
                   pl.BlockSpec((1, 1), lambda i, k: (0, 0),
                                memory_space=pltpu.SMEM)],
        out_shape=[jax.ShapeDtypeStruct((1, 1), jnp.float32),
                   jax.ShapeDtypeStruct((1, 1), jnp.float32)],
        scratch_shapes=[pltpu.VMEM((bm, n), jnp.float32)],
    )(ah, al, ph, pl_, t1, t2)
    return o1[0, 0], o2[0, 0]


def _amm_elu_ft(ahl, phl, t, bm=512, bk=1024):
    ah, al = ahl
    ph, pl_ = phl
    m, kk = ah.shape
    _, n = ph.shape
    nk = kk // bk
    out = pl.pallas_call(
        functools.partial(_amm_elu_ft_body, nk=nk),
        grid=(m // bm, nk),
        in_specs=[pl.BlockSpec((bm, bk), lambda i, k: (i, k)),
                  pl.BlockSpec((bm, bk), lambda i, k: (i, k)),
                  pl.BlockSpec((bk, n), lambda i, k: (k, 0)),
                  pl.BlockSpec((bk, n), lambda i, k: (k, 0)),
                  pl.BlockSpec((bm, n), lambda i, k: (i, 0))],
        out_specs=pl.BlockSpec((1, 1), lambda i, k: (0, 0),
                               memory_space=pltpu.SMEM),
        out_shape=jax.ShapeDtypeStruct((1, 1), jnp.float32),
        scratch_shapes=[pltpu.VMEM((bm, n), jnp.float32)],
    )(ah, al, ph, pl_, t)
    return out[0, 0]


def _coef_pass(w_ref, h_ref, acc_ref):
    w = w_ref[:, :]
    wh = w.astype(jnp.bfloat16)
    wl = (w - wh.astype(jnp.float32)).astype(jnp.bfloat16)
    h = h_ref[:, :]
    hh = h.astype(jnp.bfloat16)
    hl = (h - hh.astype(jnp.float32)).astype(jnp.bfloat16)
    acc_ref[:, :] += _split_dot(wh, wl, hh, hl)


def _coef_mm2_body(w_ref, h_ref, hi_ref, dw_ref,
                   wb_ref, hb_ref, hib_ref, dwb_ref,
                   o_ref, se_ref, ob_ref, seb_ref, acc_ref, accb_ref, *, nk):
    i = pl.program_id(0)
    k = pl.program_id(1)

    @pl.when((i == 0) & (k == 0))
    def _():
        se_ref[0, 0] = 0.0
        seb_ref[0, 0] = 0.0

    @pl.when(k == 0)
    def _():
        acc_ref[:, :] = jnp.zeros_like(acc_ref)
        accb_ref[:, :] = jnp.zeros_like(accb_ref)

    _coef_pass(w_ref, h_ref, acc_ref)
    _coef_pass(wb_ref, hb_ref, accb_ref)

    @pl.when(k == nk - 1)
    def _():
        hi = hi_ref[:, :]
        hc = acc_ref[:, :] - dw_ref[:, :] * hi
        o_ref[:, :] = hc
        d = hi - hc
        se_ref[0, 0] += jnp.sum(d * d)
        hib = hib_ref[:, :]
        hcb = accb_ref[:, :] - dwb_ref[:, :] * hib
        ob_ref[:, :] = hcb
        db = hib - hcb
        seb_ref[0, 0] += jnp.sum(db * db)


def _coef_mm2(w, h, dw, wb, hb, dwb, bm=256, bk=1024):
    m, kk = w.shape
    _, n = h.shape
    nk = kk // bk
    hc, se, hcb, seb = pl.pallas_call(
        functools.partial(_coef_mm2_body, nk=nk),
        grid=(m // bm, nk),
        in_specs=[pl.BlockSpec((bm, bk), lambda i, k: (i, k)),
                  pl.BlockSpec((bk, n), lambda i, k: (k, 0)),
                  pl.BlockSpec((bm, n), lambda i, k: (i, 0)),
                  pl.BlockSpec((bm, 1), lambda i, k: (i, 0)),
                  pl.BlockSpec((bm, bk), lambda i, k: (i, k)),
                  pl.BlockSpec((bk, n), lambda i, k: (k, 0)),
                  pl.BlockSpec((bm, n), lambda i, k: (i, 0)),
                  pl.BlockSpec((bm, 1), lambda i, k: (i, 0))],
        out_specs=[pl.BlockSpec((bm, n), lambda i, k: (i, 0)),
                   pl.BlockSpec((1, 1), lambda i, k: (0, 0),
                                memory_space=pltpu.SMEM),
                   pl.BlockSpec((bm, n), lambda i, k: (i, 0)),
                   pl.BlockSpec((1, 1), lambda i, k: (0, 0),
                                memory_space=pltpu.SMEM)],
        out_shape=[jax.ShapeDtypeStruct((m, n), jnp.float32),
                   jax.ShapeDtypeStruct((1, 1), jnp.float32),
                   jax.ShapeDtypeStruct((m, n), jnp.float32),
                   jax.ShapeDtypeStruct((1, 1), jnp.float32)],
        scratch_shapes=[pltpu.VMEM((bm, n), jnp.float32),
                        pltpu.VMEM((bm, n), jnp.float32)],
    )(w, h, h, dw, wb, hb, hb, dwb)
    return hc, se[0, 0], hcb, seb[0, 0]


def _prep_body(w_ref, w2_ref, w31_ref, w32_ref, tt_ref,
               c3_ref, zis_ref, zjs_ref, pos_ref,
               dw_ref, dw2_ref, dw31_ref, dw32_ref,
               creg_ref, cq_ref, cons_ref, *, bm):
    i = pl.program_id(0)

    @pl.when(i == 0)
    def _():
        creg_ref[0, 0] = 0.0
        cq_ref[0, 0] = 0.0
        cons_ref[0, 0] = 0.0

    n = w_ref.shape[1]
    rows = lax.broadcasted_iota(jnp.int32, (bm, n), 0) + i * bm
    cols = lax.broadcasted_iota(jnp.int32, (bm, n), 1)
    diag = rows == cols
    c = jnp.where(diag, 0.0, w_ref[:, :])
    c2 = jnp.where(diag, 0.0, w2_ref[:, :])
    c31 = jnp.where(diag, 0.0, w31_ref[:, :])
    c32 = jnp.where(diag, 0.0, w32_ref[:, :])
    dw_ref[:, :] = jnp.sum(jnp.where(diag, w_ref[:, :], 0.0),
                           axis=1, keepdims=True)
    dw2_ref[:, :] = jnp.sum(jnp.where(diag, w2_ref[:, :], 0.0),
                            axis=1, keepdims=True)
    dw31_ref[:, :] = jnp.sum(jnp.where(diag, w31_ref[:, :], 0.0),
                             axis=1, keepdims=True)
    dw32_ref[:, :] = jnp.sum(jnp.where(diag, w32_ref[:, :], 0.0),
                             axis=1, keepdims=True)
    c3 = 0.7 * c31 + 0.3 * c32
    c3_ref[:, :] = c3
    creg_ref[0, 0] += (jnp.sum(jnp.abs(c)) + jnp.sum(jnp.abs(c2))
                       + jnp.sum(jnp.abs(c31)) + jnp.sum(jnp.abs(c32)))
    cq_ref[0, 0] += jnp.sum(jnp.abs(c3 * tt_ref[:, :]))
    cons_ref[0, 0] += jnp.sum((c3 - c) ** 2) + jnp.sum((c3 - c2) ** 2)
    n31 = jnp.sqrt(jnp.sum(c31 * c31, axis=1, keepdims=True))
    n32 = jnp.sqrt(jnp.sum(c32 * c32, axis=1, keepdims=True))
    zis = c31 / jnp.maximum(n31, 1e-12)
    zjs = c32 / jnp.maximum(n32, 1e-12)
    zis_ref[:, :] = zis.astype(jnp.bfloat16)
    zjs_ref[:, :] = zjs.astype(jnp.bfloat16)
    pos_ref[:, :] = jnp.sum(zis * zjs, axis=1, keepdims=True)


def _prep(w, w2, w31, w32, theta_t, bm=128):
    n = w.shape[0]
    outs = pl.pallas_call(
        functools.partial(_prep_body, bm=bm),
        grid=(n // bm,),
        in_specs=[pl.BlockSpec((bm, n), lambda i: (i, 0))] * 5,
        out_specs=[pl.BlockSpec((bm, n), lambda i: (i, 0)),
                   pl.BlockSpec((bm, n), lambda i: (i, 0)),
                   pl.BlockSpec((bm, n), lambda i: (i, 0)),
                   pl.BlockSpec((bm, 1), lambda i: (i, 0)),
                   pl.BlockSpec((bm, 1), lambda i: (i, 0)),
                   pl.BlockSpec((bm, 1), lambda i: (i, 0)),
                   pl.BlockSpec((bm, 1), lambda i: (i, 0)),
                   pl.BlockSpec((bm, 1), lambda i: (i, 0)),
                   pl.BlockSpec((1, 1), lambda i: (0, 0),
                                memory_space=pltpu.SMEM),
                   pl.BlockSpec((1, 1), lambda i: (0, 0),
                                memory_space=pltpu.SMEM),
                   pl.BlockSpec((1, 1), lambda i: (0, 0),
                                memory_space=pltpu.SMEM)],
        out_shape=[jax.ShapeDtypeStruct((n, n), jnp.float32),
                   jax.ShapeDtypeStruct((n, n), jnp.bfloat16),
                   jax.ShapeDtypeStruct((n, n), jnp.bfloat16),
                   jax.ShapeDtypeStruct((n, 1), jnp.float32),
                   jax.ShapeDtypeStruct((n, 1), jnp.float32),
                   jax.ShapeDtypeStruct((n, 1), jnp.float32),
                   jax.ShapeDtypeStruct((n, 1), jnp.float32),
                   jax.ShapeDtypeStruct((n, 1), jnp.float32),
                   jax.ShapeDtypeStruct((1, 1), jnp.float32),
                   jax.ShapeDtypeStruct((1, 1), jnp.float32),
                   jax.ShapeDtypeStruct((1, 1), jnp.float32)],
    )(w, w2, w31, w32, theta_t)
    (c3, zis, zjs, pos, dw, dw2, dw31, dw32, creg, cq, cons) = outs
    return (c3, zis, zjs, pos, dw, dw2, dw31, dw32,
            creg[0, 0], cq[0, 0], cons[0, 0])


_DN = (((1,), (1,)), ((), ()))


def _gram_body(zis_i, zjs_i, zis_j, zjs_j, y_i, yt_j, pos_ref, post_ref,
               cl_ref, a1, a2, a3, neg1, neg2, *, nmi, nmj, nk, bm, bn):
    i = pl.program_id(0)
    j = pl.program_id(1)
    k = pl.program_id(2)

    @pl.when((i == 0) & (j == 0) & (k == 0))
    def _():
        neg1[:, :] = jnp.zeros_like(neg1)
        neg2[:, :] = jnp.zeros_like(neg2)

    @pl.when(k == 0)
    def _():
        a1[:, :] = jnp.zeros_like(a1)
        a2[:, :] = jnp.zeros_like(a2)
        a3[:, :] = jnp.zeros_like(a3)

    a1[:, :] += lax.dot_general(zis_i[:, :], zjs_j[:, :], _DN,
                                preferred_element_type=jnp.float32)
    a2[:, :] += lax.dot_general(zis_i[:, :], zis_j[:, :], _DN,
                                preferred_element_type=jnp.float32)
    a3[:, :] += lax.dot_general(zjs_i[:, :], zjs_j[:, :], _DN,
                                preferred_element_type=jnp.float32)

    @pl.when(k == nk - 1)
    def _():
        nm = (y_i[:, :] != yt_j[:, :]).astype(jnp.float32)
        e1 = jnp.exp(a1[:, :]) * nm
        e2 = jnp.exp(a2[:, :]) * nm
        e3 = jnp.exp(a3[:, :]) * nm
        neg1[pl.ds(i * bm, bm), :] += jnp.sum(e1 + e2, axis=1, keepdims=True)
        neg2[:, pl.ds(j * bn, bn)] += jnp.sum(e1 + e3, axis=0)[None, :]

        @pl.when((i == nmi - 1) & (j == nmj - 1))
        def _():
            p = pos_ref[:, :]
            pt = post_ref[:, :]
            cl_ref[0, 0] = (jnp.sum(jnp.log(jnp.exp(p) + neg1[:, :]) - p)
                            + jnp.sum(jnp.log(jnp.exp(pt) + neg2[:, :]) - pt))


def _gram(zis, zjs, y, yt, pos, post, bm=1024, bn=1024, bk=2048):
    n = zis.shape[0]
    nmi, nmj, nk = n // bm, n // bn, n // bk
    cl = pl.pallas_call(
        functools.partial(_gram_body, nmi=nmi, nmj=nmj, nk=nk, bm=bm, bn=bn),
        grid=(nmi, nmj, nk),
        in_specs=[pl.BlockSpec((bm, bk), lambda i, j, k: (i, k)),
                  pl.BlockSpec((bm, bk), lambda i, j, k: (i, k)),
                  pl.BlockSpec((bn, bk), lambda i, j, k: (j, k)),
                  pl.BlockSpec((bn, bk), lambda i, j, k: (j, k)),
                  pl.BlockSpec((bm, 1), lambda i, j, k: (i, 0)),
                  pl.BlockSpec((1, bn), lambda i, j, k: (0, j)),
                  pl.BlockSpec((n, 1), lambda i, j, k: (0, 0)),
                  pl.BlockSpec((1, n), lambda i, j, k: (0, 0))],
        out_specs=pl.BlockSpec((1, 1), lambda i, j, k: (0, 0),
                               memory_space=pltpu.SMEM),
        out_shape=jax.ShapeDtypeStruct((1, 1), jnp.float32),
        scratch_shapes=[pltpu.VMEM((bm, bn), jnp.float32),
                        pltpu.VMEM((bm, bn), jnp.float32),
                        pltpu.VMEM((bm, bn), jnp.float32),
                        pltpu.VMEM((n, 1), jnp.float32),
                        pltpu.VMEM((1, n), jnp.float32)],
    )(zis, zis, zjs, zjs, y, yt, pos, post)
    return cl[0, 0]


def _edge_dots(h, s, r):
    n, d = h.shape
    e = s.shape[0]
    info = plsc.get_sparse_core_info()
    nw = info.num_cores * info.num_subcores
    per_w = e // nw
    ch = 128
    nch = per_w // ch
    mesh = plsc.VectorSubcoreMesh(core_axis_name="c", subcore_axis_name="s")

    def body(h_hbm, s_hbm, r_hbm, out_hbm, sidx, ridx, arow, brow, ovec,
             sem1, sem2):
        wid = lax.axis_index("s") * info.num_cores + lax.axis_index("c")

        def chunk(c, carry):
            base = wid * per_w + c * ch
            pltpu.sync_copy(s_hbm.at[pl.ds(base, ch)], sidx)
            pltpu.sync_copy(r_hbm.at[pl.ds(base, ch)], ridx)
            cp1 = pltpu.async_copy(h_hbm.at[sidx], arow, sem1)
            cp2 = pltpu.async_copy(h_hbm.at[ridx], brow, sem2)
            cp1.wait()
            cp2.wait()

            def edge(eo, cc):
                for sub in range(8):
                    ei = eo * 8 + sub
                    acc = arow[ei, pl.ds(0, 16)] * brow[ei, pl.ds(0, 16)]
                    for g in range(1, d // 16):
                        acc = acc + (arow[ei, pl.ds(g * 16, 16)]
                                     * brow[ei, pl.ds(g * 16, 16)])
                    ovec[eo, pl.ds(sub * 16, 16)] = acc
                return cc

            lax.fori_loop(0, ch // 8, edge, 0)
            obase = pl.multiple_of(base // 8, 8)
            pltpu.sync_copy(ovec, out_hbm.at[pl.ds(obase, ch // 8)])
            return carry

        lax.fori_loop(0, nch, chunk, 0)

    return pl.kernel(
        body,
        out_type=jax.ShapeDtypeStruct((e // 8, 128), jnp.float32),
        mesh=mesh,
        scratch_types=[pltpu.VMEM((ch,), jnp.int32),
                       pltpu.VMEM((ch,), jnp.int32),
                       pltpu.VMEM((ch, d), jnp.float32),
                       pltpu.VMEM((ch, d), jnp.float32),
                       pltpu.VMEM((ch // 8, 128), jnp.float32),
                       pltpu.SemaphoreType.DMA,
                       pltpu.SemaphoreType.DMA],
    )(h, s, r)


def _st_body(d1, d2, d3, d4, o_ref):
    i = pl.program_id(0)

    @pl.when(i == 0)
    def _():
        o_ref[0, 0] = 0.0

    seg = (lax.broadcasted_iota(jnp.int32, (128, 128), 0) // 16
           == lax.broadcasted_iota(jnp.int32, (128, 128), 1)
           ).astype(jnp.float32)
    colmask = lax.broadcasted_iota(jnp.int32, d1.shape, 1) < 8
    tot = 0.0
    for dref in (d1, d2, d3, d4):
        dot = jnp.dot(dref[:, :], seg, preferred_element_type=jnp.float32)
        sp = jnp.maximum(-dot, 0.0) + jnp.log(1.0 + jnp.exp(-jnp.abs(dot)))
        tot += jnp.sum(jnp.where(colmask, sp, 0.0))
    o_ref[0, 0] += tot


def _st_reduce(d1, d2, d3, d4, be=4096):
    e8 = d1.shape[0]
    out = pl.pallas_call(
        _st_body,
        grid=(e8 // be,),
        in_specs=[pl.BlockSpec((be, 128), lambda i: (i, 0))] * 4,
        out_specs=pl.BlockSpec((1, 1), lambda i: (0, 0),
                               memory_space=pltpu.SMEM),
        out_shape=jax.ShapeDtypeStruct((1, 1), jnp.float32),
    )(d1, d2, d3, d4)
    return out[0, 0]


def kernel(X, A, S, R, X2, A2, S2, R2, y_pred, Theta,
           weight, weight2, weight31, weight32,
           W11, W12, Wd11, Wd12, W21, W22, Wd21, Wd22, W31, Wd31):
    n, f1 = X.shape
    f2 = X2.shape[1]
    h2 = W12.shape[1]
    h3 = W31.shape[1]

    pad = 128 - h3
    W31p = jnp.pad(W31, ((0, 0), (0, pad)))
    Wd31p = jnp.pad(Wd31, ((0, pad), (0, 0)))

    E1, Ahl = _amm_elu_split(A, _mm(X, W11))
    H = _amm_elu(Ahl, _mm(E1, W12))
    E2, A2hl = _amm_elu_split(A2, _mm(X2, W21))
    Hb = _amm_elu(A2hl, _mm(E2, W22))

    d1 = _edge_dots(H, S, R)
    d2 = _edge_dots(Hb, S2, R2)

    (c3, zis, zjs, pos, dw, dw2, dw31, dw32, creg, cq, cons) = _prep(
        weight, weight2, weight31, weight32, Theta.T)

    h1dim = Wd11.shape[1]
    HC, se1, HC2, se2 = _coef_mm2(weight, H, dw, weight2, Hb, dw2)
    dec1, H31 = _amm_elu2(Ahl, _mm2(HC, Wd11, H, W31p), h1dim)
    d3 = _edge_dots(H31, S, R)
    dec2, H32 = _amm_elu2(A2hl, _mm2(HC2, Wd21, Hb, W31p), h1dim)
    d4 = _edge_dots(H32, S2, R2)
    HC31, se3, HC32, se4 = _coef_mm2(weight31, H31, dw31, weight32, H32, dw32)
    ft1, ft3 = _amm_elu_ft2(Ahl, _mm2(dec1, Wd12, HC31, Wd31p), X, H)
    ft2, ft4 = _amm_elu_ft2(A2hl, _mm2(dec2, Wd22, HC32, Wd31p), X2, Hb)

    yt = y_pred.reshape(1, n)
    cl_sum = _gram(zis, zjs, y_pred, yt, pos, pos.reshape(1, n))

    st_loss = _st_reduce(d1, d2, d3, d4)

    ft_loss = (ft1 / (n * f1) + ft2 / (n * f2)
               + ft3 / (n * h2) + ft4 / (n * h2))
    se_loss = 0.5 * (se1 / (n * h2) + se2 / (n * h2)
                     + se3 / (n * h3) + se4 / (n * h3))
    cl_loss = cl_sum / (2.0 * n)

    loss = (ft_loss + 0.1 * st_loss + se_loss + 0.1 * creg
            + 0.1 * cl_loss + 0.1 * cq + 0.1 * cons)
    return (loss, ft_loss, st_loss, se_loss, creg, cons, cl_loss, cq, c3)


# speedup vs baseline: 3.1687x; 1.0019x over previous
import functools

import jax
import jax.numpy as jnp
from jax import lax
from jax.experimental import pallas as pl
from jax.experimental.pallas import tpu as pltpu
from jax.experimental.pallas import tpu_sc as plsc


def _mm_body(x_ref, w_ref, ph_ref, pl_ref):
    p = jnp.dot(x_ref[:, :], w_ref[:, :], preferred_element_type=jnp.float32)
    ph = p.astype(jnp.bfloat16)
    ph_ref[:, :] = ph
    pl_ref[:, :] = (p - ph.astype(jnp.float32)).astype(jnp.bfloat16)


def _mm(x, w, bm=256):
    m, k = x.shape
    _, n = w.shape
    return pl.pallas_call(
        _mm_body,
        grid=(m // bm,),
        in_specs=[pl.BlockSpec((bm, k), lambda i: (i, 0)),
                  pl.BlockSpec((k, n), lambda i: (0, 0))],
        out_specs=[pl.BlockSpec((bm, n), lambda i: (i, 0)),
                   pl.BlockSpec((bm, n), lambda i: (i, 0))],
        out_shape=[jax.ShapeDtypeStruct((m, n), jnp.bfloat16),
                   jax.ShapeDtypeStruct((m, n), jnp.bfloat16)],
    )(x, w)


def _mm2_body(x1_ref, w1_ref, x2_ref, w2_ref, ph_ref, pl_ref, *, n1):
    p1 = jnp.dot(x1_ref[:, :], w1_ref[:, :], preferred_element_type=jnp.float32)
    p2 = jnp.dot(x2_ref[:, :], w2_ref[:, :], preferred_element_type=jnp.float32)
    h1 = p1.astype(jnp.bfloat16)
    h2 = p2.astype(jnp.bfloat16)
    ph_ref[:, :n1] = h1
    ph_ref[:, n1:] = h2
    pl_ref[:, :n1] = (p1 - h1.astype(jnp.float32)).astype(jnp.bfloat16)
    pl_ref[:, n1:] = (p2 - h2.astype(jnp.float32)).astype(jnp.bfloat16)


def _mm2(x1, w1, x2, w2, bm=256):
    m, k1 = x1.shape
    _, n1 = w1.shape
    _, k2 = x2.shape
    _, n2 = w2.shape
    n = n1 + n2
    return pl.pallas_call(
        functools.partial(_mm2_body, n1=n1),
        grid=(m // bm,),
        in_specs=[pl.BlockSpec((bm, k1), lambda i: (i, 0)),
                  pl.BlockSpec((k1, n1), lambda i: (0, 0)),
                  pl.BlockSpec((bm, k2), lambda i: (i, 0)),
                  pl.BlockSpec((k2, n2), lambda i: (0, 0))],
        out_specs=[pl.BlockSpec((bm, n), lambda i: (i, 0)),
                   pl.BlockSpec((bm, n), lambda i: (i, 0))],
        out_shape=[jax.ShapeDtypeStruct((m, n), jnp.bfloat16),
                   jax.ShapeDtypeStruct((m, n), jnp.bfloat16)],
    )(x1, w1, x2, w2)


def _elu(x):
    return jnp.where(x > 0, x, jnp.exp(x) - 1.0)


def _split_dot(ah, al, ph, pl_):
    acc = jnp.dot(ah, ph, preferred_element_type=jnp.float32)
    acc += jnp.dot(al, ph, preferred_element_type=jnp.float32)
    acc += jnp.dot(ah, pl_, preferred_element_type=jnp.float32)
    return acc


def _amm_elu_body(ah_ref, al_ref, ph_ref, pl_ref, o_ref, acc_ref, *, nk):
    k = pl.program_id(1)

    @pl.when(k == 0)
    def _():
        acc_ref[:, :] = jnp.zeros_like(acc_ref)

    acc_ref[:, :] += _split_dot(ah_ref[:, :], al_ref[:, :],
                                ph_ref[:, :], pl_ref[:, :])

    @pl.when(k == nk - 1)
    def _():
        o_ref[:, :] = _elu(acc_ref[:, :])


def _amm_elu_split_body(a_ref, ph_ref, pl_ref, o_ref, ah_ref, al_ref,
                        acc_ref, *, nk):
    k = pl.program_id(1)

    @pl.when(k == 0)
    def _():
        acc_ref[:, :] = jnp.zeros_like(acc_ref)

    a = a_ref[:, :]
    ah = a.astype(jnp.bfloat16)
    al = (a - ah.astype(jnp.float32)).astype(jnp.bfloat16)
    ah_ref[:, :] = ah
    al_ref[:, :] = al
    acc_ref[:, :] += _split_dot(ah, al, ph_ref[:, :], pl_ref[:, :])

    @pl.when(k == nk - 1)
    def _():
        o_ref[:, :] = _elu(acc_ref[:, :])


def _amm_elu_split(a, phl, bm=512, bk=1024):
    ph, pl_ = phl
    m, kk = a.shape
    _, n = ph.shape
    nk = kk // bk
    o, ah, al = pl.pallas_call(
        functools.partial(_amm_elu_split_body, nk=nk),
        grid=(m // bm, nk),
        in_specs=[pl.BlockSpec((bm, bk), lambda i, k: (i, k)),
                  pl.BlockSpec((bk, n), lambda i, k: (k, 0)),
                  pl.BlockSpec((bk, n), lambda i, k: (k, 0))],
        out_specs=[pl.BlockSpec((bm, n), lambda i, k: (i, 0)),
                   pl.BlockSpec((bm, bk), lambda i, k: (i, k)),
                   pl.BlockSpec((bm, bk), lambda i, k: (i, k))],
        out_shape=[jax.ShapeDtypeStruct((m, n), jnp.float32),
                   jax.ShapeDtypeStruct((m, kk), jnp.bfloat16),
                   jax.ShapeDtypeStruct((m, kk), jnp.bfloat16)],
        scratch_shapes=[pltpu.VMEM((bm, n), jnp.float32)],
    )(a, ph, pl_)
    return o, (ah, al)


def _amm_elu(ahl, phl, bm=512, bk=1024):
    ah, al = ahl
    ph, pl_ = phl
    m, kk = ah.shape
    _, n = ph.shape
    nk = kk // bk
    return pl.pallas_call(
        functools.partial(_amm_elu_body, nk=nk),
        grid=(m // bm, nk),
        in_specs=[pl.BlockSpec((bm, bk), lambda i, k: (i, k)),
                  pl.BlockSpec((bm, bk), lambda i, k: (i, k)),
                  pl.BlockSpec((bk, n), lambda i, k: (k, 0)),
                  pl.BlockSpec((bk, n), lambda i, k: (k, 0))],
        out_specs=pl.BlockSpec((bm, n), lambda i, k: (i, 0)),
        out_shape=jax.ShapeDtypeStruct((m, n), jnp.float32),
        scratch_shapes=[pltpu.VMEM((bm, n), jnp.float32)],
    )(ah, al, ph, pl_)


def _amm_elu2_body(ah_ref, al_ref, ph_ref, pl_ref, o1_ref, o2_ref, acc_ref,
                   *, nk, n1):
    k = pl.program_id(1)

    @pl.when(k == 0)
    def _():
        acc_ref[:, :] = jnp.zeros_like(acc_ref)

    acc_ref[:, :] += _split_dot(ah_ref[:, :], al_ref[:, :],
                                ph_ref[:, :], pl_ref[:, :])

    @pl.when(k == nk - 1)
    def _():
        y = _elu(acc_ref[:, :])
        o1_ref[:, :] = y[:, :n1]
        o2_ref[:, :] = y[:, n1:]


def _amm_elu2(ahl, phl, n1, bm=512, bk=1024):
    ah, al = ahl
    ph, pl_ = phl
    m, kk = ah.shape
    _, n = ph.shape
    nk = kk // bk
    return pl.pallas_call(
        functools.partial(_amm_elu2_body, nk=nk, n1=n1),
        grid=(m // bm, nk),
        in_specs=[pl.BlockSpec((bm, bk), lambda i, k: (i, k)),
                  pl.BlockSpec((bm, bk), lambda i, k: (i, k)),
                  pl.BlockSpec((bk, n), lambda i, k: (k, 0)),
                  pl.BlockSpec((bk, n), lambda i, k: (k, 0))],
        out_specs=[pl.BlockSpec((bm, n1), lambda i, k: (i, 0)),
                   pl.BlockSpec((bm, n - n1), lambda i, k: (i, 0))],
        out_shape=[jax.ShapeDtypeStruct((m, n1), jnp.float32),
                   jax.ShapeDtypeStruct((m, n - n1), jnp.float32)],
        scratch_shapes=[pltpu.VMEM((bm, n), jnp.float32)],
    )(ah, al, ph, pl_)


def _amm_elu_ft2_body(ah_ref, al_ref, ph_ref, pl_ref, t1_ref, t2_ref,
                      o1_ref, o2_ref, acc_ref, *, nk, n1):
    i = pl.program_id(0)
    k = pl.program_id(1)

    @pl.when((i == 0) & (k == 0))
    def _():
        o1_ref[0, 0] = 0.0
        o2_ref[0, 0] = 0.0

    @pl.when(k == 0)
    def _():
        acc_ref[:, :] = jnp.zeros_like(acc_ref)

    acc_ref[:, :] += _split_dot(ah_ref[:, :], al_ref[:, :],
                                ph_ref[:, :], pl_ref[:, :])

    @pl.when(k == nk - 1)
    def _():
        y = _elu(acc_ref[:, :])
        d1 = t1_ref[:, :] - y[:, :n1]
        d2 = t2_ref[:, :] - y[:, n1:]
        o1_ref[0, 0] += jnp.sum(d1 * d1)
        o2_ref[0, 0] += jnp.sum(d2 * d2)


def _amm_elu_ft2(ahl, phl, t1, t2, bm=512, bk=1024):
    ah, al = ahl
    ph, pl_ = phl
    m, kk = ah.shape
    _, n = ph.shape
    n1 = t1.shape[1]
    nk = kk // bk
    o1, o2 = pl.pallas_call(
        functools.partial(_amm_elu_ft2_body, nk=nk, n1=n1),
        grid=(m // bm, nk),
        in_specs=[pl.BlockSpec((bm, bk), lambda i, k: (i, k)),
                  pl.BlockSpec((bm, bk), lambda i, k: (i, k)),
                  pl.BlockSpec((bk, n), lambda i, k: (k, 0)),
                  pl.BlockSpec((bk, n), lambda i, k: (k, 0)),
                  pl.BlockSpec((bm, n1), lambda i, k: (i, 0)),
                  pl.BlockSpec((bm, n - n1), lambda i, k: (i, 0))],
        out_specs=[pl.BlockSpec((1, 1), lambda i, k: (0, 0),
                                memory_space=pltpu.SMEM),
                   pl.BlockSpec((1, 1), lambda i, k: (0, 0),
                                memory_space=pltpu.SMEM)],
        out_shape=[jax.ShapeDtypeStruct((1, 1), jnp.float32),
                   jax.ShapeDtypeStruct((1, 1), jnp.float32)],
        scratch_shapes=[pltpu.VMEM((bm, n), jnp.float32)],
    )(ah, al, ph, pl_, t1, t2)
    return o1[0, 0], o2[0, 0]


def _coef_pass(w_ref, h_ref, acc_ref):
    w = w_ref[:, :]
    wh = w.astype(jnp.bfloat16)
    wl = (w - wh.astype(jnp.float32)).astype(jnp.bfloat16)
    h = h_ref[:, :]
    hh = h.astype(jnp.bfloat16)
    hl = (h - hh.astype(jnp.float32)).astype(jnp.bfloat16)
    acc_ref[:, :] += _split_dot(wh, wl, hh, hl)


def _coef_mm2_body(w_ref, h_ref, hi_ref, dw_ref,
                   wb_ref, hb_ref, hib_ref, dwb_ref,
                   o_ref, se_ref, ob_ref, seb_ref, acc_ref, accb_ref, *, nk):
    i = pl.program_id(0)
    k = pl.program_id(1)

    @pl.when((i == 0) & (k == 0))
    def _():
        se_ref[0, 0] = 0.0
        seb_ref[0, 0] = 0.0

    @pl.when(k == 0)
    def _():
        acc_ref[:, :] = jnp.zeros_like(acc_ref)
        accb_ref[:, :] = jnp.zeros_like(accb_ref)

    _coef_pass(w_ref, h_ref, acc_ref)
    _coef_pass(wb_ref, hb_ref, accb_ref)

    @pl.when(k == nk - 1)
    def _():
        hi = hi_ref[:, :]
        hc = acc_ref[:, :] - dw_ref[:, :] * hi
        o_ref[:, :] = hc
        d = hi - hc
        se_ref[0, 0] += jnp.sum(d * d)
        hib = hib_ref[:, :]
        hcb = accb_ref[:, :] - dwb_ref[:, :] * hib
        ob_ref[:, :] = hcb
        db = hib - hcb
        seb_ref[0, 0] += jnp.sum(db * db)


def _coef_mm2(w, h, dw, wb, hb, dwb, bm=256, bk=1024):
    m, kk = w.shape
    _, n = h.shape
    nk = kk // bk
    hc, se, hcb, seb = pl.pallas_call(
        functools.partial(_coef_mm2_body, nk=nk),
        grid=(m // bm, nk),
        in_specs=[pl.BlockSpec((bm, bk), lambda i, k: (i, k)),
                  pl.BlockSpec((bk, n), lambda i, k: (k, 0)),
                  pl.BlockSpec((bm, n), lambda i, k: (i, 0)),
                  pl.BlockSpec((bm, 1), lambda i, k: (i, 0)),
                  pl.BlockSpec((bm, bk), lambda i, k: (i, k)),
                  pl.BlockSpec((bk, n), lambda i, k: (k, 0)),
                  pl.BlockSpec((bm, n), lambda i, k: (i, 0)),
                  pl.BlockSpec((bm, 1), lambda i, k: (i, 0))],
        out_specs=[pl.BlockSpec((bm, n), lambda i, k: (i, 0)),
                   pl.BlockSpec((1, 1), lambda i, k: (0, 0),
                                memory_space=pltpu.SMEM),
                   pl.BlockSpec((bm, n), lambda i, k: (i, 0)),
                   pl.BlockSpec((1, 1), lambda i, k: (0, 0),
                                memory_space=pltpu.SMEM)],
        out_shape=[jax.ShapeDtypeStruct((m, n), jnp.float32),
                   jax.ShapeDtypeStruct((1, 1), jnp.float32),
                   jax.ShapeDtypeStruct((m, n), jnp.float32),
                   jax.ShapeDtypeStruct((1, 1), jnp.float32)],
        scratch_shapes=[pltpu.VMEM((bm, n), jnp.float32),
                        pltpu.VMEM((bm, n), jnp.float32)],
    )(w, h, h, dw, wb, hb, hb, dwb)
    return hc, se[0, 0], hcb, seb[0, 0]


def _prep_body(w_ref, w2_ref, w31_ref, w32_ref, tt_ref,
               c3_ref, zis_ref, zjs_ref, pos_ref,
               dw_ref, dw2_ref, dw31_ref, dw32_ref,
               creg_ref, cq_ref, cons_ref, *, bm):
    i = pl.program_id(0)

    @pl.when(i == 0)
    def _():
        creg_ref[0, 0] = 0.0
        cq_ref[0, 0] = 0.0
        cons_ref[0, 0] = 0.0

    n = w_ref.shape[1]
    rows = lax.broadcasted_iota(jnp.int32, (bm, n), 0) + i * bm
    cols = lax.broadcasted_iota(jnp.int32, (bm, n), 1)
    diag = rows == cols
    c = jnp.where(diag, 0.0, w_ref[:, :])
    c2 = jnp.where(diag, 0.0, w2_ref[:, :])
    c31 = jnp.where(diag, 0.0, w31_ref[:, :])
    c32 = jnp.where(diag, 0.0, w32_ref[:, :])
    dw_ref[:, :] = jnp.sum(jnp.where(diag, w_ref[:, :], 0.0),
                           axis=1, keepdims=True)
    dw2_ref[:, :] = jnp.sum(jnp.where(diag, w2_ref[:, :], 0.0),
                            axis=1, keepdims=True)
    dw31_ref[:, :] = jnp.sum(jnp.where(diag, w31_ref[:, :], 0.0),
                             axis=1, keepdims=True)
    dw32_ref[:, :] = jnp.sum(jnp.where(diag, w32_ref[:, :], 0.0),
                             axis=1, keepdims=True)
    c3 = 0.7 * c31 + 0.3 * c32
    c3_ref[:, :] = c3
    creg_ref[0, 0] += (jnp.sum(jnp.abs(c)) + jnp.sum(jnp.abs(c2))
                       + jnp.sum(jnp.abs(c31)) + jnp.sum(jnp.abs(c32)))
    cq_ref[0, 0] += jnp.sum(jnp.abs(c3 * tt_ref[:, :]))
    cons_ref[0, 0] += jnp.sum((c3 - c) ** 2) + jnp.sum((c3 - c2) ** 2)
    n31 = jnp.sqrt(jnp.sum(c31 * c31, axis=1, keepdims=True))
    n32 = jnp.sqrt(jnp.sum(c32 * c32, axis=1, keepdims=True))
    zis = c31 / jnp.maximum(n31, 1e-12)
    zjs = c32 / jnp.maximum(n32, 1e-12)
    zis_ref[:, :] = zis.astype(jnp.bfloat16)
    zjs_ref[:, :] = zjs.astype(jnp.bfloat16)
    pos_ref[:, :] = jnp.sum(zis * zjs, axis=1, keepdims=True)


def _prep(w, w2, w31, w32, theta_t, bm=128):
    n = w.shape[0]
    outs = pl.pallas_call(
        functools.partial(_prep_body, bm=bm),
        grid=(n // bm,),
        in_specs=[pl.BlockSpec((bm, n), lambda i: (i, 0))] * 5,
        out_specs=[pl.BlockSpec((bm, n), lambda i: (i, 0)),
                   pl.BlockSpec((bm, n), lambda i: (i, 0)),
                   pl.BlockSpec((bm, n), lambda i: (i, 0)),
                   pl.BlockSpec((bm, 1), lambda i: (i, 0)),
                   pl.BlockSpec((bm, 1), lambda i: (i, 0)),
                   pl.BlockSpec((bm, 1), lambda i: (i, 0)),
                   pl.BlockSpec((bm, 1), lambda i: (i, 0)),
                   pl.BlockSpec((bm, 1), lambda i: (i, 0)),
                   pl.BlockSpec((1, 1), lambda i: (0, 0),
                                memory_space=pltpu.SMEM),
                   pl.BlockSpec((1, 1), lambda i: (0, 0),
                                memory_space=pltpu.SMEM),
                   pl.BlockSpec((1, 1), lambda i: (0, 0),
                                memory_space=pltpu.SMEM)],
        out_shape=[jax.ShapeDtypeStruct((n, n), jnp.float32),
                   jax.ShapeDtypeStruct((n, n), jnp.bfloat16),
                   jax.ShapeDtypeStruct((n, n), jnp.bfloat16),
                   jax.ShapeDtypeStruct((n, 1), jnp.float32),
                   jax.ShapeDtypeStruct((n, 1), jnp.float32),
                   jax.ShapeDtypeStruct((n, 1), jnp.float32),
                   jax.ShapeDtypeStruct((n, 1), jnp.float32),
                   jax.ShapeDtypeStruct((n, 1), jnp.float32),
                   jax.ShapeDtypeStruct((1, 1), jnp.float32),
                   jax.ShapeDtypeStruct((1, 1), jnp.float32),
                   jax.ShapeDtypeStruct((1, 1), jnp.float32)],
    )(w, w2, w31, w32, theta_t)
    (c3, zis, zjs, pos, dw, dw2, dw31, dw32, creg, cq, cons) = outs
    return (c3, zis, zjs, pos, dw, dw2, dw31, dw32,
            creg[0, 0], cq[0, 0], cons[0, 0])


_DN = (((1,), (1,)), ((), ()))


def _gram_body(zis_i, zjs_i, zis_j, zjs_j, y_i, yt_j, pos_ref, post_ref,
               cl_ref, a1, a2, a3, neg1, neg2, *, nmi, nmj, nk, bm, bn):
    i = pl.program_id(0)
    j = pl.program_id(1)
    k = pl.program_id(2)

    @pl.when((i == 0) & (j == 0) & (k == 0))
    def _():
        neg1[:, :] = jnp.zeros_like(neg1)
        neg2[:, :] = jnp.zeros_like(neg2)

    @pl.when(k == 0)
    def _():
        a1[:, :] = jnp.zeros_like(a1)
        a2[:, :] = jnp.zeros_like(a2)
        a3[:, :] = jnp.zeros_like(a3)

    a1[:, :] += lax.dot_general(zis_i[:, :], zjs_j[:, :], _DN,
                                preferred_element_type=jnp.float32)
    a2[:, :] += lax.dot_general(zis_i[:, :], zis_j[:, :], _DN,
                                preferred_element_type=jnp.float32)
    a3[:, :] += lax.dot_general(zjs_i[:, :], zjs_j[:, :], _DN,
                                preferred_element_type=jnp.float32)

    @pl.when(k == nk - 1)
    def _():
        nm = (y_i[:, :] != yt_j[:, :]).astype(jnp.float32)
        e1 = jnp.exp(a1[:, :]) * nm
        e2 = jnp.exp(a2[:, :]) * nm
        e3 = jnp.exp(a3[:, :]) * nm
        neg1[pl.ds(i * bm, bm), :] += jnp.sum(e1 + e2, axis=1, keepdims=True)
        neg2[:, pl.ds(j * bn, bn)] += jnp.sum(e1 + e3, axis=0)[None, :]

        @pl.when((i == nmi - 1) & (j == nmj - 1))
        def _():
            p = pos_ref[:, :]
            pt = post_ref[:, :]
            cl_ref[0, 0] = (jnp.sum(jnp.log(jnp.exp(p) + neg1[:, :]) - p)
                            + jnp.sum(jnp.log(jnp.exp(pt) + neg2[:, :]) - pt))


def _gram(zis, zjs, y, yt, pos, post, bm=1024, bn=1024, bk=2048):
    n = zis.shape[0]
    nmi, nmj, nk = n // bm, n // bn, n // bk
    cl = pl.pallas_call(
        functools.partial(_gram_body, nmi=nmi, nmj=nmj, nk=nk, bm=bm, bn=bn),
        grid=(nmi, nmj, nk),
        in_specs=[pl.BlockSpec((bm, bk), lambda i, j, k: (i, k)),
                  pl.BlockSpec((bm, bk), lambda i, j, k: (i, k)),
                  pl.BlockSpec((bn, bk), lambda i, j, k: (j, k)),
                  pl.BlockSpec((bn, bk), lambda i, j, k: (j, k)),
                  pl.BlockSpec((bm, 1), lambda i, j, k: (i, 0)),
                  pl.BlockSpec((1, bn), lambda i, j, k: (0, j)),
                  pl.BlockSpec((n, 1), lambda i, j, k: (0, 0)),
                  pl.BlockSpec((1, n), lambda i, j, k: (0, 0))],
        out_specs=pl.BlockSpec((1, 1), lambda i, j, k: (0, 0),
                               memory_space=pltpu.SMEM),
        out_shape=jax.ShapeDtypeStruct((1, 1), jnp.float32),
        scratch_shapes=[pltpu.VMEM((bm, bn), jnp.float32),
                        pltpu.VMEM((bm, bn), jnp.float32),
                        pltpu.VMEM((bm, bn), jnp.float32),
                        pltpu.VMEM((n, 1), jnp.float32),
                        pltpu.VMEM((1, n), jnp.float32)],
    )(zis, zis, zjs, zjs, y, yt, pos, post)
    return cl[0, 0]


def _edge_dots(h, s, r):
    n, d = h.shape
    e = s.shape[0]
    info = plsc.get_sparse_core_info()
    nw = info.num_cores * info.num_subcores
    per_w = e // nw
    ch = 128
    nch = per_w // ch
    mesh = plsc.VectorSubcoreMesh(core_axis_name="c", subcore_axis_name="s")

    def body(h_hbm, s_hbm, r_hbm, out_hbm, sidx, ridx, arow, brow, ovec,
             sem1, sem2):
        wid = lax.axis_index("s") * info.num_cores + lax.axis_index("c")

        def chunk(c, carry):
            base = wid * per_w + c * ch
            pltpu.sync_copy(s_hbm.at[pl.ds(base, ch)], sidx)
            pltpu.sync_copy(r_hbm.at[pl.ds(base, ch)], ridx)
            cp1 = pltpu.async_copy(h_hbm.at[sidx], arow, sem1)
            cp2 = pltpu.async_copy(h_hbm.at[ridx], brow, sem2)
            cp1.wait()
            cp2.wait()

            def edge(eo, cc):
                for sub in range(8):
                    ei = eo * 8 + sub
                    acc = arow[ei, pl.ds(0, 16)] * brow[ei, pl.ds(0, 16)]
                    for g in range(1, d // 16):
                        acc = acc + (arow[ei, pl.ds(g * 16, 16)]
                                     * brow[ei, pl.ds(g * 16, 16)])
                    ovec[eo, pl.ds(sub * 16, 16)] = acc
                return cc

            lax.fori_loop(0, ch // 8, edge, 0)
            obase = pl.multiple_of(base // 8, 8)
            pltpu.sync_copy(ovec, out_hbm.at[pl.ds(obase, ch // 8)])
            return carry

        lax.fori_loop(0, nch, chunk, 0)

    return pl.kernel(
        body,
        out_type=jax.ShapeDtypeStruct((e // 8, 128), jnp.float32),
        mesh=mesh,
        scratch_types=[pltpu.VMEM((ch,), jnp.int32),
                       pltpu.VMEM((ch,), jnp.int32),
                       pltpu.VMEM((ch, d), jnp.float32),
                       pltpu.VMEM((ch, d), jnp.float32),
                       pltpu.VMEM((ch // 8, 128), jnp.float32),
                       pltpu.SemaphoreType.DMA,
                       pltpu.SemaphoreType.DMA],
    )(h, s, r)


def _st_body(d1, d2, d3, d4, o_ref):
    i = pl.program_id(0)

    @pl.when(i == 0)
    def _():
        o_ref[0, 0] = 0.0

    seg = (lax.broadcasted_iota(jnp.int32, (128, 128), 0) // 16
           == lax.broadcasted_iota(jnp.int32, (128, 128), 1)
           ).astype(jnp.float32)
    colmask = lax.broadcasted_iota(jnp.int32, d1.shape, 1) < 8
    tot = 0.0
    for dref in (d1, d2, d3, d4):
        dot = jnp.dot(dref[:, :], seg, preferred_element_type=jnp.float32)
        sp = jnp.maximum(-dot, 0.0) + jnp.log(1.0 + jnp.exp(-jnp.abs(dot)))
        tot += jnp.sum(jnp.where(colmask, sp, 0.0))
    o_ref[0, 0] += tot


def _st_reduce(d1, d2, d3, d4, be=4096):
    e8 = d1.shape[0]
    out = pl.pallas_call(
        _st_body,
        grid=(e8 // be,),
        in_specs=[pl.BlockSpec((be, 128), lambda i: (i, 0))] * 4,
        out_specs=pl.BlockSpec((1, 1), lambda i: (0, 0),
                               memory_space=pltpu.SMEM),
        out_shape=jax.ShapeDtypeStruct((1, 1), jnp.float32),
    )(d1, d2, d3, d4)
    return out[0, 0]


def kernel(X, A, S, R, X2, A2, S2, R2, y_pred, Theta,
           weight, weight2, weight31, weight32,
           W11, W12, Wd11, Wd12, W21, W22, Wd21, Wd22, W31, Wd31):
    n, f1 = X.shape
    f2 = X2.shape[1]
    h2 = W12.shape[1]
    h3 = W31.shape[1]

    pad = 128 - h3
    W31p = jnp.pad(W31, ((0, 0), (0, pad)))
    Wd31p = jnp.pad(Wd31, ((0, pad), (0, 0)))

    E1, Ahl = _amm_elu_split(A, _mm(X, W11))
    H = _amm_elu(Ahl, _mm(E1, W12))
    E2, A2hl = _amm_elu_split(A2, _mm(X2, W21))
    Hb = _amm_elu(A2hl, _mm(E2, W22))

    d1 = _edge_dots(H, S, R)
    d2 = _edge_dots(Hb, S2, R2)

    (c3, zis, zjs, pos, dw, dw2, dw31, dw32, creg, cq, cons) = _prep(
        weight, weight2, weight31, weight32, Theta.T)

    h1dim = Wd11.shape[1]
    HC, se1, HC2, se2 = _coef_mm2(weight, H, dw, weight2, Hb, dw2)
    dec1, H31 = _amm_elu2(Ahl, _mm2(HC, Wd11, H, W31p), h1dim)
    d3 = _edge_dots(H31, S, R)
    dec2, H32 = _amm_elu2(A2hl, _mm2(HC2, Wd21, Hb, W31p), h1dim)
    d4 = _edge_dots(H32, S2, R2)
    HC31, se3, HC32, se4 = _coef_mm2(weight31, H31, dw31, weight32, H32, dw32)
    ft1, ft3 = _amm_elu_ft2(Ahl, _mm2(dec1, Wd12, HC31, Wd31p), X, H)
    ft2, ft4 = _amm_elu_ft2(A2hl, _mm2(dec2, Wd22, HC32, Wd31p), X2, Hb)

    yt = y_pred.reshape(1, n)
    cl_sum = _gram(zis, zjs, y_pred, yt, pos, pos.reshape(1, n))

    st_loss = _st_reduce(d1, d2, d3, d4)

    ft_loss = (ft1 / (n * f1) + ft2 / (n * f2)
               + ft3 / (n * h2) + ft4 / (n * h2))
    se_loss = 0.5 * (se1 / (n * h2) + se2 / (n * h2)
                     + se3 / (n * h3) + se4 / (n * h3))
    cl_loss = cl_sum / (2.0 * n)

    loss = (ft_loss + 0.1 * st_loss + se_loss + 0.1 * creg
            + 0.1 * cl_loss + 0.1 * cq + 0.1 * cons)
    return (loss, ft_loss, st_loss, se_loss, creg, cons, cl_loss, cq, c3)
